# Initial kernel scaffold; baseline (speedup 1.0000x reference)
import jax
import jax.numpy as jnp
from jax.experimental import pallas as pl


def kernel(x, edge_index, W_gat1, att_src1, att_dst1, b_gat1, bn1_gamma, bn1_beta, bn1_mean, bn1_var, W_gat2, att_src2, att_dst2, b_gat2, Wl1, bl1, Wr1, bns_gamma, bns_beta, bns_mean, bns_var, Wl2, bl2, Wr2, Wf1, bf1, Wf2, bf2):
    raise NotImplementedError("write your pallas kernel here")


# speedup vs baseline: 25.1442x; 25.1442x over previous
import jax
import jax.numpy as jnp
from jax import lax
from jax.experimental import pallas as pl
from jax.experimental.pallas import tpu as pltpu
from jax.experimental.pallas import tpu_sc as plsc

N = 10000
NP = 10240
E = 320000
F_IN = 128
HID = 64
HEADS = 4
EP = 331776
CH1 = 64
CH2 = 128
NCHUNK1 = EP // CH1
NCHUNK2 = EP // CH2
REAL1 = E // CH1
REAL2 = E // CH2
NSC = 2
NTILE = 16
STRIPE = NP // NTILE
RB = 1024
GRID = NP // RB

_f32 = jnp.float32
_SC_PARAMS = dict(
    compiler_params=pltpu.CompilerParams(
        needs_layout_passes=False, use_tc_tiling_on_sc=False),
)


def _tc1_body(x_ref, wg1_ref, as_ref, ad_ref, wl1_ref, wr1_ref,
              h1p_ref, t1s_ref, t1d_ref, xwl_ref, xwr_ref):
    xb = x_ref[...]
    h1 = jnp.dot(xb, wg1_ref[...], preferred_element_type=_f32)
    h1p_ref[0] = h1[:, :128]
    h1p_ref[1] = h1[:, 128:]
    t1s_ref[...] = jnp.dot(h1, as_ref[...], preferred_element_type=_f32)
    t1d = jnp.dot(h1, ad_ref[...], preferred_element_type=_f32)
    i = pl.program_id(0)
    rows = lax.broadcasted_iota(jnp.int32, (RB, 16), 0) + i * RB
    lanev = lax.broadcasted_iota(jnp.int32, (RB, 16), 1)
    valid = rows < N
    t1d_ref[...] = jnp.where(
        valid, t1d + (lanev == 4).astype(_f32),
        jnp.where(lanev < 4, -1e30, 0.0))
    xwl = jnp.dot(xb, wl1_ref[...], preferred_element_type=_f32)
    xwl_ref[0] = xwl[:, :32]
    xwl_ref[1] = xwl[:, 32:]
    xwr_ref[...] = jnp.dot(xb, wr1_ref[...], preferred_element_type=_f32)


def _tc1(xp, wg1, as_mat, ad_mat, wl1, wr1):
    full = lambda shape: pl.BlockSpec(shape, lambda i: (0,) * len(shape))
    return pl.pallas_call(
        _tc1_body,
        grid=(GRID,),
        in_specs=[
            pl.BlockSpec((RB, F_IN), lambda i: (i, 0)),
            full((F_IN, 256)), full((256, 16)), full((256, 16)),
            full((F_IN, HID)), full((F_IN, HID)),
        ],
        out_specs=[
            pl.BlockSpec((2, RB, 128), lambda i: (0, i, 0)),
            pl.BlockSpec((RB, 16), lambda i: (i, 0)),
            pl.BlockSpec((RB, 16), lambda i: (i, 0)),
            pl.BlockSpec((2, RB, 32), lambda i: (0, i, 0)),
            pl.BlockSpec((RB, HID), lambda i: (i, 0)),
        ],
        out_shape=[
            jax.ShapeDtypeStruct((2, NP, 128), _f32),
            jax.ShapeDtypeStruct((NP, 16), _f32),
            jax.ShapeDtypeStruct((NP, 16), _f32),
            jax.ShapeDtypeStruct((2, NP, 32), _f32),
            jax.ShapeDtypeStruct((NP, HID), _f32),
        ],
    )(xp, wg1, as_mat, ad_mat, wl1, wr1)


def _sc1_body(src_ref, dst_ref, t1s_ref, t1d_ref, h1p_ref, xwl_ref,
              den_o, acc1_o, accs_o,
              idx_src, idx_dst, idx_adj, ts_rows, td_rows, e_rows, e_flat,
              h_rows, s_rows, den_sh, acc1_sh, accs_sh, sem0, sem1):
    c = lax.axis_index("c")
    s = lax.axis_index("s")

    def zrow(j, _):
        z = jnp.zeros((16,), _f32)
        e_rows[j, :] = z
        for k in range(8):
            h_rows[j, pl.ds(k * 16, 16)] = z
        for k in range(2):
            s_rows[j, pl.ds(k * 16, 16)] = z
        return 0
    lax.fori_loop(0, CH1, zrow, 0)

    def zstripe(k, _):
        base = s * STRIPE + k * CH1
        pltpu.sync_copy(e_rows, den_sh.at[pl.ds(base, CH1)])
        pltpu.sync_copy(h_rows, acc1_sh.at[pl.ds(base, CH1)])
        pltpu.sync_copy(s_rows, accs_sh.at[pl.ds(base, CH1)])
        return 0
    lax.fori_loop(0, STRIPE // CH1, zstripe, 0)
    plsc.subcore_barrier()

    lanev = lax.iota(jnp.int32, 16)
    zero16i = jnp.zeros((16,), jnp.int32)
    idxh0 = zero16i + 2 * c
    idxh1 = idxh0 + 1
    coff = c * NP
    nct = NCHUNK1 // NTILE

    def chunk_body(ci, _):
        g = s * nct + ci
        base = g * CH1
        pltpu.sync_copy(src_ref.at[pl.ds(base, CH1)], idx_src)
        pltpu.sync_copy(dst_ref.at[pl.ds(base, CH1)], idx_dst)
        cp1 = pltpu.async_copy(t1s_ref.at[idx_src], ts_rows, sem0)
        cp2 = pltpu.async_copy(t1d_ref.at[idx_dst], td_rows, sem1)
        cp1.wait()
        cp2.wait()
        realf = jnp.where(g < REAL1, 1.0, 0.0).astype(_f32)

        def ebody(j, _):
            al = ts_rows[j, :] + td_rows[j, :]
            lr = jnp.where(al > 0, al, 0.2 * al)
            ev = jnp.exp(lr)
            er = jnp.where(
                lanev < 4, ev, jnp.where(lanev == 4, al * realf, 0.0))
            e_rows[j, :] = er
            e_flat[pl.ds(j * 16, 16)] = er
            return 0
        lax.fori_loop(0, CH1, ebody, 0)
        pltpu.sync_copy(e_rows, den_sh.at[idx_dst], add=True)

        for k in range(CH1 // 16):
            idx_adj[pl.ds(k * 16, 16)] = idx_src[pl.ds(k * 16, 16)] + coff
        pltpu.async_copy(h1p_ref.at[idx_adj], h_rows, sem0).wait()

        def sbody(j, _):
            jv = zero16i + j * 16
            w0 = plsc.load_gather(e_flat, [jv + idxh0])
            w1 = plsc.load_gather(e_flat, [jv + idxh1])
            for k in range(4):
                h_rows[j, pl.ds(k * 16, 16)] = h_rows[j, pl.ds(k * 16, 16)] * w0
            for k in range(4, 8):
                h_rows[j, pl.ds(k * 16, 16)] = h_rows[j, pl.ds(k * 16, 16)] * w1
            return 0
        lax.fori_loop(0, CH1, sbody, 0)
        pltpu.sync_copy(h_rows, acc1_sh.at[idx_dst], add=True)

        @pl.when(g < REAL1)
        def _():
            pltpu.async_copy(xwl_ref.at[idx_adj], s_rows, sem0).wait()
            pltpu.sync_copy(s_rows, accs_sh.at[idx_dst], add=True)
        return 0
    lax.fori_loop(0, nct, chunk_body, 0)
    plsc.subcore_barrier()

    rbase = s * STRIPE
    obase = c * NP + s * STRIPE
    pltpu.sync_copy(den_sh.at[pl.ds(rbase, STRIPE)], den_o.at[pl.ds(obase, STRIPE)])
    pltpu.sync_copy(acc1_sh.at[pl.ds(rbase, STRIPE)], acc1_o.at[pl.ds(obase, STRIPE)])
    pltpu.sync_copy(accs_sh.at[pl.ds(rbase, STRIPE)], accs_o.at[pl.ds(obase, STRIPE)])


def _sc1(src_sl, dst_sl, t1s, t1d, h1p, xwl):
    mesh = plsc.VectorSubcoreMesh(core_axis_name="c", subcore_axis_name="s",
                                  num_cores=NSC, num_subcores=NTILE)
    return pl.kernel(
        _sc1_body,
        out_type=(
            jax.ShapeDtypeStruct((2 * NP, 16), _f32),
            jax.ShapeDtypeStruct((2 * NP, 128), _f32),
            jax.ShapeDtypeStruct((2 * NP, 32), _f32),
        ),
        mesh=mesh,
        **_SC_PARAMS,
        scratch_types=[
            pltpu.VMEM((CH1,), jnp.int32),
            pltpu.VMEM((CH1,), jnp.int32),
            pltpu.VMEM((CH1,), jnp.int32),
            pltpu.VMEM((CH1, 16), _f32),
            pltpu.VMEM((CH1, 16), _f32),
            pltpu.VMEM((CH1, 16), _f32),
            pltpu.VMEM((CH1 * 16,), _f32),
            pltpu.VMEM((CH1, 128), _f32),
            pltpu.VMEM((CH1, 32), _f32),
            pltpu.VMEM_SHARED((NP, 16), _f32),
            pltpu.VMEM_SHARED((NP, 128), _f32),
            pltpu.VMEM_SHARED((NP, 32), _f32),
            pltpu.SemaphoreType.DMA,
            pltpu.SemaphoreType.DMA,
        ],
    )(src_sl, dst_sl, t1s, t1d, h1p, xwl)


def _tc2_body(acc1a_ref, acc1b_ref, den_ref, accsa_ref, accsb_ref, xwr_ref,
              bg1_ref, s1c_ref, s1h_ref, wg2_ref, as2_ref, ad2_ref,
              bl1_ref, ssc_ref, ssh_ref, wl2_ref, wr2_ref,
              h2p_ref, t2s_ref, t2d_ref, s1wl2_ref, s1wr2_ref):
    den = den_ref[...]
    mcnt = jnp.maximum(den[:, 4:5], 1.0)
    a = acc1a_ref[...]
    b = acc1b_ref[...]
    g1 = jnp.concatenate([
        a[:, :64] / (den[:, 0:1] + 1e-16),
        a[:, 64:] / (den[:, 1:2] + 1e-16),
        b[:, :64] / (den[:, 2:3] + 1e-16),
        b[:, 64:] / (den[:, 3:4] + 1e-16)], axis=1)
    g1 = g1 + bg1_ref[...]
    g1b = g1 * s1c_ref[...] + s1h_ref[...]
    g1e = jnp.where(g1b > 0, g1b, jnp.exp(g1b) - 1.0)
    h2 = jnp.dot(g1e, wg2_ref[...], preferred_element_type=_f32)
    h2p_ref[...] = h2
    t2s_ref[...] = jnp.dot(h2, as2_ref[...], preferred_element_type=_f32)
    t2d = jnp.dot(h2, ad2_ref[...], preferred_element_type=_f32)
    i = pl.program_id(0)
    rows = lax.broadcasted_iota(jnp.int32, (RB, 16), 0) + i * RB
    lanev = lax.broadcasted_iota(jnp.int32, (RB, 16), 1)
    t2d_ref[...] = jnp.where(
        rows < N, t2d, jnp.where(lanev < 1, -1e30, 0.0))
    accs = jnp.concatenate([accsa_ref[...], accsb_ref[...]], axis=1)
    s1 = accs / mcnt + bl1_ref[...] + xwr_ref[...]
    s1b = s1 * ssc_ref[...] + ssh_ref[...]
    s1r = jnp.maximum(s1b, 0.0)
    s1wl2_ref[...] = jnp.dot(s1r, wl2_ref[...], preferred_element_type=_f32)
    s1wr2_ref[...] = jnp.dot(s1r, wr2_ref[...], preferred_element_type=_f32)


def _tc2(den_acc, acc1, accs, xwr, bg1, bn1_scale, bn1_shift, wg2, as2_mat,
         ad2_mat, bl1, bns_scale, bns_shift, wl2, wr2):
    full = lambda shape: pl.BlockSpec(shape, lambda i: (0,) * len(shape))
    blk = lambda w: pl.BlockSpec((RB, w), lambda i: (i, 0))
    blk_hi = lambda w: pl.BlockSpec((RB, w), lambda i: (i + GRID, 0))
    return pl.pallas_call(
        _tc2_body,
        grid=(GRID,),
        in_specs=[
            blk(128), blk_hi(128), blk(16), blk(32), blk_hi(32), blk(HID),
            full((1, 256)), full((1, 256)), full((1, 256)),
            full((256, HID)), full((HID, 16)), full((HID, 16)),
            full((1, HID)), full((1, HID)), full((1, HID)),
            full((HID, HID)), full((HID, HID)),
        ],
        out_specs=[blk(HID), blk(16), blk(16), blk(HID), blk(HID)],
        out_shape=[
            jax.ShapeDtypeStruct((NP, HID), _f32),
            jax.ShapeDtypeStruct((NP, 16), _f32),
            jax.ShapeDtypeStruct((NP, 16), _f32),
            jax.ShapeDtypeStruct((NP, HID), _f32),
            jax.ShapeDtypeStruct((NP, HID), _f32),
        ],
    )(acc1, acc1, den_acc, accs, accs, xwr, bg1, bn1_scale, bn1_shift,
      wg2, as2_mat, ad2_mat, bl1, bns_scale, bns_shift, wl2, wr2)


def _sc2_body(src_ref, dst_ref, t2s_ref, t2d_ref, h2p_ref, swl_ref,
              den_o, acc2_o, accs2_o,
              idx_src, idx_dst, ts_rows, td_rows, e_rows, e_flat,
              h_rows, s_rows, den_sh, acc2_sh, accs2_sh, sem0, sem1):
    c = lax.axis_index("c")
    s = lax.axis_index("s")

    def zrow(j, _):
        z = jnp.zeros((16,), _f32)
        e_rows[j, :] = z
        for k in range(4):
            h_rows[j, pl.ds(k * 16, 16)] = z
        return 0
    lax.fori_loop(0, CH2, zrow, 0)

    def zstripe(k, _):
        base = s * STRIPE + k * CH2
        pltpu.sync_copy(e_rows, den_sh.at[pl.ds(base, CH2)])
        pltpu.sync_copy(h_rows, acc2_sh.at[pl.ds(base, CH2)])
        pltpu.sync_copy(h_rows, accs2_sh.at[pl.ds(base, CH2)])
        return 0
    lax.fori_loop(0, STRIPE // CH2, zstripe, 0)
    plsc.subcore_barrier()

    lanev = lax.iota(jnp.int32, 16)
    zero16i = jnp.zeros((16,), jnp.int32)
    nw = NCHUNK2 // (NSC * NTILE)
    wid = c * NTILE + s

    def chunk_body(ci, _):
        g = wid * nw + ci
        base = g * CH2
        pltpu.sync_copy(src_ref.at[pl.ds(base, CH2)], idx_src)
        pltpu.sync_copy(dst_ref.at[pl.ds(base, CH2)], idx_dst)
        cp1 = pltpu.async_copy(t2s_ref.at[idx_src], ts_rows, sem0)
        cp2 = pltpu.async_copy(t2d_ref.at[idx_dst], td_rows, sem1)
        cp1.wait()
        cp2.wait()

        def ebody(j, _):
            al = ts_rows[j, :] + td_rows[j, :]
            lr = jnp.where(al > 0, al, 0.2 * al)
            ev = jnp.exp(lr)
            er = jnp.where(lanev < 1, ev, 0.0)
            e_rows[j, :] = er
            e_flat[pl.ds(j * 16, 16)] = er
            return 0
        lax.fori_loop(0, CH2, ebody, 0)
        pltpu.sync_copy(e_rows, den_sh.at[idx_dst], add=True)

        pltpu.async_copy(h2p_ref.at[idx_src], h_rows, sem0).wait()

        def sbody(j, _):
            jv = zero16i + j * 16
            w0 = plsc.load_gather(e_flat, [jv])
            for k in range(4):
                h_rows[j, pl.ds(k * 16, 16)] = h_rows[j, pl.ds(k * 16, 16)] * w0
            return 0
        lax.fori_loop(0, CH2, sbody, 0)
        pltpu.sync_copy(h_rows, acc2_sh.at[idx_dst], add=True)

        @pl.when(g < REAL2)
        def _():
            pltpu.async_copy(swl_ref.at[idx_src], s_rows, sem0).wait()
            pltpu.sync_copy(s_rows, accs2_sh.at[idx_dst], add=True)
        return 0
    lax.fori_loop(0, nw, chunk_body, 0)
    plsc.subcore_barrier()

    rbase = s * STRIPE
    obase = c * NP + s * STRIPE
    pltpu.sync_copy(den_sh.at[pl.ds(rbase, STRIPE)], den_o.at[pl.ds(obase, STRIPE)])
    pltpu.sync_copy(acc2_sh.at[pl.ds(rbase, STRIPE)], acc2_o.at[pl.ds(obase, STRIPE)])
    pltpu.sync_copy(accs2_sh.at[pl.ds(rbase, STRIPE)], accs2_o.at[pl.ds(obase, STRIPE)])


def _sc2(src_sl, dst_sl, t2s, t2d, h2p, s1wl2):
    mesh = plsc.VectorSubcoreMesh(core_axis_name="c", subcore_axis_name="s",
                                  num_cores=NSC, num_subcores=NTILE)
    return pl.kernel(
        _sc2_body,
        out_type=(
            jax.ShapeDtypeStruct((2 * NP, 16), _f32),
            jax.ShapeDtypeStruct((2 * NP, HID), _f32),
            jax.ShapeDtypeStruct((2 * NP, HID), _f32),
        ),
        mesh=mesh,
        **_SC_PARAMS,
        scratch_types=[
            pltpu.VMEM((CH2,), jnp.int32),
            pltpu.VMEM((CH2,), jnp.int32),
            pltpu.VMEM((CH2, 16), _f32),
            pltpu.VMEM((CH2, 16), _f32),
            pltpu.VMEM((CH2, 16), _f32),
            pltpu.VMEM((CH2 * 16,), _f32),
            pltpu.VMEM((CH2, HID), _f32),
            pltpu.VMEM((CH2, HID), _f32),
            pltpu.VMEM_SHARED((NP, 16), _f32),
            pltpu.VMEM_SHARED((NP, HID), _f32),
            pltpu.VMEM_SHARED((NP, HID), _f32),
            pltpu.SemaphoreType.DMA,
            pltpu.SemaphoreType.DMA,
        ],
    )(src_sl, dst_sl, t2s, t2d, h2p, s1wl2)


def _tc3_body(acc2a_ref, acc2b_ref, den2a_ref, den2b_ref, accs2a_ref,
              accs2b_ref, s1wr2_ref, den_ref, bg2_ref, bl2_ref, wf1_ref,
              bf1_ref, wf2_ref, bf2_ref, out_ref):
    den2 = den2a_ref[...] + den2b_ref[...]
    g2 = (acc2a_ref[...] + acc2b_ref[...]) / (den2[:, 0:1] + 1e-16)
    g2 = g2 + bg2_ref[...]
    mcnt = jnp.maximum(den_ref[:, 4:5], 1.0)
    s2 = (accs2a_ref[...] + accs2b_ref[...]) / mcnt + bl2_ref[...] + s1wr2_ref[...]
    cc = jnp.concatenate([g2, s2], axis=1)
    h = jnp.maximum(jnp.dot(cc, wf1_ref[...], preferred_element_type=_f32)
                    + bf1_ref[...], 0.0)
    out_ref[...] = jnp.dot(h, wf2_ref[...], preferred_element_type=_f32) + bf2_ref[...]


def _tc3(acc2, den2, accs2, s1wr2, den_acc, bg2, bl2, wf1, bf1, wf2p, bf2p):
    full = lambda shape: pl.BlockSpec(shape, lambda i: (0,) * len(shape))
    blk = lambda w: pl.BlockSpec((RB, w), lambda i: (i, 0))
    blk_hi = lambda w: pl.BlockSpec((RB, w), lambda i: (i + GRID, 0))
    return pl.pallas_call(
        _tc3_body,
        grid=(GRID,),
        in_specs=[
            blk(HID), blk_hi(HID), blk(16), blk_hi(16), blk(HID), blk_hi(HID),
            blk(HID), blk(16),
            full((1, HID)), full((1, HID)), full((2 * HID, HID)),
            full((1, HID)), full((HID, 128)), full((1, 128)),
        ],
        out_specs=[pl.BlockSpec((RB, 128), lambda i: (i, 0))],
        out_shape=[jax.ShapeDtypeStruct((N, 128), _f32)],
    )(acc2, acc2, den2, den2, accs2, accs2, s1wr2, den_acc, bg2, bl2, wf1,
      bf1, wf2p, bf2p)


@jax.jit
def kernel(x, edge_index, W_gat1, att_src1, att_dst1, b_gat1, bn1_gamma,
           bn1_beta, bn1_mean, bn1_var, W_gat2, att_src2, att_dst2, b_gat2,
           Wl1, bl1, Wr1, bns_gamma, bns_beta, bns_mean, bns_var, Wl2, bl2,
           Wr2, Wf1, bf1, Wf2, bf2):
    src = edge_index[0].astype(jnp.int32)
    dst = edge_index[1].astype(jnp.int32)
    loops = jnp.arange(N, dtype=jnp.int32)
    padidx = (N + (jnp.arange(EP - E - N, dtype=jnp.int32) % (NP - N)))
    src_sl = jnp.concatenate([src, loops, padidx])
    dst_sl = jnp.concatenate([dst, loops, padidx])
    xp = jnp.pad(x, ((0, NP - N), (0, 0)))

    eye4 = jnp.eye(HEADS, dtype=_f32)
    as_mat = (att_src1[:, :, None] * eye4[:, None, :]).reshape(256, HEADS)
    as_mat = jnp.concatenate([as_mat, jnp.zeros((256, 12), _f32)], axis=1)
    ad_mat = (att_dst1[:, :, None] * eye4[:, None, :]).reshape(256, HEADS)
    ad_mat = jnp.concatenate([ad_mat, jnp.zeros((256, 12), _f32)], axis=1)
    as2_mat = jnp.concatenate([att_src2.T, jnp.zeros((HID, 15), _f32)], axis=1)
    ad2_mat = jnp.concatenate([att_dst2.T, jnp.zeros((HID, 15), _f32)], axis=1)

    bn1_scale = (bn1_gamma / jnp.sqrt(bn1_var + 1e-5)).reshape(1, 256)
    bn1_shift = (bn1_beta - bn1_mean * bn1_scale[0]).reshape(1, 256)
    bns_scale = (bns_gamma / jnp.sqrt(bns_var + 1e-5)).reshape(1, HID)
    bns_shift = (bns_beta - bns_mean * bns_scale[0]).reshape(1, HID)

    h1p3, t1s, t1d, xwl3, xwr = _tc1(xp, W_gat1, as_mat, ad_mat, Wl1, Wr1)
    h1p = h1p3.reshape(2 * NP, 128)
    xwl = xwl3.reshape(2 * NP, 32)

    den_o, acc1_o, accs_o = _sc1(src_sl, dst_sl, t1s, t1d, h1p, xwl)
    den_acc = den_o[:NP]

    h2p, t2s, t2d, s1wl2, s1wr2 = _tc2(
        den_acc, acc1_o, accs_o, xwr, b_gat1.reshape(1, 256), bn1_scale,
        bn1_shift, W_gat2, as2_mat, ad2_mat, bl1.reshape(1, HID), bns_scale,
        bns_shift, Wl2, Wr2)

    den2_o, acc2_o, accs2_o = _sc2(src_sl, dst_sl, t2s, t2d, h2p, s1wl2)

    wf2p = jnp.concatenate([Wf2, jnp.zeros((HID, 126), _f32)], axis=1)
    bf2p = jnp.concatenate([bf2, jnp.zeros((126,), _f32)]).reshape(1, 128)
    outp = _tc3(acc2_o, den2_o, accs2_o, s1wr2, den_acc,
                b_gat2.reshape(1, HID), bl2.reshape(1, HID), Wf1,
                bf1.reshape(1, HID), wf2p, bf2p)[0]
    return outp[:, :2]


# speedup vs baseline: 51.9407x; 2.0657x over previous
import jax
import jax.numpy as jnp
from jax import lax
from jax.experimental import pallas as pl
from jax.experimental.pallas import tpu as pltpu
from jax.experimental.pallas import tpu_sc as plsc

N = 10000
NP = 10240
E = 320000
F_IN = 128
HID = 64
HEADS = 4
EP = 331776
CH = 64
NCHUNK = EP // CH
REAL = E // CH
NSC = 2
NTILE = 16
NWORK = NSC * NTILE
STRIPE = NP // NTILE
K1 = 12
K2 = 9
RB = 1024
GRID = NP // RB

_f32 = jnp.float32
_SC_PARAMS = dict(
    compiler_params=pltpu.CompilerParams(
        needs_layout_passes=False, use_tc_tiling_on_sc=False),
)


def _sc_mesh():
    return plsc.VectorSubcoreMesh(core_axis_name="c", subcore_axis_name="s",
                                  num_cores=NSC, num_subcores=NTILE)


def _tc1_body(x_ref, wg1_ref, as_ref, ad_ref, wl1_ref, wr1_ref,
              h1p_ref, t1s_ref, t1d_ref, xwl_ref, xwr_ref):
    xb = x_ref[...]
    h1 = jnp.dot(xb, wg1_ref[...], preferred_element_type=_f32)
    h1p_ref[0] = h1[:, :128]
    h1p_ref[1] = h1[:, 128:]
    t1s_ref[...] = jnp.dot(h1, as_ref[...], preferred_element_type=_f32)
    t1d = jnp.dot(h1, ad_ref[...], preferred_element_type=_f32)
    i = pl.program_id(0)
    rows = lax.broadcasted_iota(jnp.int32, (RB, 16), 0) + i * RB
    lanev = lax.broadcasted_iota(jnp.int32, (RB, 16), 1)
    valid = rows < N
    t1d_ref[...] = jnp.where(
        valid, t1d + (lanev == 4).astype(_f32),
        jnp.where(lanev < 4, -1e30, 0.0))
    xwl_ref[...] = jnp.dot(xb, wl1_ref[...], preferred_element_type=_f32)
    xwr_ref[...] = jnp.dot(xb, wr1_ref[...], preferred_element_type=_f32)


def _tc1(xp, wg1, as_mat, ad_mat, wl1, wr1):
    full = lambda shape: pl.BlockSpec(shape, lambda i: (0,) * len(shape))
    return pl.pallas_call(
        _tc1_body,
        grid=(GRID,),
        in_specs=[
            pl.BlockSpec((RB, F_IN), lambda i: (i, 0)),
            full((F_IN, 256)), full((256, 16)), full((256, 16)),
            full((F_IN, HID)), full((F_IN, HID)),
        ],
        out_specs=[
            pl.BlockSpec((2, RB, 128), lambda i: (0, i, 0)),
            pl.BlockSpec((RB, 16), lambda i: (i, 0)),
            pl.BlockSpec((RB, 16), lambda i: (i, 0)),
            pl.BlockSpec((RB, HID), lambda i: (i, 0)),
            pl.BlockSpec((RB, HID), lambda i: (i, 0)),
        ],
        out_shape=[
            jax.ShapeDtypeStruct((2, NP, 128), _f32),
            jax.ShapeDtypeStruct((NP, 16), _f32),
            jax.ShapeDtypeStruct((NP, 16), _f32),
            jax.ShapeDtypeStruct((NP, HID), _f32),
            jax.ShapeDtypeStruct((NP, HID), _f32),
        ],
    )(xp, wg1, as_mat, ad_mat, wl1, wr1)


def _sc1_body(src2_ref, dst2_ref, t1s_ref, t1d_ref, h1p_ref,
              den_o, acc1_o,
              idxs_blk, idxd_blk, adj0, adj1, ts0, ts1, td0, td1,
              er0, er1, ef0, ef1, hr0, hr1,
              den_sh, acc1_sh,
              sts0, sts1, std0, std1, sh0, sh1):
    c = lax.axis_index("c")
    s = lax.axis_index("s")
    adjb = [adj0, adj1]
    tsb = [ts0, ts1]
    tdb = [td0, td1]
    erb = [er0, er1]
    efb = [ef0, ef1]
    hrb = [hr0, hr1]
    sts = [sts0, sts1]
    std = [std0, std1]
    sh = [sh0, sh1]

    def zrow(j, _):
        z = jnp.zeros((16,), _f32)
        er0[j, :] = z
        for k in range(8):
            hr0[j, pl.ds(k * 16, 16)] = z
        return 0
    lax.fori_loop(0, CH, zrow, 0)

    def zstripe(k, _):
        base = s * STRIPE + k * CH
        pltpu.sync_copy(er0, den_sh.at[pl.ds(base, CH)])
        pltpu.sync_copy(hr0, acc1_sh.at[pl.ds(base, CH)])
        return 0
    lax.fori_loop(0, STRIPE // CH, zstripe, 0)
    plsc.subcore_barrier()

    lanev = lax.iota(jnp.int32, 16)
    zero16i = jnp.zeros((16,), jnp.int32)
    idxh0v = zero16i + 2 * c
    idxh1v = idxh0v + 1
    coff = c * NP
    nct = NCHUNK // NTILE
    nblk = nct // K1

    def issue(jj, si):
        for k in range(CH // 16):
            adjb[si][pl.ds(k * 16, 16)] = (
                idxs_blk[jj, pl.ds(k * 16, 16)] + coff)
        dts = pltpu.async_copy(t1s_ref.at[idxs_blk.at[jj]], tsb[si], sts[si])
        dtd = pltpu.async_copy(t1d_ref.at[idxd_blk.at[jj]], tdb[si], std[si])
        dh = pltpu.async_copy(h1p_ref.at[adjb[si]], hrb[si], sh[si])
        return dts, dtd, dh

    def blk_body(bi, _):
        row0 = s * nct + bi * K1
        pltpu.sync_copy(src2_ref.at[pl.ds(row0, K1)], idxs_blk)
        pltpu.sync_copy(dst2_ref.at[pl.ds(row0, K1)], idxd_blk)
        d = [issue(0, 0), None]
        for j in range(K1):
            cur = j % 2
            nxt = 1 - cur
            if j + 1 < K1:
                d[nxt] = issue(j + 1, nxt)
            g = row0 + j
            realf = jnp.where(g < REAL, 1.0, 0.0).astype(_f32)
            dts, dtd, dh = d[cur]
            dts.wait()
            dtd.wait()
            ts_c, td_c, er_c, ef_c, hr_c = (
                tsb[cur], tdb[cur], erb[cur], efb[cur], hrb[cur])

            def ebody(jj, _):
                al = ts_c[jj, :] + td_c[jj, :]
                lr = jnp.where(al > 0, al, 0.2 * al)
                ev = jnp.exp(lr)
                out = jnp.where(
                    lanev < 4, ev, jnp.where(lanev == 4, al * realf, 0.0))
                er_c[jj, :] = out
                ef_c[pl.ds(jj * 16, 16)] = out
                return 0
            lax.fori_loop(0, CH, ebody, 0)
            pltpu.sync_copy(er_c, den_sh.at[idxd_blk.at[j]], add=True)
            dh.wait()

            def sbody(jj, _):
                jv = zero16i + jj * 16
                w0 = plsc.load_gather(ef_c, [jv + idxh0v])
                w1 = plsc.load_gather(ef_c, [jv + idxh1v])
                for k in range(4):
                    hr_c[jj, pl.ds(k * 16, 16)] = (
                        hr_c[jj, pl.ds(k * 16, 16)] * w0)
                for k in range(4, 8):
                    hr_c[jj, pl.ds(k * 16, 16)] = (
                        hr_c[jj, pl.ds(k * 16, 16)] * w1)
                return 0
            lax.fori_loop(0, CH, sbody, 0)
            pltpu.sync_copy(hr_c, acc1_sh.at[idxd_blk.at[j]], add=True)
        return 0
    lax.fori_loop(0, nblk, blk_body, 0)
    plsc.subcore_barrier()

    rbase = s * STRIPE
    obase = c * NP + rbase
    pltpu.sync_copy(den_sh.at[pl.ds(rbase, STRIPE)], den_o.at[pl.ds(obase, STRIPE)])
    pltpu.sync_copy(acc1_sh.at[pl.ds(rbase, STRIPE)], acc1_o.at[pl.ds(obase, STRIPE)])


def _sc1(src2, dst2, t1s, t1d, h1p):
    return pl.kernel(
        _sc1_body,
        out_type=(
            jax.ShapeDtypeStruct((2 * NP, 16), _f32),
            jax.ShapeDtypeStruct((2 * NP, 128), _f32),
        ),
        mesh=_sc_mesh(),
        **_SC_PARAMS,
        scratch_types=[
            pltpu.VMEM((K1, CH), jnp.int32),
            pltpu.VMEM((K1, CH), jnp.int32),
            pltpu.VMEM((CH,), jnp.int32),
            pltpu.VMEM((CH,), jnp.int32),
            pltpu.VMEM((CH, 16), _f32),
            pltpu.VMEM((CH, 16), _f32),
            pltpu.VMEM((CH, 16), _f32),
            pltpu.VMEM((CH, 16), _f32),
            pltpu.VMEM((CH, 16), _f32),
            pltpu.VMEM((CH, 16), _f32),
            pltpu.VMEM((CH * 16,), _f32),
            pltpu.VMEM((CH * 16,), _f32),
            pltpu.VMEM((CH, 128), _f32),
            pltpu.VMEM((CH, 128), _f32),
            pltpu.VMEM_SHARED((NP, 16), _f32),
            pltpu.VMEM_SHARED((NP, 128), _f32),
            pltpu.SemaphoreType.DMA,
            pltpu.SemaphoreType.DMA,
            pltpu.SemaphoreType.DMA,
            pltpu.SemaphoreType.DMA,
            pltpu.SemaphoreType.DMA,
            pltpu.SemaphoreType.DMA,
        ],
    )(src2, dst2, t1s, t1d, h1p)


def _sc1b_body(src2_ref, dst2_ref, xwl_ref, accs_o,
               idxs_blk, idxd_blk, sg0, sg1, accs_sh, ss0, ss1):
    c = lax.axis_index("c")
    s = lax.axis_index("s")
    sgb = [sg0, sg1]
    ssb = [ss0, ss1]

    def zrow(j, _):
        z = jnp.zeros((16,), _f32)
        for k in range(4):
            sg0[j, pl.ds(k * 16, 16)] = z
        return 0
    lax.fori_loop(0, CH, zrow, 0)

    def zstripe(k, _):
        pltpu.sync_copy(sg0, accs_sh.at[pl.ds(s * STRIPE + k * CH, CH)])
        return 0
    lax.fori_loop(0, STRIPE // CH, zstripe, 0)
    plsc.subcore_barrier()

    nct = NCHUNK // NWORK
    nblk = nct // K2
    wid = c * NTILE + s

    def blk_body(bi, _):
        row0 = wid * nct + bi * K2
        pltpu.sync_copy(src2_ref.at[pl.ds(row0, K2)], idxs_blk)
        pltpu.sync_copy(dst2_ref.at[pl.ds(row0, K2)], idxd_blk)
        d = [pltpu.async_copy(xwl_ref.at[idxs_blk.at[0]], sg0, ss0), None]
        for j in range(K2):
            cur = j % 2
            nxt = 1 - cur
            if j + 1 < K2:
                d[nxt] = pltpu.async_copy(
                    xwl_ref.at[idxs_blk.at[j + 1]], sgb[nxt], ssb[nxt])
            g = row0 + j
            d[cur].wait()

            @pl.when(g < REAL)
            def _():
                pltpu.sync_copy(sgb[cur], accs_sh.at[idxd_blk.at[j]], add=True)
        return 0
    lax.fori_loop(0, nblk, blk_body, 0)
    plsc.subcore_barrier()

    rbase = s * STRIPE
    pltpu.sync_copy(accs_sh.at[pl.ds(rbase, STRIPE)],
                    accs_o.at[pl.ds(c * NP + rbase, STRIPE)])


def _sc1b(src2, dst2, xwl):
    return pl.kernel(
        _sc1b_body,
        out_type=jax.ShapeDtypeStruct((2 * NP, HID), _f32),
        mesh=_sc_mesh(),
        **_SC_PARAMS,
        scratch_types=[
            pltpu.VMEM((K2, CH), jnp.int32),
            pltpu.VMEM((K2, CH), jnp.int32),
            pltpu.VMEM((CH, HID), _f32),
            pltpu.VMEM((CH, HID), _f32),
            pltpu.VMEM_SHARED((NP, HID), _f32),
            pltpu.SemaphoreType.DMA,
            pltpu.SemaphoreType.DMA,
        ],
    )(src2, dst2, xwl)


def _tc2_body(acc1a_ref, acc1b_ref, den_ref, accsa_ref, accsb_ref, xwr_ref,
              bg1_ref, s1c_ref, s1h_ref, wg2_ref, as2_ref, ad2_ref,
              bl1_ref, ssc_ref, ssh_ref, wl2_ref, wr2_ref,
              h2p_ref, t2s_ref, t2d_ref, s1wl2_ref, s1wr2_ref):
    den = den_ref[...]
    mcnt = jnp.maximum(den[:, 4:5], 1.0)
    a = acc1a_ref[...]
    b = acc1b_ref[...]
    g1 = jnp.concatenate([
        a[:, :64] / (den[:, 0:1] + 1e-16),
        a[:, 64:] / (den[:, 1:2] + 1e-16),
        b[:, :64] / (den[:, 2:3] + 1e-16),
        b[:, 64:] / (den[:, 3:4] + 1e-16)], axis=1)
    g1 = g1 + bg1_ref[...]
    g1b = g1 * s1c_ref[...] + s1h_ref[...]
    g1e = jnp.where(g1b > 0, g1b, jnp.exp(g1b) - 1.0)
    h2 = jnp.dot(g1e, wg2_ref[...], preferred_element_type=_f32)
    h2p_ref[...] = h2
    t2s_ref[...] = jnp.dot(h2, as2_ref[...], preferred_element_type=_f32)
    t2d = jnp.dot(h2, ad2_ref[...], preferred_element_type=_f32)
    i = pl.program_id(0)
    rows = lax.broadcasted_iota(jnp.int32, (RB, 16), 0) + i * RB
    lanev = lax.broadcasted_iota(jnp.int32, (RB, 16), 1)
    t2d_ref[...] = jnp.where(
        rows < N, t2d, jnp.where(lanev < 1, -1e30, 0.0))
    accs = accsa_ref[...] + accsb_ref[...]
    s1 = accs / mcnt + bl1_ref[...] + xwr_ref[...]
    s1b = s1 * ssc_ref[...] + ssh_ref[...]
    s1r = jnp.maximum(s1b, 0.0)
    s1wl2_ref[...] = jnp.dot(s1r, wl2_ref[...], preferred_element_type=_f32)
    s1wr2_ref[...] = jnp.dot(s1r, wr2_ref[...], preferred_element_type=_f32)


def _tc2(den_acc, acc1, accs, xwr, bg1, bn1_scale, bn1_shift, wg2, as2_mat,
         ad2_mat, bl1, bns_scale, bns_shift, wl2, wr2):
    full = lambda shape: pl.BlockSpec(shape, lambda i: (0,) * len(shape))
    blk = lambda w: pl.BlockSpec((RB, w), lambda i: (i, 0))
    blk_hi = lambda w: pl.BlockSpec((RB, w), lambda i: (i + GRID, 0))
    return pl.pallas_call(
        _tc2_body,
        grid=(GRID,),
        in_specs=[
            blk(128), blk_hi(128), blk(16), blk(HID), blk_hi(HID), blk(HID),
            full((1, 256)), full((1, 256)), full((1, 256)),
            full((256, HID)), full((HID, 16)), full((HID, 16)),
            full((1, HID)), full((1, HID)), full((1, HID)),
            full((HID, HID)), full((HID, HID)),
        ],
        out_specs=[blk(HID), blk(16), blk(16), blk(HID), blk(HID)],
        out_shape=[
            jax.ShapeDtypeStruct((NP, HID), _f32),
            jax.ShapeDtypeStruct((NP, 16), _f32),
            jax.ShapeDtypeStruct((NP, 16), _f32),
            jax.ShapeDtypeStruct((NP, HID), _f32),
            jax.ShapeDtypeStruct((NP, HID), _f32),
        ],
    )(acc1, acc1, den_acc, accs, accs, xwr, bg1, bn1_scale, bn1_shift,
      wg2, as2_mat, ad2_mat, bl1, bns_scale, bns_shift, wl2, wr2)


def _sc2_body(src2_ref, dst2_ref, t2s_ref, t2d_ref, h2p_ref, swl_ref,
              den_o, acc2_o, accs2_o,
              idxs_blk, idxd_blk, ts0, ts1, td0, td1,
              er0, er1, ef0, ef1, hr0, hr1, sg0, sg1,
              den_sh, acc2_sh, accs2_sh,
              sts0, sts1, std0, std1, sh0, sh1, ss0, ss1):
    c = lax.axis_index("c")
    s = lax.axis_index("s")
    tsb = [ts0, ts1]
    tdb = [td0, td1]
    erb = [er0, er1]
    efb = [ef0, ef1]
    hrb = [hr0, hr1]
    sgb = [sg0, sg1]
    sts = [sts0, sts1]
    std = [std0, std1]
    sh = [sh0, sh1]
    ssb = [ss0, ss1]

    def zrow(j, _):
        z = jnp.zeros((16,), _f32)
        er0[j, :] = z
        for k in range(4):
            hr0[j, pl.ds(k * 16, 16)] = z
        return 0
    lax.fori_loop(0, CH, zrow, 0)

    def zstripe(k, _):
        base = s * STRIPE + k * CH
        pltpu.sync_copy(er0, den_sh.at[pl.ds(base, CH)])
        pltpu.sync_copy(hr0, acc2_sh.at[pl.ds(base, CH)])
        pltpu.sync_copy(hr0, accs2_sh.at[pl.ds(base, CH)])
        return 0
    lax.fori_loop(0, STRIPE // CH, zstripe, 0)
    plsc.subcore_barrier()

    lanev = lax.iota(jnp.int32, 16)
    zero16i = jnp.zeros((16,), jnp.int32)
    nct = NCHUNK // NWORK
    nblk = nct // K2
    wid = c * NTILE + s

    def issue(jj, si):
        dts = pltpu.async_copy(t2s_ref.at[idxs_blk.at[jj]], tsb[si], sts[si])
        dtd = pltpu.async_copy(t2d_ref.at[idxd_blk.at[jj]], tdb[si], std[si])
        dh = pltpu.async_copy(h2p_ref.at[idxs_blk.at[jj]], hrb[si], sh[si])
        dsg = pltpu.async_copy(swl_ref.at[idxs_blk.at[jj]], sgb[si], ssb[si])
        return dts, dtd, dh, dsg

    def blk_body(bi, _):
        row0 = wid * nct + bi * K2
        pltpu.sync_copy(src2_ref.at[pl.ds(row0, K2)], idxs_blk)
        pltpu.sync_copy(dst2_ref.at[pl.ds(row0, K2)], idxd_blk)
        d = [issue(0, 0), None]
        for j in range(K2):
            cur = j % 2
            nxt = 1 - cur
            if j + 1 < K2:
                d[nxt] = issue(j + 1, nxt)
            g = row0 + j
            dts, dtd, dh, dsg = d[cur]
            dts.wait()
            dtd.wait()
            ts_c, td_c, er_c, ef_c, hr_c = (
                tsb[cur], tdb[cur], erb[cur], efb[cur], hrb[cur])

            def ebody(jj, _):
                al = ts_c[jj, :] + td_c[jj, :]
                lr = jnp.where(al > 0, al, 0.2 * al)
                ev = jnp.exp(lr)
                out = jnp.where(lanev < 1, ev, 0.0)
                er_c[jj, :] = out
                ef_c[pl.ds(jj * 16, 16)] = out
                return 0
            lax.fori_loop(0, CH, ebody, 0)
            pltpu.sync_copy(er_c, den_sh.at[idxd_blk.at[j]], add=True)
            dh.wait()

            def sbody(jj, _):
                jv = zero16i + jj * 16
                w0 = plsc.load_gather(ef_c, [jv])
                for k in range(4):
                    hr_c[jj, pl.ds(k * 16, 16)] = (
                        hr_c[jj, pl.ds(k * 16, 16)] * w0)
                return 0
            lax.fori_loop(0, CH, sbody, 0)
            pltpu.sync_copy(hr_c, acc2_sh.at[idxd_blk.at[j]], add=True)
            dsg.wait()

            @pl.when(g < REAL)
            def _():
                pltpu.sync_copy(sgb[cur], accs2_sh.at[idxd_blk.at[j]], add=True)
        return 0
    lax.fori_loop(0, nblk, blk_body, 0)
    plsc.subcore_barrier()

    rbase = s * STRIPE
    obase = c * NP + rbase
    pltpu.sync_copy(den_sh.at[pl.ds(rbase, STRIPE)], den_o.at[pl.ds(obase, STRIPE)])
    pltpu.sync_copy(acc2_sh.at[pl.ds(rbase, STRIPE)], acc2_o.at[pl.ds(obase, STRIPE)])
    pltpu.sync_copy(accs2_sh.at[pl.ds(rbase, STRIPE)], accs2_o.at[pl.ds(obase, STRIPE)])


def _sc2(src2, dst2, t2s, t2d, h2p, s1wl2):
    return pl.kernel(
        _sc2_body,
        out_type=(
            jax.ShapeDtypeStruct((2 * NP, 16), _f32),
            jax.ShapeDtypeStruct((2 * NP, HID), _f32),
            jax.ShapeDtypeStruct((2 * NP, HID), _f32),
        ),
        mesh=_sc_mesh(),
        **_SC_PARAMS,
        scratch_types=[
            pltpu.VMEM((K2, CH), jnp.int32),
            pltpu.VMEM((K2, CH), jnp.int32),
            pltpu.VMEM((CH, 16), _f32),
            pltpu.VMEM((CH, 16), _f32),
            pltpu.VMEM((CH, 16), _f32),
            pltpu.VMEM((CH, 16), _f32),
            pltpu.VMEM((CH, 16), _f32),
            pltpu.VMEM((CH, 16), _f32),
            pltpu.VMEM((CH * 16,), _f32),
            pltpu.VMEM((CH * 16,), _f32),
            pltpu.VMEM((CH, HID), _f32),
            pltpu.VMEM((CH, HID), _f32),
            pltpu.VMEM((CH, HID), _f32),
            pltpu.VMEM((CH, HID), _f32),
            pltpu.VMEM_SHARED((NP, 16), _f32),
            pltpu.VMEM_SHARED((NP, HID), _f32),
            pltpu.VMEM_SHARED((NP, HID), _f32),
            pltpu.SemaphoreType.DMA,
            pltpu.SemaphoreType.DMA,
            pltpu.SemaphoreType.DMA,
            pltpu.SemaphoreType.DMA,
            pltpu.SemaphoreType.DMA,
            pltpu.SemaphoreType.DMA,
            pltpu.SemaphoreType.DMA,
            pltpu.SemaphoreType.DMA,
        ],
    )(src2, dst2, t2s, t2d, h2p, s1wl2)


def _tc3_body(acc2a_ref, acc2b_ref, den2a_ref, den2b_ref, accs2a_ref,
              accs2b_ref, s1wr2_ref, den_ref, bg2_ref, bl2_ref, wf1_ref,
              bf1_ref, wf2_ref, bf2_ref, out_ref):
    den2 = den2a_ref[...] + den2b_ref[...]
    g2 = (acc2a_ref[...] + acc2b_ref[...]) / (den2[:, 0:1] + 1e-16)
    g2 = g2 + bg2_ref[...]
    mcnt = jnp.maximum(den_ref[:, 4:5], 1.0)
    s2 = (accs2a_ref[...] + accs2b_ref[...]) / mcnt + bl2_ref[...] + s1wr2_ref[...]
    cc = jnp.concatenate([g2, s2], axis=1)
    h = jnp.maximum(jnp.dot(cc, wf1_ref[...], preferred_element_type=_f32)
                    + bf1_ref[...], 0.0)
    out_ref[...] = jnp.dot(h, wf2_ref[...], preferred_element_type=_f32) + bf2_ref[...]


def _tc3(acc2, den2, accs2, s1wr2, den_acc, bg2, bl2, wf1, bf1, wf2p, bf2p):
    full = lambda shape: pl.BlockSpec(shape, lambda i: (0,) * len(shape))
    blk = lambda w: pl.BlockSpec((RB, w), lambda i: (i, 0))
    blk_hi = lambda w: pl.BlockSpec((RB, w), lambda i: (i + GRID, 0))
    return pl.pallas_call(
        _tc3_body,
        grid=(GRID,),
        in_specs=[
            blk(HID), blk_hi(HID), blk(16), blk_hi(16), blk(HID), blk_hi(HID),
            blk(HID), blk(16),
            full((1, HID)), full((1, HID)), full((2 * HID, HID)),
            full((1, HID)), full((HID, 128)), full((1, 128)),
        ],
        out_specs=[pl.BlockSpec((RB, 128), lambda i: (i, 0))],
        out_shape=[jax.ShapeDtypeStruct((N, 128), _f32)],
    )(acc2, acc2, den2, den2, accs2, accs2, s1wr2, den_acc, bg2, bl2, wf1,
      bf1, wf2p, bf2p)


@jax.jit
def kernel(x, edge_index, W_gat1, att_src1, att_dst1, b_gat1, bn1_gamma,
           bn1_beta, bn1_mean, bn1_var, W_gat2, att_src2, att_dst2, b_gat2,
           Wl1, bl1, Wr1, bns_gamma, bns_beta, bns_mean, bns_var, Wl2, bl2,
           Wr2, Wf1, bf1, Wf2, bf2):
    src = edge_index[0].astype(jnp.int32)
    dst = edge_index[1].astype(jnp.int32)
    loops = jnp.arange(N, dtype=jnp.int32)
    padidx = (N + (jnp.arange(EP - E - N, dtype=jnp.int32) % (NP - N)))
    src2 = jnp.concatenate([src, loops, padidx]).reshape(NCHUNK, CH)
    dst2 = jnp.concatenate([dst, loops, padidx]).reshape(NCHUNK, CH)
    xp = jnp.pad(x, ((0, NP - N), (0, 0)))

    eye4 = jnp.eye(HEADS, dtype=_f32)
    as_mat = (att_src1[:, :, None] * eye4[:, None, :]).reshape(256, HEADS)
    as_mat = jnp.concatenate([as_mat, jnp.zeros((256, 12), _f32)], axis=1)
    ad_mat = (att_dst1[:, :, None] * eye4[:, None, :]).reshape(256, HEADS)
    ad_mat = jnp.concatenate([ad_mat, jnp.zeros((256, 12), _f32)], axis=1)
    as2_mat = jnp.concatenate([att_src2.T, jnp.zeros((HID, 15), _f32)], axis=1)
    ad2_mat = jnp.concatenate([att_dst2.T, jnp.zeros((HID, 15), _f32)], axis=1)

    bn1_scale = (bn1_gamma / jnp.sqrt(bn1_var + 1e-5)).reshape(1, 256)
    bn1_shift = (bn1_beta - bn1_mean * bn1_scale[0]).reshape(1, 256)
    bns_scale = (bns_gamma / jnp.sqrt(bns_var + 1e-5)).reshape(1, HID)
    bns_shift = (bns_beta - bns_mean * bns_scale[0]).reshape(1, HID)

    h1p3, t1s, t1d, xwl, xwr = _tc1(xp, W_gat1, as_mat, ad_mat, Wl1, Wr1)
    h1p = h1p3.reshape(2 * NP, 128)

    den_o, acc1_o = _sc1(src2, dst2, t1s, t1d, h1p)
    accs_o = _sc1b(src2, dst2, xwl)
    den_acc = den_o[:NP]

    h2p, t2s, t2d, s1wl2, s1wr2 = _tc2(
        den_acc, acc1_o, accs_o, xwr, b_gat1.reshape(1, 256), bn1_scale,
        bn1_shift, W_gat2, as2_mat, ad2_mat, bl1.reshape(1, HID), bns_scale,
        bns_shift, Wl2, Wr2)

    den2_o, acc2_o, accs2_o = _sc2(src2, dst2, t2s, t2d, h2p, s1wl2)

    wf2p = jnp.concatenate([Wf2, jnp.zeros((HID, 126), _f32)], axis=1)
    bf2p = jnp.concatenate([bf2, jnp.zeros((126,), _f32)]).reshape(1, 128)
    outp = _tc3(acc2_o, den2_o, accs2_o, s1wr2, den_acc,
                b_gat2.reshape(1, HID), bl2.reshape(1, HID), Wf1,
                bf1.reshape(1, HID), wf2p, bf2p)[0]
    return outp[:, :2]


# speedup vs baseline: 62.7865x; 1.2088x over previous
import jax
import jax.numpy as jnp
from jax import lax
from jax.experimental import pallas as pl
from jax.experimental.pallas import tpu as pltpu
from jax.experimental.pallas import tpu_sc as plsc

N = 10000
NP = 10240
E = 320000
F_IN = 128
HID = 64
HEADS = 4
EP = 331776
CH = 64
NCHUNK = EP // CH
REAL = E // CH
NSC = 2
NTILE = 16
NWORK = NSC * NTILE
STRIPE = NP // NTILE
K1 = 12
K2 = 9
RB = 1024
GRID = NP // RB

_f32 = jnp.float32
_SC_PARAMS = dict(
    compiler_params=pltpu.CompilerParams(
        needs_layout_passes=False, use_tc_tiling_on_sc=False),
)


def _sc_mesh():
    return plsc.VectorSubcoreMesh(core_axis_name="c", subcore_axis_name="s",
                                  num_cores=NSC, num_subcores=NTILE)


def _tc1_body(x_ref, wg1_ref, as_ref, ad_ref, wl1_ref, wr1_ref,
              h1p_ref, t1s_ref, t1d_ref, xwl_ref, xwr_ref):
    xb = x_ref[...]
    h1 = jnp.dot(xb, wg1_ref[...], preferred_element_type=_f32)
    h1p_ref[0] = h1[:, :128]
    h1p_ref[1] = h1[:, 128:]
    t1s_ref[...] = jnp.dot(h1, as_ref[...], preferred_element_type=_f32)
    t1d = jnp.dot(h1, ad_ref[...], preferred_element_type=_f32)
    i = pl.program_id(0)
    rows = lax.broadcasted_iota(jnp.int32, (RB, 16), 0) + i * RB
    lanev = lax.broadcasted_iota(jnp.int32, (RB, 16), 1)
    valid = rows < N
    t1d_ref[...] = jnp.where(
        valid, t1d + (lanev == 4).astype(_f32),
        jnp.where(lanev < 4, -1e30, 0.0))
    xwl_ref[...] = jnp.dot(xb, wl1_ref[...], preferred_element_type=_f32)
    xwr_ref[...] = jnp.dot(xb, wr1_ref[...], preferred_element_type=_f32)


def _tc1(xp, wg1, as_mat, ad_mat, wl1, wr1):
    full = lambda shape: pl.BlockSpec(shape, lambda i: (0,) * len(shape))
    return pl.pallas_call(
        _tc1_body,
        grid=(GRID,),
        in_specs=[
            pl.BlockSpec((RB, F_IN), lambda i: (i, 0)),
            full((F_IN, 256)), full((256, 16)), full((256, 16)),
            full((F_IN, HID)), full((F_IN, HID)),
        ],
        out_specs=[
            pl.BlockSpec((2, RB, 128), lambda i: (0, i, 0)),
            pl.BlockSpec((RB, 16), lambda i: (i, 0)),
            pl.BlockSpec((RB, 16), lambda i: (i, 0)),
            pl.BlockSpec((RB, HID), lambda i: (i, 0)),
            pl.BlockSpec((RB, HID), lambda i: (i, 0)),
        ],
        out_shape=[
            jax.ShapeDtypeStruct((2, NP, 128), _f32),
            jax.ShapeDtypeStruct((NP, 16), _f32),
            jax.ShapeDtypeStruct((NP, 16), _f32),
            jax.ShapeDtypeStruct((NP, HID), _f32),
            jax.ShapeDtypeStruct((NP, HID), _f32),
        ],
    )(xp, wg1, as_mat, ad_mat, wl1, wr1)


def _sc1_body(src2_ref, dst2_ref, t1s_ref, t1d_ref, h1p_ref,
              den_o, acc1_o,
              idxs_blk, idxd_blk, adj0, adj1, ts0, ts1, td0, td1,
              er0, er1, ef0, ef1, hr0, hr1,
              den_sh, acc1_sh,
              sts0, sts1, std0, std1, sh0, sh1, sd0, sd1, sa0, sa1):
    c = lax.axis_index("c")
    s = lax.axis_index("s")
    adjb = [adj0, adj1]
    tsb = [ts0, ts1]
    tdb = [td0, td1]
    erb = [er0, er1]
    efb = [ef0, ef1]
    hrb = [hr0, hr1]
    sts = [sts0, sts1]
    std = [std0, std1]
    sh = [sh0, sh1]
    sd = [sd0, sd1]
    sa = [sa0, sa1]

    def zrow(j, _):
        z = jnp.zeros((16,), _f32)
        er0[j, :] = z
        for k in range(8):
            hr0[j, pl.ds(k * 16, 16)] = z
        return 0
    lax.fori_loop(0, CH, zrow, 0)

    def zstripe(k, _):
        base = s * STRIPE + k * CH
        pltpu.sync_copy(er0, den_sh.at[pl.ds(base, CH)])
        pltpu.sync_copy(hr0, acc1_sh.at[pl.ds(base, CH)])
        return 0
    lax.fori_loop(0, STRIPE // CH, zstripe, 0)
    plsc.subcore_barrier()

    lanev = lax.iota(jnp.int32, 16)
    zero16i = jnp.zeros((16,), jnp.int32)
    idxh0v = zero16i + 2 * c
    idxh1v = idxh0v + 1
    coff = c * NP
    nct = NCHUNK // NTILE
    nblk = nct // K1

    def blk_body(bi, _):
        row0 = s * nct + bi * K1
        pltpu.sync_copy(src2_ref.at[pl.ds(row0, K1)], idxs_blk)
        pltpu.sync_copy(dst2_ref.at[pl.ds(row0, K1)], idxd_blk)
        scat = [None, None]

        def issue(jj, si):
            if scat[si] is not None:
                scat[si][0].wait()
                scat[si][1].wait()
                scat[si] = None
            for k in range(CH // 16):
                adjb[si][pl.ds(k * 16, 16)] = (
                    idxs_blk[jj, pl.ds(k * 16, 16)] + coff)
            dts = pltpu.async_copy(t1s_ref.at[idxs_blk.at[jj]], tsb[si], sts[si])
            dtd = pltpu.async_copy(t1d_ref.at[idxd_blk.at[jj]], tdb[si], std[si])
            dh = pltpu.async_copy(h1p_ref.at[adjb[si]], hrb[si], sh[si])
            return dts, dtd, dh

        d = [issue(0, 0), None]
        for j in range(K1):
            cur = j % 2
            nxt = 1 - cur
            if j + 1 < K1:
                d[nxt] = issue(j + 1, nxt)
            g = row0 + j
            realf = jnp.where(g < REAL, 1.0, 0.0).astype(_f32)
            dts, dtd, dh = d[cur]
            dts.wait()
            dtd.wait()
            dh.wait()
            ts_c, td_c, er_c, ef_c, hr_c = (
                tsb[cur], tdb[cur], erb[cur], efb[cur], hrb[cur])

            @plsc.parallel_loop(0, CH, unroll=2)
            def _(jj):
                al = ts_c[jj, :] + td_c[jj, :]
                lr = jnp.where(al > 0, al, 0.2 * al)
                ev = jnp.exp(lr)
                out = jnp.where(
                    lanev < 4, ev, jnp.where(lanev == 4, al * realf, 0.0))
                er_c[jj, :] = out
                ef_c[pl.ds(jj * 16, 16)] = out
                jv = zero16i + jj * 16
                w0 = plsc.load_gather(ef_c, [jv + idxh0v])
                w1 = plsc.load_gather(ef_c, [jv + idxh1v])
                for k in range(4):
                    hr_c[jj, pl.ds(k * 16, 16)] = (
                        hr_c[jj, pl.ds(k * 16, 16)] * w0)
                for k in range(4, 8):
                    hr_c[jj, pl.ds(k * 16, 16)] = (
                        hr_c[jj, pl.ds(k * 16, 16)] * w1)

            dden = pltpu.async_copy(
                er_c, den_sh.at[idxd_blk.at[j]], sd[cur], add=True)
            dacc = pltpu.async_copy(
                hr_c, acc1_sh.at[idxd_blk.at[j]], sa[cur], add=True)
            scat[cur] = (dden, dacc)
        for si in range(2):
            if scat[si] is not None:
                scat[si][0].wait()
                scat[si][1].wait()
        return 0
    lax.fori_loop(0, nblk, blk_body, 0)
    plsc.subcore_barrier()

    rbase = s * STRIPE
    obase = c * NP + rbase
    pltpu.sync_copy(den_sh.at[pl.ds(rbase, STRIPE)], den_o.at[pl.ds(obase, STRIPE)])
    pltpu.sync_copy(acc1_sh.at[pl.ds(rbase, STRIPE)], acc1_o.at[pl.ds(obase, STRIPE)])


def _sc1(src2, dst2, t1s, t1d, h1p):
    return pl.kernel(
        _sc1_body,
        out_type=(
            jax.ShapeDtypeStruct((2 * NP, 16), _f32),
            jax.ShapeDtypeStruct((2 * NP, 128), _f32),
        ),
        mesh=_sc_mesh(),
        **_SC_PARAMS,
        scratch_types=[
            pltpu.VMEM((K1, CH), jnp.int32),
            pltpu.VMEM((K1, CH), jnp.int32),
            pltpu.VMEM((CH,), jnp.int32),
            pltpu.VMEM((CH,), jnp.int32),
            pltpu.VMEM((CH, 16), _f32),
            pltpu.VMEM((CH, 16), _f32),
            pltpu.VMEM((CH, 16), _f32),
            pltpu.VMEM((CH, 16), _f32),
            pltpu.VMEM((CH, 16), _f32),
            pltpu.VMEM((CH, 16), _f32),
            pltpu.VMEM((CH * 16,), _f32),
            pltpu.VMEM((CH * 16,), _f32),
            pltpu.VMEM((CH, 128), _f32),
            pltpu.VMEM((CH, 128), _f32),
            pltpu.VMEM_SHARED((NP, 16), _f32),
            pltpu.VMEM_SHARED((NP, 128), _f32),
        ] + [pltpu.SemaphoreType.DMA] * 10,
    )(src2, dst2, t1s, t1d, h1p)


def _sc1b_body(src2_ref, dst2_ref, xwl_ref, accs_o,
               idxs_blk, idxd_blk, sg0, sg1, accs_sh, ss0, ss1):
    c = lax.axis_index("c")
    s = lax.axis_index("s")
    sgb = [sg0, sg1]
    ssb = [ss0, ss1]

    def zrow(j, _):
        z = jnp.zeros((16,), _f32)
        for k in range(4):
            sg0[j, pl.ds(k * 16, 16)] = z
        return 0
    lax.fori_loop(0, CH, zrow, 0)

    def zstripe(k, _):
        pltpu.sync_copy(sg0, accs_sh.at[pl.ds(s * STRIPE + k * CH, CH)])
        return 0
    lax.fori_loop(0, STRIPE // CH, zstripe, 0)
    plsc.subcore_barrier()

    nct = NCHUNK // NWORK
    nblk = nct // K2
    wid = c * NTILE + s

    def blk_body(bi, _):
        row0 = wid * nct + bi * K2
        pltpu.sync_copy(src2_ref.at[pl.ds(row0, K2)], idxs_blk)
        pltpu.sync_copy(dst2_ref.at[pl.ds(row0, K2)], idxd_blk)
        d = [pltpu.async_copy(xwl_ref.at[idxs_blk.at[0]], sg0, ss0), None]
        for j in range(K2):
            cur = j % 2
            nxt = 1 - cur
            if j + 1 < K2:
                d[nxt] = pltpu.async_copy(
                    xwl_ref.at[idxs_blk.at[j + 1]], sgb[nxt], ssb[nxt])
            g = row0 + j
            d[cur].wait()

            @pl.when(g < REAL)
            def _():
                pltpu.sync_copy(sgb[cur], accs_sh.at[idxd_blk.at[j]], add=True)
        return 0
    lax.fori_loop(0, nblk, blk_body, 0)
    plsc.subcore_barrier()

    rbase = s * STRIPE
    pltpu.sync_copy(accs_sh.at[pl.ds(rbase, STRIPE)],
                    accs_o.at[pl.ds(c * NP + rbase, STRIPE)])


def _sc1b(src2, dst2, xwl):
    return pl.kernel(
        _sc1b_body,
        out_type=jax.ShapeDtypeStruct((2 * NP, HID), _f32),
        mesh=_sc_mesh(),
        **_SC_PARAMS,
        scratch_types=[
            pltpu.VMEM((K2, CH), jnp.int32),
            pltpu.VMEM((K2, CH), jnp.int32),
            pltpu.VMEM((CH, HID), _f32),
            pltpu.VMEM((CH, HID), _f32),
            pltpu.VMEM_SHARED((NP, HID), _f32),
            pltpu.SemaphoreType.DMA,
            pltpu.SemaphoreType.DMA,
        ],
    )(src2, dst2, xwl)


def _tc2_body(acc1a_ref, acc1b_ref, den_ref, accsa_ref, accsb_ref, xwr_ref,
              bg1_ref, s1c_ref, s1h_ref, wg2_ref, as2_ref, ad2_ref,
              bl1_ref, ssc_ref, ssh_ref, wl2_ref, wr2_ref,
              h2p_ref, t2s_ref, t2d_ref, s1wl2_ref, s1wr2_ref):
    den = den_ref[...]
    mcnt = jnp.maximum(den[:, 4:5], 1.0)
    a = acc1a_ref[...]
    b = acc1b_ref[...]
    g1 = jnp.concatenate([
        a[:, :64] / (den[:, 0:1] + 1e-16),
        a[:, 64:] / (den[:, 1:2] + 1e-16),
        b[:, :64] / (den[:, 2:3] + 1e-16),
        b[:, 64:] / (den[:, 3:4] + 1e-16)], axis=1)
    g1 = g1 + bg1_ref[...]
    g1b = g1 * s1c_ref[...] + s1h_ref[...]
    g1e = jnp.where(g1b > 0, g1b, jnp.exp(g1b) - 1.0)
    h2 = jnp.dot(g1e, wg2_ref[...], preferred_element_type=_f32)
    h2p_ref[...] = h2
    t2s_ref[...] = jnp.dot(h2, as2_ref[...], preferred_element_type=_f32)
    t2d = jnp.dot(h2, ad2_ref[...], preferred_element_type=_f32)
    i = pl.program_id(0)
    rows = lax.broadcasted_iota(jnp.int32, (RB, 16), 0) + i * RB
    lanev = lax.broadcasted_iota(jnp.int32, (RB, 16), 1)
    t2d_ref[...] = jnp.where(
        rows < N, t2d, jnp.where(lanev < 1, -1e30, 0.0))
    accs = accsa_ref[...] + accsb_ref[...]
    s1 = accs / mcnt + bl1_ref[...] + xwr_ref[...]
    s1b = s1 * ssc_ref[...] + ssh_ref[...]
    s1r = jnp.maximum(s1b, 0.0)
    s1wl2_ref[...] = jnp.dot(s1r, wl2_ref[...], preferred_element_type=_f32)
    s1wr2_ref[...] = jnp.dot(s1r, wr2_ref[...], preferred_element_type=_f32)


def _tc2(den_acc, acc1, accs, xwr, bg1, bn1_scale, bn1_shift, wg2, as2_mat,
         ad2_mat, bl1, bns_scale, bns_shift, wl2, wr2):
    full = lambda shape: pl.BlockSpec(shape, lambda i: (0,) * len(shape))
    blk = lambda w: pl.BlockSpec((RB, w), lambda i: (i, 0))
    blk_hi = lambda w: pl.BlockSpec((RB, w), lambda i: (i + GRID, 0))
    return pl.pallas_call(
        _tc2_body,
        grid=(GRID,),
        in_specs=[
            blk(128), blk_hi(128), blk(16), blk(HID), blk_hi(HID), blk(HID),
            full((1, 256)), full((1, 256)), full((1, 256)),
            full((256, HID)), full((HID, 16)), full((HID, 16)),
            full((1, HID)), full((1, HID)), full((1, HID)),
            full((HID, HID)), full((HID, HID)),
        ],
        out_specs=[blk(HID), blk(16), blk(16), blk(HID), blk(HID)],
        out_shape=[
            jax.ShapeDtypeStruct((NP, HID), _f32),
            jax.ShapeDtypeStruct((NP, 16), _f32),
            jax.ShapeDtypeStruct((NP, 16), _f32),
            jax.ShapeDtypeStruct((NP, HID), _f32),
            jax.ShapeDtypeStruct((NP, HID), _f32),
        ],
    )(acc1, acc1, den_acc, accs, accs, xwr, bg1, bn1_scale, bn1_shift,
      wg2, as2_mat, ad2_mat, bl1, bns_scale, bns_shift, wl2, wr2)


def _sc2_body(src2_ref, dst2_ref, t2s_ref, t2d_ref, h2p_ref, swl_ref,
              den_o, acc2_o, accs2_o,
              idxs_blk, idxd_blk, ts0, ts1, td0, td1,
              er0, er1, ef0, ef1, hr0, hr1, sg0, sg1,
              den_sh, acc2_sh, accs2_sh,
              sts0, sts1, std0, std1, sh0, sh1, ss0, ss1,
              sd0, sd1, sa0, sa1):
    c = lax.axis_index("c")
    s = lax.axis_index("s")
    tsb = [ts0, ts1]
    tdb = [td0, td1]
    erb = [er0, er1]
    efb = [ef0, ef1]
    hrb = [hr0, hr1]
    sgb = [sg0, sg1]
    sts = [sts0, sts1]
    std = [std0, std1]
    sh = [sh0, sh1]
    ssb = [ss0, ss1]
    sd = [sd0, sd1]
    sa = [sa0, sa1]

    def zrow(j, _):
        z = jnp.zeros((16,), _f32)
        er0[j, :] = z
        for k in range(4):
            hr0[j, pl.ds(k * 16, 16)] = z
        return 0
    lax.fori_loop(0, CH, zrow, 0)

    def zstripe(k, _):
        base = s * STRIPE + k * CH
        pltpu.sync_copy(er0, den_sh.at[pl.ds(base, CH)])
        pltpu.sync_copy(hr0, acc2_sh.at[pl.ds(base, CH)])
        pltpu.sync_copy(hr0, accs2_sh.at[pl.ds(base, CH)])
        return 0
    lax.fori_loop(0, STRIPE // CH, zstripe, 0)
    plsc.subcore_barrier()

    lanev = lax.iota(jnp.int32, 16)
    zero16i = jnp.zeros((16,), jnp.int32)
    nct = NCHUNK // NWORK
    nblk = nct // K2
    wid = c * NTILE + s

    def blk_body(bi, _):
        row0 = wid * nct + bi * K2
        pltpu.sync_copy(src2_ref.at[pl.ds(row0, K2)], idxs_blk)
        pltpu.sync_copy(dst2_ref.at[pl.ds(row0, K2)], idxd_blk)
        scat = [None, None]

        def issue(jj, si):
            if scat[si] is not None:
                scat[si][0].wait()
                scat[si][1].wait()
                scat[si] = None
            dts = pltpu.async_copy(t2s_ref.at[idxs_blk.at[jj]], tsb[si], sts[si])
            dtd = pltpu.async_copy(t2d_ref.at[idxd_blk.at[jj]], tdb[si], std[si])
            dh = pltpu.async_copy(h2p_ref.at[idxs_blk.at[jj]], hrb[si], sh[si])
            dsg = pltpu.async_copy(swl_ref.at[idxs_blk.at[jj]], sgb[si], ssb[si])
            return dts, dtd, dh, dsg

        d = [issue(0, 0), None]
        for j in range(K2):
            cur = j % 2
            nxt = 1 - cur
            if j + 1 < K2:
                d[nxt] = issue(j + 1, nxt)
            g = row0 + j
            dts, dtd, dh, dsg = d[cur]
            dts.wait()
            dtd.wait()
            dh.wait()
            ts_c, td_c, er_c, ef_c, hr_c = (
                tsb[cur], tdb[cur], erb[cur], efb[cur], hrb[cur])

            @plsc.parallel_loop(0, CH, unroll=2)
            def _(jj):
                al = ts_c[jj, :] + td_c[jj, :]
                lr = jnp.where(al > 0, al, 0.2 * al)
                ev = jnp.exp(lr)
                out = jnp.where(lanev < 1, ev, 0.0)
                er_c[jj, :] = out
                ef_c[pl.ds(jj * 16, 16)] = out
                jv = zero16i + jj * 16
                w0 = plsc.load_gather(ef_c, [jv])
                for k in range(4):
                    hr_c[jj, pl.ds(k * 16, 16)] = (
                        hr_c[jj, pl.ds(k * 16, 16)] * w0)

            dden = pltpu.async_copy(
                er_c, den_sh.at[idxd_blk.at[j]], sd[cur], add=True)
            dacc = pltpu.async_copy(
                hr_c, acc2_sh.at[idxd_blk.at[j]], sa[cur], add=True)
            scat[cur] = (dden, dacc)
            dsg.wait()

            @pl.when(g < REAL)
            def _():
                pltpu.sync_copy(sgb[cur], accs2_sh.at[idxd_blk.at[j]], add=True)
        for si in range(2):
            if scat[si] is not None:
                scat[si][0].wait()
                scat[si][1].wait()
        return 0
    lax.fori_loop(0, nblk, blk_body, 0)
    plsc.subcore_barrier()

    rbase = s * STRIPE
    obase = c * NP + rbase
    pltpu.sync_copy(den_sh.at[pl.ds(rbase, STRIPE)], den_o.at[pl.ds(obase, STRIPE)])
    pltpu.sync_copy(acc2_sh.at[pl.ds(rbase, STRIPE)], acc2_o.at[pl.ds(obase, STRIPE)])
    pltpu.sync_copy(accs2_sh.at[pl.ds(rbase, STRIPE)], accs2_o.at[pl.ds(obase, STRIPE)])


def _sc2(src2, dst2, t2s, t2d, h2p, s1wl2):
    return pl.kernel(
        _sc2_body,
        out_type=(
            jax.ShapeDtypeStruct((2 * NP, 16), _f32),
            jax.ShapeDtypeStruct((2 * NP, HID), _f32),
            jax.ShapeDtypeStruct((2 * NP, HID), _f32),
        ),
        mesh=_sc_mesh(),
        **_SC_PARAMS,
        scratch_types=[
            pltpu.VMEM((K2, CH), jnp.int32),
            pltpu.VMEM((K2, CH), jnp.int32),
            pltpu.VMEM((CH, 16), _f32),
            pltpu.VMEM((CH, 16), _f32),
            pltpu.VMEM((CH, 16), _f32),
            pltpu.VMEM((CH, 16), _f32),
            pltpu.VMEM((CH, 16), _f32),
            pltpu.VMEM((CH, 16), _f32),
            pltpu.VMEM((CH * 16,), _f32),
            pltpu.VMEM((CH * 16,), _f32),
            pltpu.VMEM((CH, HID), _f32),
            pltpu.VMEM((CH, HID), _f32),
            pltpu.VMEM((CH, HID), _f32),
            pltpu.VMEM((CH, HID), _f32),
            pltpu.VMEM_SHARED((NP, 16), _f32),
            pltpu.VMEM_SHARED((NP, HID), _f32),
            pltpu.VMEM_SHARED((NP, HID), _f32),
        ] + [pltpu.SemaphoreType.DMA] * 12,
    )(src2, dst2, t2s, t2d, h2p, s1wl2)


def _tc3_body(acc2a_ref, acc2b_ref, den2a_ref, den2b_ref, accs2a_ref,
              accs2b_ref, s1wr2_ref, den_ref, bg2_ref, bl2_ref, wf1_ref,
              bf1_ref, wf2_ref, bf2_ref, out_ref):
    den2 = den2a_ref[...] + den2b_ref[...]
    g2 = (acc2a_ref[...] + acc2b_ref[...]) / (den2[:, 0:1] + 1e-16)
    g2 = g2 + bg2_ref[...]
    mcnt = jnp.maximum(den_ref[:, 4:5], 1.0)
    s2 = (accs2a_ref[...] + accs2b_ref[...]) / mcnt + bl2_ref[...] + s1wr2_ref[...]
    cc = jnp.concatenate([g2, s2], axis=1)
    h = jnp.maximum(jnp.dot(cc, wf1_ref[...], preferred_element_type=_f32)
                    + bf1_ref[...], 0.0)
    out_ref[...] = jnp.dot(h, wf2_ref[...], preferred_element_type=_f32) + bf2_ref[...]


def _tc3(acc2, den2, accs2, s1wr2, den_acc, bg2, bl2, wf1, bf1, wf2p, bf2p):
    full = lambda shape: pl.BlockSpec(shape, lambda i: (0,) * len(shape))
    blk = lambda w: pl.BlockSpec((RB, w), lambda i: (i, 0))
    blk_hi = lambda w: pl.BlockSpec((RB, w), lambda i: (i + GRID, 0))
    return pl.pallas_call(
        _tc3_body,
        grid=(GRID,),
        in_specs=[
            blk(HID), blk_hi(HID), blk(16), blk_hi(16), blk(HID), blk_hi(HID),
            blk(HID), blk(16),
            full((1, HID)), full((1, HID)), full((2 * HID, HID)),
            full((1, HID)), full((HID, 128)), full((1, 128)),
        ],
        out_specs=[pl.BlockSpec((RB, 128), lambda i: (i, 0))],
        out_shape=[jax.ShapeDtypeStruct((N, 128), _f32)],
    )(acc2, acc2, den2, den2, accs2, accs2, s1wr2, den_acc, bg2, bl2, wf1,
      bf1, wf2p, bf2p)


@jax.jit
def kernel(x, edge_index, W_gat1, att_src1, att_dst1, b_gat1, bn1_gamma,
           bn1_beta, bn1_mean, bn1_var, W_gat2, att_src2, att_dst2, b_gat2,
           Wl1, bl1, Wr1, bns_gamma, bns_beta, bns_mean, bns_var, Wl2, bl2,
           Wr2, Wf1, bf1, Wf2, bf2):
    src = edge_index[0].astype(jnp.int32)
    dst = edge_index[1].astype(jnp.int32)
    loops = jnp.arange(N, dtype=jnp.int32)
    padidx = (N + (jnp.arange(EP - E - N, dtype=jnp.int32) % (NP - N)))
    src2 = jnp.concatenate([src, loops, padidx]).reshape(NCHUNK, CH)
    dst2 = jnp.concatenate([dst, loops, padidx]).reshape(NCHUNK, CH)
    xp = jnp.pad(x, ((0, NP - N), (0, 0)))

    eye4 = jnp.eye(HEADS, dtype=_f32)
    as_mat = (att_src1[:, :, None] * eye4[:, None, :]).reshape(256, HEADS)
    as_mat = jnp.concatenate([as_mat, jnp.zeros((256, 12), _f32)], axis=1)
    ad_mat = (att_dst1[:, :, None] * eye4[:, None, :]).reshape(256, HEADS)
    ad_mat = jnp.concatenate([ad_mat, jnp.zeros((256, 12), _f32)], axis=1)
    as2_mat = jnp.concatenate([att_src2.T, jnp.zeros((HID, 15), _f32)], axis=1)
    ad2_mat = jnp.concatenate([att_dst2.T, jnp.zeros((HID, 15), _f32)], axis=1)

    bn1_scale = (bn1_gamma / jnp.sqrt(bn1_var + 1e-5)).reshape(1, 256)
    bn1_shift = (bn1_beta - bn1_mean * bn1_scale[0]).reshape(1, 256)
    bns_scale = (bns_gamma / jnp.sqrt(bns_var + 1e-5)).reshape(1, HID)
    bns_shift = (bns_beta - bns_mean * bns_scale[0]).reshape(1, HID)

    h1p3, t1s, t1d, xwl, xwr = _tc1(xp, W_gat1, as_mat, ad_mat, Wl1, Wr1)
    h1p = h1p3.reshape(2 * NP, 128)

    den_o, acc1_o = _sc1(src2, dst2, t1s, t1d, h1p)
    accs_o = _sc1b(src2, dst2, xwl)
    den_acc = den_o[:NP]

    h2p, t2s, t2d, s1wl2, s1wr2 = _tc2(
        den_acc, acc1_o, accs_o, xwr, b_gat1.reshape(1, 256), bn1_scale,
        bn1_shift, W_gat2, as2_mat, ad2_mat, bl1.reshape(1, HID), bns_scale,
        bns_shift, Wl2, Wr2)

    den2_o, acc2_o, accs2_o = _sc2(src2, dst2, t2s, t2d, h2p, s1wl2)

    wf2p = jnp.concatenate([Wf2, jnp.zeros((HID, 126), _f32)], axis=1)
    bf2p = jnp.concatenate([bf2, jnp.zeros((126,), _f32)]).reshape(1, 128)
    outp = _tc3(acc2_o, den2_o, accs2_o, s1wr2, den_acc,
                b_gat2.reshape(1, HID), bl2.reshape(1, HID), Wf1,
                bf1.reshape(1, HID), wf2p, bf2p)[0]
    return outp[:, :2]


# speedup vs baseline: 64.5124x; 1.0275x over previous
import jax
import jax.numpy as jnp
from jax import lax
from jax.experimental import pallas as pl
from jax.experimental.pallas import tpu as pltpu
from jax.experimental.pallas import tpu_sc as plsc

N = 10000
NP = 10240
E = 320000
F_IN = 128
HID = 64
HEADS = 4
EP = 331776
CH = 64
NCHUNK = EP // CH
REAL = E // CH
NSC = 2
NTILE = 16
NWORK = NSC * NTILE
STRIPE = NP // NTILE
K1 = 12
K2 = 9
RB = 1024
GRID = NP // RB

_f32 = jnp.float32
_SC_PARAMS = dict(
    compiler_params=pltpu.CompilerParams(
        needs_layout_passes=False, use_tc_tiling_on_sc=False),
)


def _sc_mesh():
    return plsc.VectorSubcoreMesh(core_axis_name="c", subcore_axis_name="s",
                                  num_cores=NSC, num_subcores=NTILE)


def _tc1_body(x_ref, wg1_ref, as_ref, ad_ref, wl1_ref, wr1_ref,
              h1p_ref, t1s_ref, t1d_ref, xwl_ref, xwr_ref):
    xb = x_ref[...]
    h1 = jnp.dot(xb, wg1_ref[...], preferred_element_type=_f32)
    h1p_ref[0] = h1[:, :128]
    h1p_ref[1] = h1[:, 128:]
    t1s_ref[...] = jnp.dot(h1, as_ref[...], preferred_element_type=_f32)
    t1d = jnp.dot(h1, ad_ref[...], preferred_element_type=_f32)
    i = pl.program_id(0)
    rows = lax.broadcasted_iota(jnp.int32, (RB, 16), 0) + i * RB
    lanev = lax.broadcasted_iota(jnp.int32, (RB, 16), 1)
    valid = rows < N
    t1d_ref[...] = jnp.where(
        valid, t1d + (lanev == 4).astype(_f32),
        jnp.where(lanev < 4, -1e30, 0.0))
    xwl_ref[...] = jnp.dot(xb, wl1_ref[...], preferred_element_type=_f32)
    xwr_ref[...] = jnp.dot(xb, wr1_ref[...], preferred_element_type=_f32)


def _tc1(xp, wg1, as_mat, ad_mat, wl1, wr1):
    full = lambda shape: pl.BlockSpec(shape, lambda i: (0,) * len(shape))
    return pl.pallas_call(
        _tc1_body,
        grid=(GRID,),
        in_specs=[
            pl.BlockSpec((RB, F_IN), lambda i: (i, 0)),
            full((F_IN, 256)), full((256, 16)), full((256, 16)),
            full((F_IN, HID)), full((F_IN, HID)),
        ],
        out_specs=[
            pl.BlockSpec((2, RB, 128), lambda i: (0, i, 0)),
            pl.BlockSpec((RB, 16), lambda i: (i, 0)),
            pl.BlockSpec((RB, 16), lambda i: (i, 0)),
            pl.BlockSpec((RB, HID), lambda i: (i, 0)),
            pl.BlockSpec((RB, HID), lambda i: (i, 0)),
        ],
        out_shape=[
            jax.ShapeDtypeStruct((2, NP, 128), _f32),
            jax.ShapeDtypeStruct((NP, 16), _f32),
            jax.ShapeDtypeStruct((NP, 16), _f32),
            jax.ShapeDtypeStruct((NP, HID), _f32),
            jax.ShapeDtypeStruct((NP, HID), _f32),
        ],
    )(xp, wg1, as_mat, ad_mat, wl1, wr1)


def _sc1_body(src2_ref, dst2_ref, t1s_ref, t1d_ref, h1p_ref,
              den_o, acc1_o,
              idxs_blk, idxd_blk, adj0, adj1, ts0, ts1, td0, td1,
              er0, er1, ef0, ef1, hr0, hr1,
              den_sh, acc1_sh,
              sts0, sts1, std0, std1, sh0, sh1, sd0, sd1, sa0, sa1):
    c = lax.axis_index("c")
    s = lax.axis_index("s")
    adjb = [adj0, adj1]
    tsb = [ts0, ts1]
    tdb = [td0, td1]
    erb = [er0, er1]
    efb = [ef0, ef1]
    hrb = [hr0, hr1]
    sts = [sts0, sts1]
    std = [std0, std1]
    sh = [sh0, sh1]
    sd = [sd0, sd1]
    sa = [sa0, sa1]

    def zrow(j, _):
        z = jnp.zeros((16,), _f32)
        er0[j, :] = z
        for k in range(8):
            hr0[j, pl.ds(k * 16, 16)] = z
        return 0
    lax.fori_loop(0, CH, zrow, 0)

    def zstripe(k, _):
        base = s * STRIPE + k * CH
        pltpu.sync_copy(er0, den_sh.at[pl.ds(base, CH)])
        pltpu.sync_copy(hr0, acc1_sh.at[pl.ds(base, CH)])
        return 0
    lax.fori_loop(0, STRIPE // CH, zstripe, 0)
    plsc.subcore_barrier()

    lanev = lax.iota(jnp.int32, 16)
    zero16i = jnp.zeros((16,), jnp.int32)
    idxh0v = zero16i + 2 * c
    idxh1v = idxh0v + 1
    coff = c * NP
    nct = NCHUNK // NTILE
    nblk = nct // K1

    def blk_body(bi, _):
        row0 = s * nct + bi * K1
        pltpu.sync_copy(src2_ref.at[pl.ds(row0, K1)], idxs_blk)
        pltpu.sync_copy(dst2_ref.at[pl.ds(row0, K1)], idxd_blk)
        scat = [None, None]

        def issue(jj, si):
            if scat[si] is not None:
                scat[si][0].wait()
                scat[si][1].wait()
                scat[si] = None
            for k in range(CH // 16):
                adjb[si][pl.ds(k * 16, 16)] = (
                    idxs_blk[jj, pl.ds(k * 16, 16)] + coff)
            dts = pltpu.async_copy(t1s_ref.at[idxs_blk.at[jj]], tsb[si], sts[si])
            dtd = pltpu.async_copy(t1d_ref.at[idxd_blk.at[jj]], tdb[si], std[si])
            dh = pltpu.async_copy(h1p_ref.at[adjb[si]], hrb[si], sh[si])
            return dts, dtd, dh

        d = [issue(0, 0), None]
        for j in range(K1):
            cur = j % 2
            nxt = 1 - cur
            if j + 1 < K1:
                d[nxt] = issue(j + 1, nxt)
            g = row0 + j
            realf = jnp.where(g < REAL, 1.0, 0.0).astype(_f32)
            dts, dtd, dh = d[cur]
            dts.wait()
            dtd.wait()
            dh.wait()
            ts_c, td_c, er_c, ef_c, hr_c = (
                tsb[cur], tdb[cur], erb[cur], efb[cur], hrb[cur])

            @plsc.parallel_loop(0, CH, unroll=4)
            def _(jj):
                al = ts_c[jj, :] + td_c[jj, :]
                lr = jnp.where(al > 0, al, 0.2 * al)
                ev = jnp.exp(lr)
                out = jnp.where(
                    lanev < 4, ev, jnp.where(lanev == 4, al * realf, 0.0))
                er_c[jj, :] = out
                ef_c[pl.ds(jj * 16, 16)] = out
                jv = zero16i + jj * 16
                w0 = plsc.load_gather(ef_c, [jv + idxh0v])
                w1 = plsc.load_gather(ef_c, [jv + idxh1v])
                for k in range(4):
                    hr_c[jj, pl.ds(k * 16, 16)] = (
                        hr_c[jj, pl.ds(k * 16, 16)] * w0)
                for k in range(4, 8):
                    hr_c[jj, pl.ds(k * 16, 16)] = (
                        hr_c[jj, pl.ds(k * 16, 16)] * w1)

            dden = pltpu.async_copy(
                er_c, den_sh.at[idxd_blk.at[j]], sd[cur], add=True)
            dacc = pltpu.async_copy(
                hr_c, acc1_sh.at[idxd_blk.at[j]], sa[cur], add=True)
            scat[cur] = (dden, dacc)
        for si in range(2):
            if scat[si] is not None:
                scat[si][0].wait()
                scat[si][1].wait()
        return 0
    lax.fori_loop(0, nblk, blk_body, 0)
    plsc.subcore_barrier()

    rbase = s * STRIPE
    obase = c * NP + rbase
    pltpu.sync_copy(den_sh.at[pl.ds(rbase, STRIPE)], den_o.at[pl.ds(obase, STRIPE)])
    pltpu.sync_copy(acc1_sh.at[pl.ds(rbase, STRIPE)], acc1_o.at[pl.ds(obase, STRIPE)])


def _sc1(src2, dst2, t1s, t1d, h1p):
    return pl.kernel(
        _sc1_body,
        out_type=(
            jax.ShapeDtypeStruct((2 * NP, 16), _f32),
            jax.ShapeDtypeStruct((2 * NP, 128), _f32),
        ),
        mesh=_sc_mesh(),
        **_SC_PARAMS,
        scratch_types=[
            pltpu.VMEM((K1, CH), jnp.int32),
            pltpu.VMEM((K1, CH), jnp.int32),
            pltpu.VMEM((CH,), jnp.int32),
            pltpu.VMEM((CH,), jnp.int32),
            pltpu.VMEM((CH, 16), _f32),
            pltpu.VMEM((CH, 16), _f32),
            pltpu.VMEM((CH, 16), _f32),
            pltpu.VMEM((CH, 16), _f32),
            pltpu.VMEM((CH, 16), _f32),
            pltpu.VMEM((CH, 16), _f32),
            pltpu.VMEM((CH * 16,), _f32),
            pltpu.VMEM((CH * 16,), _f32),
            pltpu.VMEM((CH, 128), _f32),
            pltpu.VMEM((CH, 128), _f32),
            pltpu.VMEM_SHARED((NP, 16), _f32),
            pltpu.VMEM_SHARED((NP, 128), _f32),
        ] + [pltpu.SemaphoreType.DMA] * 10,
    )(src2, dst2, t1s, t1d, h1p)


def _sc1b_body(src2_ref, dst2_ref, xwl_ref, accs_o,
               idxs_blk, idxd_blk, sg0, sg1, sg2, sg3, accs_sh,
               ss0, ss1, ss2, ss3, sa0, sa1, sa2, sa3):
    c = lax.axis_index("c")
    s = lax.axis_index("s")
    sgb = [sg0, sg1, sg2, sg3]
    ssb = [ss0, ss1, ss2, ss3]
    sab = [sa0, sa1, sa2, sa3]
    NB = 4

    def zrow(j, _):
        z = jnp.zeros((16,), _f32)
        for k in range(4):
            sg0[j, pl.ds(k * 16, 16)] = z
        return 0
    lax.fori_loop(0, CH, zrow, 0)

    def zstripe(k, _):
        pltpu.sync_copy(sg0, accs_sh.at[pl.ds(s * STRIPE + k * CH, CH)])
        return 0
    lax.fori_loop(0, STRIPE // CH, zstripe, 0)
    plsc.subcore_barrier()

    lanev = lax.iota(jnp.int32, 16)
    nct = NCHUNK // NWORK
    nblk = nct // K2
    wid = c * NTILE + s

    def blk_body(bi, _):
        row0 = wid * nct + bi * K2
        pltpu.sync_copy(src2_ref.at[pl.ds(row0, K2)], idxs_blk)
        pltpu.sync_copy(dst2_ref.at[pl.ds(row0, K2)], idxd_blk)
        scat = [None] * NB

        def issue(jj):
            b = jj % NB
            if scat[b] is not None:
                scat[b].wait()
                scat[b] = None
            return pltpu.async_copy(xwl_ref.at[idxs_blk.at[jj]], sgb[b], ssb[b])

        d = {}
        for jj in range(min(NB - 1, K2)):
            d[jj] = issue(jj)
        for j in range(K2):
            b = j % NB
            if j + NB - 1 < K2:
                d[j + NB - 1] = issue(j + NB - 1)
            g = row0 + j
            d[j].wait()

            @pl.when(g >= REAL)
            def _():
                for k in range(CH // 16):
                    idxd_blk[j, pl.ds(k * 16, 16)] = (N + k * 16) + lanev
            scat[b] = pltpu.async_copy(
                sgb[b], accs_sh.at[idxd_blk.at[j]], sab[b], add=True)
        for b in range(NB):
            if scat[b] is not None:
                scat[b].wait()
        return 0
    lax.fori_loop(0, nblk, blk_body, 0)
    plsc.subcore_barrier()

    rbase = s * STRIPE
    pltpu.sync_copy(accs_sh.at[pl.ds(rbase, STRIPE)],
                    accs_o.at[pl.ds(c * NP + rbase, STRIPE)])


def _sc1b(src2, dst2, xwl):
    return pl.kernel(
        _sc1b_body,
        out_type=jax.ShapeDtypeStruct((2 * NP, HID), _f32),
        mesh=_sc_mesh(),
        **_SC_PARAMS,
        scratch_types=[
            pltpu.VMEM((K2, CH), jnp.int32),
            pltpu.VMEM((K2, CH), jnp.int32),
            pltpu.VMEM((CH, HID), _f32),
            pltpu.VMEM((CH, HID), _f32),
            pltpu.VMEM((CH, HID), _f32),
            pltpu.VMEM((CH, HID), _f32),
            pltpu.VMEM_SHARED((NP, HID), _f32),
        ] + [pltpu.SemaphoreType.DMA] * 8,
    )(src2, dst2, xwl)


def _tc2_body(acc1a_ref, acc1b_ref, den_ref, accsa_ref, accsb_ref, xwr_ref,
              bg1_ref, s1c_ref, s1h_ref, wg2_ref, as2_ref, ad2_ref,
              bl1_ref, ssc_ref, ssh_ref, wl2_ref, wr2_ref,
              h2p_ref, t2s_ref, t2d_ref, s1wl2_ref, s1wr2_ref):
    den = den_ref[...]
    mcnt = jnp.maximum(den[:, 4:5], 1.0)
    a = acc1a_ref[...]
    b = acc1b_ref[...]
    g1 = jnp.concatenate([
        a[:, :64] / (den[:, 0:1] + 1e-16),
        a[:, 64:] / (den[:, 1:2] + 1e-16),
        b[:, :64] / (den[:, 2:3] + 1e-16),
        b[:, 64:] / (den[:, 3:4] + 1e-16)], axis=1)
    g1 = g1 + bg1_ref[...]
    g1b = g1 * s1c_ref[...] + s1h_ref[...]
    g1e = jnp.where(g1b > 0, g1b, jnp.exp(g1b) - 1.0)
    h2 = jnp.dot(g1e, wg2_ref[...], preferred_element_type=_f32)
    h2p_ref[...] = h2
    t2s_ref[...] = jnp.dot(h2, as2_ref[...], preferred_element_type=_f32)
    t2d = jnp.dot(h2, ad2_ref[...], preferred_element_type=_f32)
    i = pl.program_id(0)
    rows = lax.broadcasted_iota(jnp.int32, (RB, 16), 0) + i * RB
    lanev = lax.broadcasted_iota(jnp.int32, (RB, 16), 1)
    t2d_ref[...] = jnp.where(
        rows < N, t2d, jnp.where(lanev < 1, -1e30, 0.0))
    accs = accsa_ref[...] + accsb_ref[...]
    s1 = accs / mcnt + bl1_ref[...] + xwr_ref[...]
    s1b = s1 * ssc_ref[...] + ssh_ref[...]
    s1r = jnp.maximum(s1b, 0.0)
    s1wl2_ref[...] = jnp.dot(s1r, wl2_ref[...], preferred_element_type=_f32)
    s1wr2_ref[...] = jnp.dot(s1r, wr2_ref[...], preferred_element_type=_f32)


def _tc2(den_acc, acc1, accs, xwr, bg1, bn1_scale, bn1_shift, wg2, as2_mat,
         ad2_mat, bl1, bns_scale, bns_shift, wl2, wr2):
    full = lambda shape: pl.BlockSpec(shape, lambda i: (0,) * len(shape))
    blk = lambda w: pl.BlockSpec((RB, w), lambda i: (i, 0))
    blk_hi = lambda w: pl.BlockSpec((RB, w), lambda i: (i + GRID, 0))
    return pl.pallas_call(
        _tc2_body,
        grid=(GRID,),
        in_specs=[
            blk(128), blk_hi(128), blk(16), blk(HID), blk_hi(HID), blk(HID),
            full((1, 256)), full((1, 256)), full((1, 256)),
            full((256, HID)), full((HID, 16)), full((HID, 16)),
            full((1, HID)), full((1, HID)), full((1, HID)),
            full((HID, HID)), full((HID, HID)),
        ],
        out_specs=[blk(HID), blk(16), blk(16), blk(HID), blk(HID)],
        out_shape=[
            jax.ShapeDtypeStruct((NP, HID), _f32),
            jax.ShapeDtypeStruct((NP, 16), _f32),
            jax.ShapeDtypeStruct((NP, 16), _f32),
            jax.ShapeDtypeStruct((NP, HID), _f32),
            jax.ShapeDtypeStruct((NP, HID), _f32),
        ],
    )(acc1, acc1, den_acc, accs, accs, xwr, bg1, bn1_scale, bn1_shift,
      wg2, as2_mat, ad2_mat, bl1, bns_scale, bns_shift, wl2, wr2)


def _sc2_body(src2_ref, dst2_ref, t2s_ref, t2d_ref, h2p_ref, swl_ref,
              den_o, acc2_o, accs2_o,
              idxs_blk, idxd_blk, ixg0, ixg1, ts0, ts1, td0, td1,
              er0, er1, ef0, ef1, hr0, hr1, sg0, sg1,
              den_sh, acc2_sh, accs2_sh,
              sts0, sts1, std0, std1, sh0, sh1, ss0, ss1,
              sd0, sd1, sa0, sa1, sb0, sb1):
    c = lax.axis_index("c")
    s = lax.axis_index("s")
    tsb = [ts0, ts1]
    tdb = [td0, td1]
    erb = [er0, er1]
    efb = [ef0, ef1]
    hrb = [hr0, hr1]
    sgb = [sg0, sg1]
    sts = [sts0, sts1]
    std = [std0, std1]
    sh = [sh0, sh1]
    ssb = [ss0, ss1]
    sd = [sd0, sd1]
    sa = [sa0, sa1]
    sbb = [sb0, sb1]
    ixg = [ixg0, ixg1]

    def zrow(j, _):
        z = jnp.zeros((16,), _f32)
        er0[j, :] = z
        for k in range(4):
            hr0[j, pl.ds(k * 16, 16)] = z
        return 0
    lax.fori_loop(0, CH, zrow, 0)

    def zstripe(k, _):
        base = s * STRIPE + k * CH
        pltpu.sync_copy(er0, den_sh.at[pl.ds(base, CH)])
        pltpu.sync_copy(hr0, acc2_sh.at[pl.ds(base, CH)])
        pltpu.sync_copy(hr0, accs2_sh.at[pl.ds(base, CH)])
        return 0
    lax.fori_loop(0, STRIPE // CH, zstripe, 0)
    plsc.subcore_barrier()

    lanev = lax.iota(jnp.int32, 16)
    zero16i = jnp.zeros((16,), jnp.int32)
    nct = NCHUNK // NWORK
    nblk = nct // K2
    wid = c * NTILE + s

    def blk_body(bi, _):
        row0 = wid * nct + bi * K2
        pltpu.sync_copy(src2_ref.at[pl.ds(row0, K2)], idxs_blk)
        pltpu.sync_copy(dst2_ref.at[pl.ds(row0, K2)], idxd_blk)
        scat = [None, None]

        def issue(jj, si):
            if scat[si] is not None:
                for dd in scat[si]:
                    dd.wait()
                scat[si] = None
            dts = pltpu.async_copy(t2s_ref.at[idxs_blk.at[jj]], tsb[si], sts[si])
            dtd = pltpu.async_copy(t2d_ref.at[idxd_blk.at[jj]], tdb[si], std[si])
            dh = pltpu.async_copy(h2p_ref.at[idxs_blk.at[jj]], hrb[si], sh[si])
            dsg = pltpu.async_copy(swl_ref.at[idxs_blk.at[jj]], sgb[si], ssb[si])
            return dts, dtd, dh, dsg

        d = [issue(0, 0), None]
        for j in range(K2):
            cur = j % 2
            nxt = 1 - cur
            if j + 1 < K2:
                d[nxt] = issue(j + 1, nxt)
            g = row0 + j
            dts, dtd, dh, dsg = d[cur]
            dts.wait()
            dtd.wait()
            dh.wait()
            ts_c, td_c, er_c, ef_c, hr_c = (
                tsb[cur], tdb[cur], erb[cur], efb[cur], hrb[cur])

            @plsc.parallel_loop(0, CH, unroll=4)
            def _(jj):
                al = ts_c[jj, :] + td_c[jj, :]
                lr = jnp.where(al > 0, al, 0.2 * al)
                ev = jnp.exp(lr)
                out = jnp.where(lanev < 1, ev, 0.0)
                er_c[jj, :] = out
                ef_c[pl.ds(jj * 16, 16)] = out
                jv = zero16i + jj * 16
                w0 = plsc.load_gather(ef_c, [jv])
                for k in range(4):
                    hr_c[jj, pl.ds(k * 16, 16)] = (
                        hr_c[jj, pl.ds(k * 16, 16)] * w0)

            dden = pltpu.async_copy(
                er_c, den_sh.at[idxd_blk.at[j]], sd[cur], add=True)
            dacc = pltpu.async_copy(
                hr_c, acc2_sh.at[idxd_blk.at[j]], sa[cur], add=True)
            dsg.wait()
            for k in range(CH // 16):
                ixg[cur][pl.ds(k * 16, 16)] = idxd_blk[j, pl.ds(k * 16, 16)]

            @pl.when(g >= REAL)
            def _():
                for k in range(CH // 16):
                    ixg[cur][pl.ds(k * 16, 16)] = (N + k * 16) + lanev
            dsage = pltpu.async_copy(
                sgb[cur], accs2_sh.at[ixg[cur]], sbb[cur], add=True)
            scat[cur] = (dden, dacc, dsage)
        for si in range(2):
            if scat[si] is not None:
                for dd in scat[si]:
                    dd.wait()
        return 0
    lax.fori_loop(0, nblk, blk_body, 0)
    plsc.subcore_barrier()

    rbase = s * STRIPE
    obase = c * NP + rbase
    pltpu.sync_copy(den_sh.at[pl.ds(rbase, STRIPE)], den_o.at[pl.ds(obase, STRIPE)])
    pltpu.sync_copy(acc2_sh.at[pl.ds(rbase, STRIPE)], acc2_o.at[pl.ds(obase, STRIPE)])
    pltpu.sync_copy(accs2_sh.at[pl.ds(rbase, STRIPE)], accs2_o.at[pl.ds(obase, STRIPE)])


def _sc2(src2, dst2, t2s, t2d, h2p, s1wl2):
    return pl.kernel(
        _sc2_body,
        out_type=(
            jax.ShapeDtypeStruct((2 * NP, 16), _f32),
            jax.ShapeDtypeStruct((2 * NP, HID), _f32),
            jax.ShapeDtypeStruct((2 * NP, HID), _f32),
        ),
        mesh=_sc_mesh(),
        **_SC_PARAMS,
        scratch_types=[
            pltpu.VMEM((K2, CH), jnp.int32),
            pltpu.VMEM((K2, CH), jnp.int32),
            pltpu.VMEM((CH,), jnp.int32),
            pltpu.VMEM((CH,), jnp.int32),
            pltpu.VMEM((CH, 16), _f32),
            pltpu.VMEM((CH, 16), _f32),
            pltpu.VMEM((CH, 16), _f32),
            pltpu.VMEM((CH, 16), _f32),
            pltpu.VMEM((CH, 16), _f32),
            pltpu.VMEM((CH, 16), _f32),
            pltpu.VMEM((CH * 16,), _f32),
            pltpu.VMEM((CH * 16,), _f32),
            pltpu.VMEM((CH, HID), _f32),
            pltpu.VMEM((CH, HID), _f32),
            pltpu.VMEM((CH, HID), _f32),
            pltpu.VMEM((CH, HID), _f32),
            pltpu.VMEM_SHARED((NP, 16), _f32),
            pltpu.VMEM_SHARED((NP, HID), _f32),
            pltpu.VMEM_SHARED((NP, HID), _f32),
        ] + [pltpu.SemaphoreType.DMA] * 14,
    )(src2, dst2, t2s, t2d, h2p, s1wl2)


def _tc3_body(acc2a_ref, acc2b_ref, den2a_ref, den2b_ref, accs2a_ref,
              accs2b_ref, s1wr2_ref, den_ref, bg2_ref, bl2_ref, wf1_ref,
              bf1_ref, wf2_ref, bf2_ref, out_ref):
    den2 = den2a_ref[...] + den2b_ref[...]
    g2 = (acc2a_ref[...] + acc2b_ref[...]) / (den2[:, 0:1] + 1e-16)
    g2 = g2 + bg2_ref[...]
    mcnt = jnp.maximum(den_ref[:, 4:5], 1.0)
    s2 = (accs2a_ref[...] + accs2b_ref[...]) / mcnt + bl2_ref[...] + s1wr2_ref[...]
    cc = jnp.concatenate([g2, s2], axis=1)
    h = jnp.maximum(jnp.dot(cc, wf1_ref[...], preferred_element_type=_f32)
                    + bf1_ref[...], 0.0)
    out_ref[...] = jnp.dot(h, wf2_ref[...], preferred_element_type=_f32) + bf2_ref[...]


def _tc3(acc2, den2, accs2, s1wr2, den_acc, bg2, bl2, wf1, bf1, wf2p, bf2p):
    full = lambda shape: pl.BlockSpec(shape, lambda i: (0,) * len(shape))
    blk = lambda w: pl.BlockSpec((RB, w), lambda i: (i, 0))
    blk_hi = lambda w: pl.BlockSpec((RB, w), lambda i: (i + GRID, 0))
    return pl.pallas_call(
        _tc3_body,
        grid=(GRID,),
        in_specs=[
            blk(HID), blk_hi(HID), blk(16), blk_hi(16), blk(HID), blk_hi(HID),
            blk(HID), blk(16),
            full((1, HID)), full((1, HID)), full((2 * HID, HID)),
            full((1, HID)), full((HID, 128)), full((1, 128)),
        ],
        out_specs=[pl.BlockSpec((RB, 128), lambda i: (i, 0))],
        out_shape=[jax.ShapeDtypeStruct((N, 128), _f32)],
    )(acc2, acc2, den2, den2, accs2, accs2, s1wr2, den_acc, bg2, bl2, wf1,
      bf1, wf2p, bf2p)


@jax.jit
def kernel(x, edge_index, W_gat1, att_src1, att_dst1, b_gat1, bn1_gamma,
           bn1_beta, bn1_mean, bn1_var, W_gat2, att_src2, att_dst2, b_gat2,
           Wl1, bl1, Wr1, bns_gamma, bns_beta, bns_mean, bns_var, Wl2, bl2,
           Wr2, Wf1, bf1, Wf2, bf2):
    src = edge_index[0].astype(jnp.int32)
    dst = edge_index[1].astype(jnp.int32)
    loops = jnp.arange(N, dtype=jnp.int32)
    padidx = (N + (jnp.arange(EP - E - N, dtype=jnp.int32) % (NP - N)))
    src2 = jnp.concatenate([src, loops, padidx]).reshape(NCHUNK, CH)
    dst2 = jnp.concatenate([dst, loops, padidx]).reshape(NCHUNK, CH)
    xp = jnp.pad(x, ((0, NP - N), (0, 0)))

    eye4 = jnp.eye(HEADS, dtype=_f32)
    as_mat = (att_src1[:, :, None] * eye4[:, None, :]).reshape(256, HEADS)
    as_mat = jnp.concatenate([as_mat, jnp.zeros((256, 12), _f32)], axis=1)
    ad_mat = (att_dst1[:, :, None] * eye4[:, None, :]).reshape(256, HEADS)
    ad_mat = jnp.concatenate([ad_mat, jnp.zeros((256, 12), _f32)], axis=1)
    as2_mat = jnp.concatenate([att_src2.T, jnp.zeros((HID, 15), _f32)], axis=1)
    ad2_mat = jnp.concatenate([att_dst2.T, jnp.zeros((HID, 15), _f32)], axis=1)

    bn1_scale = (bn1_gamma / jnp.sqrt(bn1_var + 1e-5)).reshape(1, 256)
    bn1_shift = (bn1_beta - bn1_mean * bn1_scale[0]).reshape(1, 256)
    bns_scale = (bns_gamma / jnp.sqrt(bns_var + 1e-5)).reshape(1, HID)
    bns_shift = (bns_beta - bns_mean * bns_scale[0]).reshape(1, HID)

    h1p3, t1s, t1d, xwl, xwr = _tc1(xp, W_gat1, as_mat, ad_mat, Wl1, Wr1)
    h1p = h1p3.reshape(2 * NP, 128)

    den_o, acc1_o = _sc1(src2, dst2, t1s, t1d, h1p)
    accs_o = _sc1b(src2, dst2, xwl)

    h2p, t2s, t2d, s1wl2, s1wr2 = _tc2(
        den_o, acc1_o, accs_o, xwr, b_gat1.reshape(1, 256), bn1_scale,
        bn1_shift, W_gat2, as2_mat, ad2_mat, bl1.reshape(1, HID), bns_scale,
        bns_shift, Wl2, Wr2)

    den2_o, acc2_o, accs2_o = _sc2(src2, dst2, t2s, t2d, h2p, s1wl2)

    wf2p = jnp.concatenate([Wf2, jnp.zeros((HID, 126), _f32)], axis=1)
    bf2p = jnp.concatenate([bf2, jnp.zeros((126,), _f32)]).reshape(1, 128)
    outp = _tc3(acc2_o, den2_o, accs2_o, s1wr2, den_o,
                b_gat2.reshape(1, HID), bl2.reshape(1, HID), Wf1,
                bf1.reshape(1, HID), wf2p, bf2p)[0]
    return outp[:, :2]


# speedup vs baseline: 69.9192x; 1.0838x over previous
import jax
import jax.numpy as jnp
from jax import lax
from jax.experimental import pallas as pl
from jax.experimental.pallas import tpu as pltpu
from jax.experimental.pallas import tpu_sc as plsc

N = 10000
NP = 10240
E = 320000
F_IN = 128
HID = 64
HEADS = 4
EP = 331776
CH = 64
NCHUNK = EP // CH
REAL = E // CH
NSC = 2
NTILE = 16
NWORK = NSC * NTILE
STRIPE = NP // NTILE
K1 = 12
K2 = 9
RB = 1024
GRID = NP // RB

_f32 = jnp.float32
_SC_PARAMS = dict(
    compiler_params=pltpu.CompilerParams(
        needs_layout_passes=False, use_tc_tiling_on_sc=False),
)


def _sc_mesh():
    return plsc.VectorSubcoreMesh(core_axis_name="c", subcore_axis_name="s",
                                  num_cores=NSC, num_subcores=NTILE)


def _tc1_body(x_ref, wg1_ref, as_ref, ad_ref, wl1_ref, wr1_ref,
              h1p_ref, t1s_ref, t1d_ref, xwl_ref, xwr_ref):
    xb = x_ref[...]
    h1 = jnp.dot(xb, wg1_ref[...], preferred_element_type=_f32)
    h1p_ref[0] = h1[:, :128]
    h1p_ref[1] = h1[:, 128:]
    t1s_ref[...] = jnp.dot(h1, as_ref[...], preferred_element_type=_f32)
    t1d = jnp.dot(h1, ad_ref[...], preferred_element_type=_f32)
    i = pl.program_id(0)
    rows = lax.broadcasted_iota(jnp.int32, (RB, 16), 0) + i * RB
    lanev = lax.broadcasted_iota(jnp.int32, (RB, 16), 1)
    valid = rows < N
    t1d_ref[...] = jnp.where(
        valid, t1d + (lanev == 4).astype(_f32),
        jnp.where(lanev < 4, -1e30, 0.0))
    xwl_ref[...] = jnp.dot(xb, wl1_ref[...], preferred_element_type=_f32)
    xwr_ref[...] = jnp.dot(xb, wr1_ref[...], preferred_element_type=_f32)


def _tc1(xp, wg1, as_mat, ad_mat, wl1, wr1):
    full = lambda shape: pl.BlockSpec(shape, lambda i: (0,) * len(shape))
    return pl.pallas_call(
        _tc1_body,
        grid=(GRID,),
        in_specs=[
            pl.BlockSpec((RB, F_IN), lambda i: (i, 0)),
            full((F_IN, 256)), full((256, 16)), full((256, 16)),
            full((F_IN, HID)), full((F_IN, HID)),
        ],
        out_specs=[
            pl.BlockSpec((2, RB, 128), lambda i: (0, i, 0)),
            pl.BlockSpec((RB, 16), lambda i: (i, 0)),
            pl.BlockSpec((RB, 16), lambda i: (i, 0)),
            pl.BlockSpec((RB, HID), lambda i: (i, 0)),
            pl.BlockSpec((RB, HID), lambda i: (i, 0)),
        ],
        out_shape=[
            jax.ShapeDtypeStruct((2, NP, 128), _f32),
            jax.ShapeDtypeStruct((NP, 16), _f32),
            jax.ShapeDtypeStruct((NP, 16), _f32),
            jax.ShapeDtypeStruct((NP, HID), _f32),
            jax.ShapeDtypeStruct((NP, HID), _f32),
        ],
    )(xp, wg1, as_mat, ad_mat, wl1, wr1)


def _sc1_body(src2_ref, dst2_ref, t1s_ref, t1d_ref, h1p_ref,
              den_o, acc1_o,
              idxs_blk, idxd_blk, adj0, adj1, ts0, ts1, td0, td1,
              er0, er1, ef0, ef1, hr0, hr1, hr2,
              den_sh, acc1_sh,
              sts0, sts1, std0, std1, sh0, sh1, sh2,
              sd0, sd1, sa0, sa1, sa2):
    c = lax.axis_index("c")
    s = lax.axis_index("s")
    adjb = [adj0, adj1]
    tsb = [ts0, ts1]
    tdb = [td0, td1]
    erb = [er0, er1]
    efb = [ef0, ef1]
    hrb = [hr0, hr1, hr2]
    sts = [sts0, sts1]
    std = [std0, std1]
    sh = [sh0, sh1, sh2]
    sd = [sd0, sd1]
    sa = [sa0, sa1, sa2]

    def zrow(j, _):
        z = jnp.zeros((16,), _f32)
        er0[j, :] = z
        for k in range(8):
            hr0[j, pl.ds(k * 16, 16)] = z
        return 0
    lax.fori_loop(0, CH, zrow, 0)

    def zstripe(k, _):
        base = s * STRIPE + k * CH
        pltpu.sync_copy(er0, den_sh.at[pl.ds(base, CH)])
        pltpu.sync_copy(hr0, acc1_sh.at[pl.ds(base, CH)])
        return 0
    lax.fori_loop(0, STRIPE // CH, zstripe, 0)
    plsc.subcore_barrier()

    lanev = lax.iota(jnp.int32, 16)
    zero16i = jnp.zeros((16,), jnp.int32)
    idxh0v = zero16i + 2 * c
    idxh1v = idxh0v + 1
    coff = c * NP
    nct = NCHUNK // NTILE
    nblk = nct // K1

    def blk_body(bi, _):
        row0 = s * nct + bi * K1
        pltpu.sync_copy(src2_ref.at[pl.ds(row0, K1)], idxs_blk)
        pltpu.sync_copy(dst2_ref.at[pl.ds(row0, K1)], idxd_blk)
        sden = [None, None]
        sacc = [None, None, None]

        def issue(jj):
            b2 = jj % 2
            b3 = jj % 3
            for k in range(CH // 16):
                adjb[b2][pl.ds(k * 16, 16)] = (
                    idxs_blk[jj, pl.ds(k * 16, 16)] + coff)
            dts = pltpu.async_copy(t1s_ref.at[idxs_blk.at[jj]], tsb[b2], sts[b2])
            dtd = pltpu.async_copy(t1d_ref.at[idxd_blk.at[jj]], tdb[b2], std[b2])
            if sacc[b3] is not None:
                sacc[b3].wait()
                sacc[b3] = None
            dh = pltpu.async_copy(h1p_ref.at[adjb[b2]], hrb[b3], sh[b3])
            return dts, dtd, dh

        d = [None, None]
        d[0] = issue(0)
        for j in range(K1):
            cur = j % 2
            nxt = 1 - cur
            c3 = j % 3
            if j + 1 < K1:
                d[nxt] = issue(j + 1)
            g = row0 + j
            realf = jnp.where(g < REAL, 1.0, 0.0).astype(_f32)
            dts, dtd, dh = d[cur]
            dts.wait()
            dtd.wait()
            dh.wait()
            if sden[cur] is not None:
                sden[cur].wait()
                sden[cur] = None
            ts_c, td_c, er_c, ef_c, hr_c = (
                tsb[cur], tdb[cur], erb[cur], efb[cur], hrb[c3])

            @plsc.parallel_loop(0, CH, unroll=4)
            def _(jj):
                al = ts_c[jj, :] + td_c[jj, :]
                lr = jnp.where(al > 0, al, 0.2 * al)
                ev = jnp.exp(lr)
                out = jnp.where(
                    lanev < 4, ev, jnp.where(lanev == 4, al * realf, 0.0))
                er_c[jj, :] = out
                ef_c[pl.ds(jj * 16, 16)] = out
                jv = zero16i + jj * 16
                w0 = plsc.load_gather(ef_c, [jv + idxh0v])
                w1 = plsc.load_gather(ef_c, [jv + idxh1v])
                for k in range(4):
                    hr_c[jj, pl.ds(k * 16, 16)] = (
                        hr_c[jj, pl.ds(k * 16, 16)] * w0)
                for k in range(4, 8):
                    hr_c[jj, pl.ds(k * 16, 16)] = (
                        hr_c[jj, pl.ds(k * 16, 16)] * w1)

            sden[cur] = pltpu.async_copy(
                er_c, den_sh.at[idxd_blk.at[j]], sd[cur], add=True)
            sacc[c3] = pltpu.async_copy(
                hr_c, acc1_sh.at[idxd_blk.at[j]], sa[c3], add=True)
        for dd in sden + sacc:
            if dd is not None:
                dd.wait()
        return 0
    lax.fori_loop(0, nblk, blk_body, 0)
    plsc.subcore_barrier()

    rbase = s * STRIPE
    obase = c * NP + rbase
    pltpu.sync_copy(den_sh.at[pl.ds(rbase, STRIPE)], den_o.at[pl.ds(obase, STRIPE)])
    pltpu.sync_copy(acc1_sh.at[pl.ds(rbase, STRIPE)], acc1_o.at[pl.ds(obase, STRIPE)])


def _sc1(src2, dst2, t1s, t1d, h1p):
    return pl.kernel(
        _sc1_body,
        out_type=(
            jax.ShapeDtypeStruct((2 * NP, 16), _f32),
            jax.ShapeDtypeStruct((2 * NP, 128), _f32),
        ),
        mesh=_sc_mesh(),
        **_SC_PARAMS,
        scratch_types=[
            pltpu.VMEM((K1, CH), jnp.int32),
            pltpu.VMEM((K1, CH), jnp.int32),
            pltpu.VMEM((CH,), jnp.int32),
            pltpu.VMEM((CH,), jnp.int32),
            pltpu.VMEM((CH, 16), _f32),
            pltpu.VMEM((CH, 16), _f32),
            pltpu.VMEM((CH, 16), _f32),
            pltpu.VMEM((CH, 16), _f32),
            pltpu.VMEM((CH, 16), _f32),
            pltpu.VMEM((CH, 16), _f32),
            pltpu.VMEM((CH * 16,), _f32),
            pltpu.VMEM((CH * 16,), _f32),
            pltpu.VMEM((CH, 128), _f32),
            pltpu.VMEM((CH, 128), _f32),
            pltpu.VMEM((CH, 128), _f32),
            pltpu.VMEM_SHARED((NP, 16), _f32),
            pltpu.VMEM_SHARED((NP, 128), _f32),
        ] + [pltpu.SemaphoreType.DMA] * 12,
    )(src2, dst2, t1s, t1d, h1p)


def _sc1b_body(src2_ref, dst2_ref, xwl_ref, accs_o,
               idxs_blk, idxd_blk, sg0, sg1, sg2, sg3, accs_sh,
               ss0, ss1, ss2, ss3, sa0, sa1, sa2, sa3):
    c = lax.axis_index("c")
    s = lax.axis_index("s")
    sgb = [sg0, sg1, sg2, sg3]
    ssb = [ss0, ss1, ss2, ss3]
    sab = [sa0, sa1, sa2, sa3]
    NB = 4

    def zrow(j, _):
        z = jnp.zeros((16,), _f32)
        for k in range(4):
            sg0[j, pl.ds(k * 16, 16)] = z
        return 0
    lax.fori_loop(0, CH, zrow, 0)

    def zstripe(k, _):
        pltpu.sync_copy(sg0, accs_sh.at[pl.ds(s * STRIPE + k * CH, CH)])
        return 0
    lax.fori_loop(0, STRIPE // CH, zstripe, 0)
    plsc.subcore_barrier()

    lanev = lax.iota(jnp.int32, 16)
    nct = NCHUNK // NWORK
    nblk = nct // K2
    wid = c * NTILE + s

    def blk_body(bi, _):
        row0 = wid * nct + bi * K2
        pltpu.sync_copy(src2_ref.at[pl.ds(row0, K2)], idxs_blk)
        pltpu.sync_copy(dst2_ref.at[pl.ds(row0, K2)], idxd_blk)
        scat = [None] * NB

        def issue(jj):
            b = jj % NB
            if scat[b] is not None:
                scat[b].wait()
                scat[b] = None
            return pltpu.async_copy(xwl_ref.at[idxs_blk.at[jj]], sgb[b], ssb[b])

        d = {}
        for jj in range(min(NB - 1, K2)):
            d[jj] = issue(jj)
        for j in range(K2):
            b = j % NB
            if j + NB - 1 < K2:
                d[j + NB - 1] = issue(j + NB - 1)
            g = row0 + j
            d[j].wait()

            @pl.when(g >= REAL)
            def _():
                for k in range(CH // 16):
                    idxd_blk[j, pl.ds(k * 16, 16)] = (N + k * 16) + lanev
            scat[b] = pltpu.async_copy(
                sgb[b], accs_sh.at[idxd_blk.at[j]], sab[b], add=True)
        for b in range(NB):
            if scat[b] is not None:
                scat[b].wait()
        return 0
    lax.fori_loop(0, nblk, blk_body, 0)
    plsc.subcore_barrier()

    rbase = s * STRIPE
    pltpu.sync_copy(accs_sh.at[pl.ds(rbase, STRIPE)],
                    accs_o.at[pl.ds(c * NP + rbase, STRIPE)])


def _sc1b(src2, dst2, xwl):
    return pl.kernel(
        _sc1b_body,
        out_type=jax.ShapeDtypeStruct((2 * NP, HID), _f32),
        mesh=_sc_mesh(),
        **_SC_PARAMS,
        scratch_types=[
            pltpu.VMEM((K2, CH), jnp.int32),
            pltpu.VMEM((K2, CH), jnp.int32),
            pltpu.VMEM((CH, HID), _f32),
            pltpu.VMEM((CH, HID), _f32),
            pltpu.VMEM((CH, HID), _f32),
            pltpu.VMEM((CH, HID), _f32),
            pltpu.VMEM_SHARED((NP, HID), _f32),
        ] + [pltpu.SemaphoreType.DMA] * 8,
    )(src2, dst2, xwl)


def _tc2_body(acc1a_ref, acc1b_ref, den_ref, accsa_ref, accsb_ref, xwr_ref,
              bg1_ref, s1c_ref, s1h_ref, wg2_ref, as2_ref, ad2_ref,
              bl1_ref, ssc_ref, ssh_ref, wl2_ref, wr2_ref,
              h2p_ref, t2s_ref, t2d_ref, s1wl2_ref, s1wr2_ref):
    den = den_ref[...]
    mcnt = jnp.maximum(den[:, 4:5], 1.0)
    a = acc1a_ref[...]
    b = acc1b_ref[...]
    g1 = jnp.concatenate([
        a[:, :64] / (den[:, 0:1] + 1e-16),
        a[:, 64:] / (den[:, 1:2] + 1e-16),
        b[:, :64] / (den[:, 2:3] + 1e-16),
        b[:, 64:] / (den[:, 3:4] + 1e-16)], axis=1)
    g1 = g1 + bg1_ref[...]
    g1b = g1 * s1c_ref[...] + s1h_ref[...]
    g1e = jnp.where(g1b > 0, g1b, jnp.exp(g1b) - 1.0)
    h2 = jnp.dot(g1e, wg2_ref[...], preferred_element_type=_f32)
    h2p_ref[...] = h2
    t2s_ref[...] = jnp.dot(h2, as2_ref[...], preferred_element_type=_f32)
    t2d = jnp.dot(h2, ad2_ref[...], preferred_element_type=_f32)
    i = pl.program_id(0)
    rows = lax.broadcasted_iota(jnp.int32, (RB, 16), 0) + i * RB
    lanev = lax.broadcasted_iota(jnp.int32, (RB, 16), 1)
    t2d_ref[...] = jnp.where(
        rows < N, t2d, jnp.where(lanev < 1, -1e30, 0.0))
    accs = accsa_ref[...] + accsb_ref[...]
    s1 = accs / mcnt + bl1_ref[...] + xwr_ref[...]
    s1b = s1 * ssc_ref[...] + ssh_ref[...]
    s1r = jnp.maximum(s1b, 0.0)
    s1wl2_ref[...] = jnp.dot(s1r, wl2_ref[...], preferred_element_type=_f32)
    s1wr2_ref[...] = jnp.dot(s1r, wr2_ref[...], preferred_element_type=_f32)


def _tc2(den_acc, acc1, accs, xwr, bg1, bn1_scale, bn1_shift, wg2, as2_mat,
         ad2_mat, bl1, bns_scale, bns_shift, wl2, wr2):
    full = lambda shape: pl.BlockSpec(shape, lambda i: (0,) * len(shape))
    blk = lambda w: pl.BlockSpec((RB, w), lambda i: (i, 0))
    blk_hi = lambda w: pl.BlockSpec((RB, w), lambda i: (i + GRID, 0))
    return pl.pallas_call(
        _tc2_body,
        grid=(GRID,),
        in_specs=[
            blk(128), blk_hi(128), blk(16), blk(HID), blk_hi(HID), blk(HID),
            full((1, 256)), full((1, 256)), full((1, 256)),
            full((256, HID)), full((HID, 16)), full((HID, 16)),
            full((1, HID)), full((1, HID)), full((1, HID)),
            full((HID, HID)), full((HID, HID)),
        ],
        out_specs=[blk(HID), blk(16), blk(16), blk(HID), blk(HID)],
        out_shape=[
            jax.ShapeDtypeStruct((NP, HID), _f32),
            jax.ShapeDtypeStruct((NP, 16), _f32),
            jax.ShapeDtypeStruct((NP, 16), _f32),
            jax.ShapeDtypeStruct((NP, HID), _f32),
            jax.ShapeDtypeStruct((NP, HID), _f32),
        ],
    )(acc1, acc1, den_acc, accs, accs, xwr, bg1, bn1_scale, bn1_shift,
      wg2, as2_mat, ad2_mat, bl1, bns_scale, bns_shift, wl2, wr2)


def _sc2_body(src2_ref, dst2_ref, t2s_ref, t2d_ref, h2p_ref, swl_ref,
              den_o, acc2_o, accs2_o,
              idxs_blk, idxd_blk, ixg0, ixg1, ixg2, ts0, ts1, td0, td1,
              er0, er1, ef0, ef1, hr0, hr1, hr2, sg0, sg1, sg2,
              den_sh, acc2_sh, accs2_sh,
              sts0, sts1, std0, std1, sh0, sh1, sh2, ss0, ss1, ss2,
              sd0, sd1, sa0, sa1, sa2, sb0, sb1, sb2):
    c = lax.axis_index("c")
    s = lax.axis_index("s")
    tsb = [ts0, ts1]
    tdb = [td0, td1]
    erb = [er0, er1]
    efb = [ef0, ef1]
    hrb = [hr0, hr1, hr2]
    sgb = [sg0, sg1, sg2]
    sts = [sts0, sts1]
    std = [std0, std1]
    sh = [sh0, sh1, sh2]
    ssb = [ss0, ss1, ss2]
    sd = [sd0, sd1]
    sa = [sa0, sa1, sa2]
    sbb = [sb0, sb1, sb2]
    ixg = [ixg0, ixg1, ixg2]

    def zrow(j, _):
        z = jnp.zeros((16,), _f32)
        er0[j, :] = z
        for k in range(4):
            hr0[j, pl.ds(k * 16, 16)] = z
        return 0
    lax.fori_loop(0, CH, zrow, 0)

    def zstripe(k, _):
        base = s * STRIPE + k * CH
        pltpu.sync_copy(er0, den_sh.at[pl.ds(base, CH)])
        pltpu.sync_copy(hr0, acc2_sh.at[pl.ds(base, CH)])
        pltpu.sync_copy(hr0, accs2_sh.at[pl.ds(base, CH)])
        return 0
    lax.fori_loop(0, STRIPE // CH, zstripe, 0)
    plsc.subcore_barrier()

    lanev = lax.iota(jnp.int32, 16)
    zero16i = jnp.zeros((16,), jnp.int32)
    nct = NCHUNK // NWORK
    nblk = nct // K2
    wid = c * NTILE + s

    def blk_body(bi, _):
        row0 = wid * nct + bi * K2
        pltpu.sync_copy(src2_ref.at[pl.ds(row0, K2)], idxs_blk)
        pltpu.sync_copy(dst2_ref.at[pl.ds(row0, K2)], idxd_blk)
        sden = [None, None]
        sacc = [None, None, None]
        ssage = [None, None, None]

        def issue(jj):
            b2 = jj % 2
            b3 = jj % 3
            dts = pltpu.async_copy(t2s_ref.at[idxs_blk.at[jj]], tsb[b2], sts[b2])
            dtd = pltpu.async_copy(t2d_ref.at[idxd_blk.at[jj]], tdb[b2], std[b2])
            if sacc[b3] is not None:
                sacc[b3].wait()
                sacc[b3] = None
            dh = pltpu.async_copy(h2p_ref.at[idxs_blk.at[jj]], hrb[b3], sh[b3])
            if ssage[b3] is not None:
                ssage[b3].wait()
                ssage[b3] = None
            dsg = pltpu.async_copy(swl_ref.at[idxs_blk.at[jj]], sgb[b3], ssb[b3])
            return dts, dtd, dh, dsg

        d = [None, None]
        d[0] = issue(0)
        for j in range(K2):
            cur = j % 2
            nxt = 1 - cur
            c3 = j % 3
            if j + 1 < K2:
                d[nxt] = issue(j + 1)
            g = row0 + j
            dts, dtd, dh, dsg = d[cur]
            dts.wait()
            dtd.wait()
            dh.wait()
            if sden[cur] is not None:
                sden[cur].wait()
                sden[cur] = None
            ts_c, td_c, er_c, ef_c, hr_c = (
                tsb[cur], tdb[cur], erb[cur], efb[cur], hrb[c3])

            @plsc.parallel_loop(0, CH, unroll=4)
            def _(jj):
                al = ts_c[jj, :] + td_c[jj, :]
                lr = jnp.where(al > 0, al, 0.2 * al)
                ev = jnp.exp(lr)
                out = jnp.where(lanev < 1, ev, 0.0)
                er_c[jj, :] = out
                ef_c[pl.ds(jj * 16, 16)] = out
                jv = zero16i + jj * 16
                w0 = plsc.load_gather(ef_c, [jv])
                for k in range(4):
                    hr_c[jj, pl.ds(k * 16, 16)] = (
                        hr_c[jj, pl.ds(k * 16, 16)] * w0)

            sden[cur] = pltpu.async_copy(
                er_c, den_sh.at[idxd_blk.at[j]], sd[cur], add=True)
            sacc[c3] = pltpu.async_copy(
                hr_c, acc2_sh.at[idxd_blk.at[j]], sa[c3], add=True)
            dsg.wait()
            for k in range(CH // 16):
                ixg[c3][pl.ds(k * 16, 16)] = idxd_blk[j, pl.ds(k * 16, 16)]

            @pl.when(g >= REAL)
            def _():
                for k in range(CH // 16):
                    ixg[c3][pl.ds(k * 16, 16)] = (N + k * 16) + lanev
            ssage[c3] = pltpu.async_copy(
                sgb[c3], accs2_sh.at[ixg[c3]], sbb[c3], add=True)
        for dd in sden + sacc + ssage:
            if dd is not None:
                dd.wait()
        return 0
    lax.fori_loop(0, nblk, blk_body, 0)
    plsc.subcore_barrier()

    rbase = s * STRIPE
    obase = c * NP + rbase
    pltpu.sync_copy(den_sh.at[pl.ds(rbase, STRIPE)], den_o.at[pl.ds(obase, STRIPE)])
    pltpu.sync_copy(acc2_sh.at[pl.ds(rbase, STRIPE)], acc2_o.at[pl.ds(obase, STRIPE)])
    pltpu.sync_copy(accs2_sh.at[pl.ds(rbase, STRIPE)], accs2_o.at[pl.ds(obase, STRIPE)])


def _sc2(src2, dst2, t2s, t2d, h2p, s1wl2):
    return pl.kernel(
        _sc2_body,
        out_type=(
            jax.ShapeDtypeStruct((2 * NP, 16), _f32),
            jax.ShapeDtypeStruct((2 * NP, HID), _f32),
            jax.ShapeDtypeStruct((2 * NP, HID), _f32),
        ),
        mesh=_sc_mesh(),
        **_SC_PARAMS,
        scratch_types=[
            pltpu.VMEM((K2, CH), jnp.int32),
            pltpu.VMEM((K2, CH), jnp.int32),
            pltpu.VMEM((CH,), jnp.int32),
            pltpu.VMEM((CH,), jnp.int32),
            pltpu.VMEM((CH,), jnp.int32),
            pltpu.VMEM((CH, 16), _f32),
            pltpu.VMEM((CH, 16), _f32),
            pltpu.VMEM((CH, 16), _f32),
            pltpu.VMEM((CH, 16), _f32),
            pltpu.VMEM((CH, 16), _f32),
            pltpu.VMEM((CH, 16), _f32),
            pltpu.VMEM((CH * 16,), _f32),
            pltpu.VMEM((CH * 16,), _f32),
            pltpu.VMEM((CH, HID), _f32),
            pltpu.VMEM((CH, HID), _f32),
            pltpu.VMEM((CH, HID), _f32),
            pltpu.VMEM((CH, HID), _f32),
            pltpu.VMEM((CH, HID), _f32),
            pltpu.VMEM((CH, HID), _f32),
            pltpu.VMEM_SHARED((NP, 16), _f32),
            pltpu.VMEM_SHARED((NP, HID), _f32),
            pltpu.VMEM_SHARED((NP, HID), _f32),
        ] + [pltpu.SemaphoreType.DMA] * 18,
    )(src2, dst2, t2s, t2d, h2p, s1wl2)


def _tc3_body(acc2a_ref, acc2b_ref, den2a_ref, den2b_ref, accs2a_ref,
              accs2b_ref, s1wr2_ref, den_ref, bg2_ref, bl2_ref, wf1_ref,
              bf1_ref, wf2_ref, bf2_ref, out_ref):
    den2 = den2a_ref[...] + den2b_ref[...]
    g2 = (acc2a_ref[...] + acc2b_ref[...]) / (den2[:, 0:1] + 1e-16)
    g2 = g2 + bg2_ref[...]
    mcnt = jnp.maximum(den_ref[:, 4:5], 1.0)
    s2 = (accs2a_ref[...] + accs2b_ref[...]) / mcnt + bl2_ref[...] + s1wr2_ref[...]
    cc = jnp.concatenate([g2, s2], axis=1)
    h = jnp.maximum(jnp.dot(cc, wf1_ref[...], preferred_element_type=_f32)
                    + bf1_ref[...], 0.0)
    out_ref[...] = jnp.dot(h, wf2_ref[...], preferred_element_type=_f32) + bf2_ref[...]


def _tc3(acc2, den2, accs2, s1wr2, den_acc, bg2, bl2, wf1, bf1, wf2p, bf2p):
    full = lambda shape: pl.BlockSpec(shape, lambda i: (0,) * len(shape))
    blk = lambda w: pl.BlockSpec((RB, w), lambda i: (i, 0))
    blk_hi = lambda w: pl.BlockSpec((RB, w), lambda i: (i + GRID, 0))
    return pl.pallas_call(
        _tc3_body,
        grid=(GRID,),
        in_specs=[
            blk(HID), blk_hi(HID), blk(16), blk_hi(16), blk(HID), blk_hi(HID),
            blk(HID), blk(16),
            full((1, HID)), full((1, HID)), full((2 * HID, HID)),
            full((1, HID)), full((HID, 128)), full((1, 128)),
        ],
        out_specs=[pl.BlockSpec((RB, 128), lambda i: (i, 0))],
        out_shape=[jax.ShapeDtypeStruct((N, 128), _f32)],
    )(acc2, acc2, den2, den2, accs2, accs2, s1wr2, den_acc, bg2, bl2, wf1,
      bf1, wf2p, bf2p)


@jax.jit
def kernel(x, edge_index, W_gat1, att_src1, att_dst1, b_gat1, bn1_gamma,
           bn1_beta, bn1_mean, bn1_var, W_gat2, att_src2, att_dst2, b_gat2,
           Wl1, bl1, Wr1, bns_gamma, bns_beta, bns_mean, bns_var, Wl2, bl2,
           Wr2, Wf1, bf1, Wf2, bf2):
    src = edge_index[0].astype(jnp.int32)
    dst = edge_index[1].astype(jnp.int32)
    loops = jnp.arange(N, dtype=jnp.int32)
    padidx = (N + (jnp.arange(EP - E - N, dtype=jnp.int32) % (NP - N)))
    src2 = jnp.concatenate([src, loops, padidx]).reshape(NCHUNK, CH)
    dst2 = jnp.concatenate([dst, loops, padidx]).reshape(NCHUNK, CH)
    xp = jnp.pad(x, ((0, NP - N), (0, 0)))

    eye4 = jnp.eye(HEADS, dtype=_f32)
    as_mat = (att_src1[:, :, None] * eye4[:, None, :]).reshape(256, HEADS)
    as_mat = jnp.concatenate([as_mat, jnp.zeros((256, 12), _f32)], axis=1)
    ad_mat = (att_dst1[:, :, None] * eye4[:, None, :]).reshape(256, HEADS)
    ad_mat = jnp.concatenate([ad_mat, jnp.zeros((256, 12), _f32)], axis=1)
    as2_mat = jnp.concatenate([att_src2.T, jnp.zeros((HID, 15), _f32)], axis=1)
    ad2_mat = jnp.concatenate([att_dst2.T, jnp.zeros((HID, 15), _f32)], axis=1)

    bn1_scale = (bn1_gamma / jnp.sqrt(bn1_var + 1e-5)).reshape(1, 256)
    bn1_shift = (bn1_beta - bn1_mean * bn1_scale[0]).reshape(1, 256)
    bns_scale = (bns_gamma / jnp.sqrt(bns_var + 1e-5)).reshape(1, HID)
    bns_shift = (bns_beta - bns_mean * bns_scale[0]).reshape(1, HID)

    h1p3, t1s, t1d, xwl, xwr = _tc1(xp, W_gat1, as_mat, ad_mat, Wl1, Wr1)
    h1p = h1p3.reshape(2 * NP, 128)

    den_o, acc1_o = _sc1(src2, dst2, t1s, t1d, h1p)
    accs_o = _sc1b(src2, dst2, xwl)

    h2p, t2s, t2d, s1wl2, s1wr2 = _tc2(
        den_o, acc1_o, accs_o, xwr, b_gat1.reshape(1, 256), bn1_scale,
        bn1_shift, W_gat2, as2_mat, ad2_mat, bl1.reshape(1, HID), bns_scale,
        bns_shift, Wl2, Wr2)

    den2_o, acc2_o, accs2_o = _sc2(src2, dst2, t2s, t2d, h2p, s1wl2)

    wf2p = jnp.concatenate([Wf2, jnp.zeros((HID, 126), _f32)], axis=1)
    bf2p = jnp.concatenate([bf2, jnp.zeros((126,), _f32)]).reshape(1, 128)
    outp = _tc3(acc2_o, den2_o, accs2_o, s1wr2, den_o,
                b_gat2.reshape(1, HID), bl2.reshape(1, HID), Wf1,
                bf1.reshape(1, HID), wf2p, bf2p)[0]
    return outp[:, :2]


# speedup vs baseline: 71.2763x; 1.0194x over previous
import jax
import jax.numpy as jnp
from jax import lax
from jax.experimental import pallas as pl
from jax.experimental.pallas import tpu as pltpu
from jax.experimental.pallas import tpu_sc as plsc

N = 10000
NP = 10240
E = 320000
F_IN = 128
HID = 64
HEADS = 4
EP = 331776
CH = 64
NCHUNK = EP // CH
REAL = E // CH
NSC = 2
NTILE = 16
NWORK = NSC * NTILE
STRIPE = NP // NTILE
K1 = 12
K2 = 9
RB = 1024
GRID = NP // RB

_f32 = jnp.float32
_SC_PARAMS = dict(
    compiler_params=pltpu.CompilerParams(
        needs_layout_passes=False, use_tc_tiling_on_sc=False),
)


def _sc_mesh():
    return plsc.VectorSubcoreMesh(core_axis_name="c", subcore_axis_name="s",
                                  num_cores=NSC, num_subcores=NTILE)


def _tc1_body(x_ref, wg1_ref, as_ref, ad_ref, wl1_ref, wr1_ref,
              h1p_ref, t1s_ref, t1d_ref, xwl_ref, xwr_ref):
    xb = x_ref[...]
    h1 = jnp.dot(xb, wg1_ref[...], preferred_element_type=_f32)
    h1p_ref[0] = h1[:, :128]
    h1p_ref[1] = h1[:, 128:]
    t1s_ref[...] = jnp.dot(h1, as_ref[...], preferred_element_type=_f32)
    t1d = jnp.dot(h1, ad_ref[...], preferred_element_type=_f32)
    i = pl.program_id(0)
    rows = lax.broadcasted_iota(jnp.int32, (RB, 16), 0) + i * RB
    lanev = lax.broadcasted_iota(jnp.int32, (RB, 16), 1)
    valid = rows < N
    t1d_ref[...] = jnp.where(
        valid, t1d + (lanev == 4).astype(_f32),
        jnp.where(lanev < 4, -1e30, 0.0))
    xwl_ref[...] = jnp.dot(xb, wl1_ref[...], preferred_element_type=_f32)
    xwr_ref[...] = jnp.dot(xb, wr1_ref[...], preferred_element_type=_f32)


def _tc1(xp, wg1, as_mat, ad_mat, wl1, wr1):
    full = lambda shape: pl.BlockSpec(shape, lambda i: (0,) * len(shape))
    return pl.pallas_call(
        _tc1_body,
        grid=(GRID,),
        in_specs=[
            pl.BlockSpec((RB, F_IN), lambda i: (i, 0)),
            full((F_IN, 256)), full((256, 16)), full((256, 16)),
            full((F_IN, HID)), full((F_IN, HID)),
        ],
        out_specs=[
            pl.BlockSpec((2, RB, 128), lambda i: (0, i, 0)),
            pl.BlockSpec((RB, 16), lambda i: (i, 0)),
            pl.BlockSpec((RB, 16), lambda i: (i, 0)),
            pl.BlockSpec((RB, HID), lambda i: (i, 0)),
            pl.BlockSpec((RB, HID), lambda i: (i, 0)),
        ],
        out_shape=[
            jax.ShapeDtypeStruct((2, NP, 128), _f32),
            jax.ShapeDtypeStruct((NP, 16), _f32),
            jax.ShapeDtypeStruct((NP, 16), _f32),
            jax.ShapeDtypeStruct((NP, HID), _f32),
            jax.ShapeDtypeStruct((NP, HID), _f32),
        ],
    )(xp, wg1, as_mat, ad_mat, wl1, wr1)


def _sc1_body(src2_ref, dst2_ref, t1s_ref, t1d_ref, h1p_ref,
              den_o, acc1_o,
              idxs_blk, idxd_blk, adj0, adj1, ts0, ts1, td0, td1,
              er0, er1, ef0, ef1, hr0, hr1, hr2,
              den_sh, acc1_sh,
              sts0, sts1, std0, std1, sh0, sh1, sh2,
              sd0, sd1, sa0, sa1, sa2):
    c = lax.axis_index("c")
    s = lax.axis_index("s")
    adjb = [adj0, adj1]
    tsb = [ts0, ts1]
    tdb = [td0, td1]
    erb = [er0, er1]
    efb = [ef0, ef1]
    hrb = [hr0, hr1, hr2]
    sts = [sts0, sts1]
    std = [std0, std1]
    sh = [sh0, sh1, sh2]
    sd = [sd0, sd1]
    sa = [sa0, sa1, sa2]

    def zrow(j, _):
        z = jnp.zeros((16,), _f32)
        er0[j, :] = z
        for k in range(8):
            hr0[j, pl.ds(k * 16, 16)] = z
        return 0
    lax.fori_loop(0, CH, zrow, 0)

    def zstripe(k, _):
        base = s * STRIPE + k * CH
        pltpu.sync_copy(er0, den_sh.at[pl.ds(base, CH)])
        pltpu.sync_copy(hr0, acc1_sh.at[pl.ds(base, CH)])
        return 0
    lax.fori_loop(0, STRIPE // CH, zstripe, 0)
    plsc.subcore_barrier()

    lanev = lax.iota(jnp.int32, 16)
    zero16i = jnp.zeros((16,), jnp.int32)
    idxh0v = zero16i + 2 * c
    idxh1v = idxh0v + 1
    coff = c * NP
    nct = NCHUNK // NTILE
    nblk = nct // K1

    def blk_body(bi, _):
        row0 = s * nct + bi * K1
        pltpu.sync_copy(src2_ref.at[pl.ds(row0, K1)], idxs_blk)
        pltpu.sync_copy(dst2_ref.at[pl.ds(row0, K1)], idxd_blk)
        sden = [None, None]
        sacc = [None, None, None]

        def issue(jj):
            b2 = jj % 2
            b3 = jj % 3
            for k in range(CH // 16):
                adjb[b2][pl.ds(k * 16, 16)] = (
                    idxs_blk[jj, pl.ds(k * 16, 16)] + coff)
            dts = pltpu.async_copy(t1s_ref.at[idxs_blk.at[jj]], tsb[b2], sts[b2])
            dtd = pltpu.async_copy(t1d_ref.at[idxd_blk.at[jj]], tdb[b2], std[b2])
            if sacc[b3] is not None:
                sacc[b3].wait()
                sacc[b3] = None
            dh = pltpu.async_copy(h1p_ref.at[adjb[b2]], hrb[b3], sh[b3])
            return dts, dtd, dh

        d = [None, None]
        d[0] = issue(0)
        for j in range(K1):
            cur = j % 2
            nxt = 1 - cur
            c3 = j % 3
            if j + 1 < K1:
                d[nxt] = issue(j + 1)
            g = row0 + j
            realf = jnp.where(g < REAL, 1.0, 0.0).astype(_f32)
            dts, dtd, dh = d[cur]
            dts.wait()
            dtd.wait()
            dh.wait()
            if sden[cur] is not None:
                sden[cur].wait()
                sden[cur] = None
            ts_c, td_c, er_c, ef_c, hr_c = (
                tsb[cur], tdb[cur], erb[cur], efb[cur], hrb[c3])

            @plsc.parallel_loop(0, CH, unroll=4)
            def _(jj):
                al = ts_c[jj, :] + td_c[jj, :]
                lr = jnp.where(al > 0, al, 0.2 * al)
                ev = jnp.exp(lr)
                out = jnp.where(
                    lanev < 4, ev, jnp.where(lanev == 4, al * realf, 0.0))
                er_c[jj, :] = out
                w0 = jnp.sum(jnp.where(lanev == 2 * c, out, 0.0))
                w1 = jnp.sum(jnp.where(lanev == 2 * c + 1, out, 0.0))
                for k in range(4):
                    hr_c[jj, pl.ds(k * 16, 16)] = (
                        hr_c[jj, pl.ds(k * 16, 16)] * w0)
                for k in range(4, 8):
                    hr_c[jj, pl.ds(k * 16, 16)] = (
                        hr_c[jj, pl.ds(k * 16, 16)] * w1)

            sden[cur] = pltpu.async_copy(
                er_c, den_sh.at[idxd_blk.at[j]], sd[cur], add=True)
            sacc[c3] = pltpu.async_copy(
                hr_c, acc1_sh.at[idxd_blk.at[j]], sa[c3], add=True)
        for dd in sden + sacc:
            if dd is not None:
                dd.wait()
        return 0
    lax.fori_loop(0, nblk, blk_body, 0)
    plsc.subcore_barrier()

    rbase = s * STRIPE
    obase = c * NP + rbase
    pltpu.sync_copy(den_sh.at[pl.ds(rbase, STRIPE)], den_o.at[pl.ds(obase, STRIPE)])
    pltpu.sync_copy(acc1_sh.at[pl.ds(rbase, STRIPE)], acc1_o.at[pl.ds(obase, STRIPE)])


def _sc1(src2, dst2, t1s, t1d, h1p):
    return pl.kernel(
        _sc1_body,
        out_type=(
            jax.ShapeDtypeStruct((2 * NP, 16), _f32),
            jax.ShapeDtypeStruct((2 * NP, 128), _f32),
        ),
        mesh=_sc_mesh(),
        **_SC_PARAMS,
        scratch_types=[
            pltpu.VMEM((K1, CH), jnp.int32),
            pltpu.VMEM((K1, CH), jnp.int32),
            pltpu.VMEM((CH,), jnp.int32),
            pltpu.VMEM((CH,), jnp.int32),
            pltpu.VMEM((CH, 16), _f32),
            pltpu.VMEM((CH, 16), _f32),
            pltpu.VMEM((CH, 16), _f32),
            pltpu.VMEM((CH, 16), _f32),
            pltpu.VMEM((CH, 16), _f32),
            pltpu.VMEM((CH, 16), _f32),
            pltpu.VMEM((CH * 16,), _f32),
            pltpu.VMEM((CH * 16,), _f32),
            pltpu.VMEM((CH, 128), _f32),
            pltpu.VMEM((CH, 128), _f32),
            pltpu.VMEM((CH, 128), _f32),
            pltpu.VMEM_SHARED((NP, 16), _f32),
            pltpu.VMEM_SHARED((NP, 128), _f32),
        ] + [pltpu.SemaphoreType.DMA] * 12,
    )(src2, dst2, t1s, t1d, h1p)


def _sc1b_body(src2_ref, dst2_ref, xwl_ref, accs_o,
               idxs_blk, idxd_blk, sg0, sg1, sg2, sg3, accs_sh,
               ss0, ss1, ss2, ss3, sa0, sa1, sa2, sa3):
    c = lax.axis_index("c")
    s = lax.axis_index("s")
    sgb = [sg0, sg1, sg2, sg3]
    ssb = [ss0, ss1, ss2, ss3]
    sab = [sa0, sa1, sa2, sa3]
    NB = 4

    def zrow(j, _):
        z = jnp.zeros((16,), _f32)
        for k in range(4):
            sg0[j, pl.ds(k * 16, 16)] = z
        return 0
    lax.fori_loop(0, CH, zrow, 0)

    def zstripe(k, _):
        pltpu.sync_copy(sg0, accs_sh.at[pl.ds(s * STRIPE + k * CH, CH)])
        return 0
    lax.fori_loop(0, STRIPE // CH, zstripe, 0)
    plsc.subcore_barrier()

    lanev = lax.iota(jnp.int32, 16)
    nct = NCHUNK // NWORK
    nblk = nct // K2
    wid = c * NTILE + s

    def blk_body(bi, _):
        row0 = wid * nct + bi * K2
        pltpu.sync_copy(src2_ref.at[pl.ds(row0, K2)], idxs_blk)
        pltpu.sync_copy(dst2_ref.at[pl.ds(row0, K2)], idxd_blk)
        scat = [None] * NB

        def issue(jj):
            b = jj % NB
            if scat[b] is not None:
                scat[b].wait()
                scat[b] = None
            return pltpu.async_copy(xwl_ref.at[idxs_blk.at[jj]], sgb[b], ssb[b])

        d = {}
        for jj in range(min(NB - 1, K2)):
            d[jj] = issue(jj)
        for j in range(K2):
            b = j % NB
            if j + NB - 1 < K2:
                d[j + NB - 1] = issue(j + NB - 1)
            g = row0 + j
            d[j].wait()

            @pl.when(g >= REAL)
            def _():
                for k in range(CH // 16):
                    idxd_blk[j, pl.ds(k * 16, 16)] = (N + k * 16) + lanev
            scat[b] = pltpu.async_copy(
                sgb[b], accs_sh.at[idxd_blk.at[j]], sab[b], add=True)
        for b in range(NB):
            if scat[b] is not None:
                scat[b].wait()
        return 0
    lax.fori_loop(0, nblk, blk_body, 0)
    plsc.subcore_barrier()

    rbase = s * STRIPE
    pltpu.sync_copy(accs_sh.at[pl.ds(rbase, STRIPE)],
                    accs_o.at[pl.ds(c * NP + rbase, STRIPE)])


def _sc1b(src2, dst2, xwl):
    return pl.kernel(
        _sc1b_body,
        out_type=jax.ShapeDtypeStruct((2 * NP, HID), _f32),
        mesh=_sc_mesh(),
        **_SC_PARAMS,
        scratch_types=[
            pltpu.VMEM((K2, CH), jnp.int32),
            pltpu.VMEM((K2, CH), jnp.int32),
            pltpu.VMEM((CH, HID), _f32),
            pltpu.VMEM((CH, HID), _f32),
            pltpu.VMEM((CH, HID), _f32),
            pltpu.VMEM((CH, HID), _f32),
            pltpu.VMEM_SHARED((NP, HID), _f32),
        ] + [pltpu.SemaphoreType.DMA] * 8,
    )(src2, dst2, xwl)


def _tc2_body(acc1a_ref, acc1b_ref, den_ref, accsa_ref, accsb_ref, xwr_ref,
              bg1_ref, s1c_ref, s1h_ref, wg2_ref, as2_ref, ad2_ref,
              bl1_ref, ssc_ref, ssh_ref, wl2_ref, wr2_ref,
              h2p_ref, t2s_ref, t2d_ref, s1wl2_ref, s1wr2_ref):
    den = den_ref[...]
    mcnt = jnp.maximum(den[:, 4:5], 1.0)
    a = acc1a_ref[...]
    b = acc1b_ref[...]
    g1 = jnp.concatenate([
        a[:, :64] / (den[:, 0:1] + 1e-16),
        a[:, 64:] / (den[:, 1:2] + 1e-16),
        b[:, :64] / (den[:, 2:3] + 1e-16),
        b[:, 64:] / (den[:, 3:4] + 1e-16)], axis=1)
    g1 = g1 + bg1_ref[...]
    g1b = g1 * s1c_ref[...] + s1h_ref[...]
    g1e = jnp.where(g1b > 0, g1b, jnp.exp(g1b) - 1.0)
    h2 = jnp.dot(g1e, wg2_ref[...], preferred_element_type=_f32)
    h2p_ref[...] = h2
    t2s_ref[...] = jnp.dot(h2, as2_ref[...], preferred_element_type=_f32)
    t2d = jnp.dot(h2, ad2_ref[...], preferred_element_type=_f32)
    i = pl.program_id(0)
    rows = lax.broadcasted_iota(jnp.int32, (RB, 16), 0) + i * RB
    lanev = lax.broadcasted_iota(jnp.int32, (RB, 16), 1)
    t2d_ref[...] = jnp.where(
        rows < N, t2d, jnp.where(lanev < 1, -1e30, 0.0))
    accs = accsa_ref[...] + accsb_ref[...]
    s1 = accs / mcnt + bl1_ref[...] + xwr_ref[...]
    s1b = s1 * ssc_ref[...] + ssh_ref[...]
    s1r = jnp.maximum(s1b, 0.0)
    s1wl2_ref[...] = jnp.dot(s1r, wl2_ref[...], preferred_element_type=_f32)
    s1wr2_ref[...] = jnp.dot(s1r, wr2_ref[...], preferred_element_type=_f32)


def _tc2(den_acc, acc1, accs, xwr, bg1, bn1_scale, bn1_shift, wg2, as2_mat,
         ad2_mat, bl1, bns_scale, bns_shift, wl2, wr2):
    full = lambda shape: pl.BlockSpec(shape, lambda i: (0,) * len(shape))
    blk = lambda w: pl.BlockSpec((RB, w), lambda i: (i, 0))
    blk_hi = lambda w: pl.BlockSpec((RB, w), lambda i: (i + GRID, 0))
    return pl.pallas_call(
        _tc2_body,
        grid=(GRID,),
        in_specs=[
            blk(128), blk_hi(128), blk(16), blk(HID), blk_hi(HID), blk(HID),
            full((1, 256)), full((1, 256)), full((1, 256)),
            full((256, HID)), full((HID, 16)), full((HID, 16)),
            full((1, HID)), full((1, HID)), full((1, HID)),
            full((HID, HID)), full((HID, HID)),
        ],
        out_specs=[blk(HID), blk(16), blk(16), blk(HID), blk(HID)],
        out_shape=[
            jax.ShapeDtypeStruct((NP, HID), _f32),
            jax.ShapeDtypeStruct((NP, 16), _f32),
            jax.ShapeDtypeStruct((NP, 16), _f32),
            jax.ShapeDtypeStruct((NP, HID), _f32),
            jax.ShapeDtypeStruct((NP, HID), _f32),
        ],
    )(acc1, acc1, den_acc, accs, accs, xwr, bg1, bn1_scale, bn1_shift,
      wg2, as2_mat, ad2_mat, bl1, bns_scale, bns_shift, wl2, wr2)


def _sc2_body(src2_ref, dst2_ref, t2s_ref, t2d_ref, h2p_ref, swl_ref,
              den_o, acc2_o, accs2_o,
              idxs_blk, idxd_blk, ixg0, ixg1, ixg2, ts0, ts1, td0, td1,
              er0, er1, ef0, ef1, hr0, hr1, hr2, sg0, sg1, sg2,
              den_sh, acc2_sh, accs2_sh,
              sts0, sts1, std0, std1, sh0, sh1, sh2, ss0, ss1, ss2,
              sd0, sd1, sa0, sa1, sa2, sb0, sb1, sb2):
    c = lax.axis_index("c")
    s = lax.axis_index("s")
    tsb = [ts0, ts1]
    tdb = [td0, td1]
    erb = [er0, er1]
    efb = [ef0, ef1]
    hrb = [hr0, hr1, hr2]
    sgb = [sg0, sg1, sg2]
    sts = [sts0, sts1]
    std = [std0, std1]
    sh = [sh0, sh1, sh2]
    ssb = [ss0, ss1, ss2]
    sd = [sd0, sd1]
    sa = [sa0, sa1, sa2]
    sbb = [sb0, sb1, sb2]
    ixg = [ixg0, ixg1, ixg2]

    def zrow(j, _):
        z = jnp.zeros((16,), _f32)
        er0[j, :] = z
        for k in range(4):
            hr0[j, pl.ds(k * 16, 16)] = z
        return 0
    lax.fori_loop(0, CH, zrow, 0)

    def zstripe(k, _):
        base = s * STRIPE + k * CH
        pltpu.sync_copy(er0, den_sh.at[pl.ds(base, CH)])
        pltpu.sync_copy(hr0, acc2_sh.at[pl.ds(base, CH)])
        pltpu.sync_copy(hr0, accs2_sh.at[pl.ds(base, CH)])
        return 0
    lax.fori_loop(0, STRIPE // CH, zstripe, 0)
    plsc.subcore_barrier()

    lanev = lax.iota(jnp.int32, 16)
    zero16i = jnp.zeros((16,), jnp.int32)
    nct = NCHUNK // NWORK
    nblk = nct // K2
    wid = c * NTILE + s

    def blk_body(bi, _):
        row0 = wid * nct + bi * K2
        pltpu.sync_copy(src2_ref.at[pl.ds(row0, K2)], idxs_blk)
        pltpu.sync_copy(dst2_ref.at[pl.ds(row0, K2)], idxd_blk)
        sden = [None, None]
        sacc = [None, None, None]
        ssage = [None, None, None]

        def issue(jj):
            b2 = jj % 2
            b3 = jj % 3
            dts = pltpu.async_copy(t2s_ref.at[idxs_blk.at[jj]], tsb[b2], sts[b2])
            dtd = pltpu.async_copy(t2d_ref.at[idxd_blk.at[jj]], tdb[b2], std[b2])
            if sacc[b3] is not None:
                sacc[b3].wait()
                sacc[b3] = None
            dh = pltpu.async_copy(h2p_ref.at[idxs_blk.at[jj]], hrb[b3], sh[b3])
            if ssage[b3] is not None:
                ssage[b3].wait()
                ssage[b3] = None
            dsg = pltpu.async_copy(swl_ref.at[idxs_blk.at[jj]], sgb[b3], ssb[b3])
            return dts, dtd, dh, dsg

        d = [None, None]
        d[0] = issue(0)
        for j in range(K2):
            cur = j % 2
            nxt = 1 - cur
            c3 = j % 3
            if j + 1 < K2:
                d[nxt] = issue(j + 1)
            g = row0 + j
            dts, dtd, dh, dsg = d[cur]
            dts.wait()
            dtd.wait()
            dh.wait()
            if sden[cur] is not None:
                sden[cur].wait()
                sden[cur] = None
            ts_c, td_c, er_c, ef_c, hr_c = (
                tsb[cur], tdb[cur], erb[cur], efb[cur], hrb[c3])

            @plsc.parallel_loop(0, CH, unroll=4)
            def _(jj):
                al = ts_c[jj, :] + td_c[jj, :]
                lr = jnp.where(al > 0, al, 0.2 * al)
                ev = jnp.exp(lr)
                out = jnp.where(lanev < 1, ev, 0.0)
                er_c[jj, :] = out
                w0 = jnp.sum(jnp.where(lanev == 0, out, 0.0))
                for k in range(4):
                    hr_c[jj, pl.ds(k * 16, 16)] = (
                        hr_c[jj, pl.ds(k * 16, 16)] * w0)

            sden[cur] = pltpu.async_copy(
                er_c, den_sh.at[idxd_blk.at[j]], sd[cur], add=True)
            sacc[c3] = pltpu.async_copy(
                hr_c, acc2_sh.at[idxd_blk.at[j]], sa[c3], add=True)
            dsg.wait()
            for k in range(CH // 16):
                ixg[c3][pl.ds(k * 16, 16)] = idxd_blk[j, pl.ds(k * 16, 16)]

            @pl.when(g >= REAL)
            def _():
                for k in range(CH // 16):
                    ixg[c3][pl.ds(k * 16, 16)] = (N + k * 16) + lanev
            ssage[c3] = pltpu.async_copy(
                sgb[c3], accs2_sh.at[ixg[c3]], sbb[c3], add=True)
        for dd in sden + sacc + ssage:
            if dd is not None:
                dd.wait()
        return 0
    lax.fori_loop(0, nblk, blk_body, 0)
    plsc.subcore_barrier()

    rbase = s * STRIPE
    obase = c * NP + rbase
    pltpu.sync_copy(den_sh.at[pl.ds(rbase, STRIPE)], den_o.at[pl.ds(obase, STRIPE)])
    pltpu.sync_copy(acc2_sh.at[pl.ds(rbase, STRIPE)], acc2_o.at[pl.ds(obase, STRIPE)])
    pltpu.sync_copy(accs2_sh.at[pl.ds(rbase, STRIPE)], accs2_o.at[pl.ds(obase, STRIPE)])


def _sc2(src2, dst2, t2s, t2d, h2p, s1wl2):
    return pl.kernel(
        _sc2_body,
        out_type=(
            jax.ShapeDtypeStruct((2 * NP, 16), _f32),
            jax.ShapeDtypeStruct((2 * NP, HID), _f32),
            jax.ShapeDtypeStruct((2 * NP, HID), _f32),
        ),
        mesh=_sc_mesh(),
        **_SC_PARAMS,
        scratch_types=[
            pltpu.VMEM((K2, CH), jnp.int32),
            pltpu.VMEM((K2, CH), jnp.int32),
            pltpu.VMEM((CH,), jnp.int32),
            pltpu.VMEM((CH,), jnp.int32),
            pltpu.VMEM((CH,), jnp.int32),
            pltpu.VMEM((CH, 16), _f32),
            pltpu.VMEM((CH, 16), _f32),
            pltpu.VMEM((CH, 16), _f32),
            pltpu.VMEM((CH, 16), _f32),
            pltpu.VMEM((CH, 16), _f32),
            pltpu.VMEM((CH, 16), _f32),
            pltpu.VMEM((CH * 16,), _f32),
            pltpu.VMEM((CH * 16,), _f32),
            pltpu.VMEM((CH, HID), _f32),
            pltpu.VMEM((CH, HID), _f32),
            pltpu.VMEM((CH, HID), _f32),
            pltpu.VMEM((CH, HID), _f32),
            pltpu.VMEM((CH, HID), _f32),
            pltpu.VMEM((CH, HID), _f32),
            pltpu.VMEM_SHARED((NP, 16), _f32),
            pltpu.VMEM_SHARED((NP, HID), _f32),
            pltpu.VMEM_SHARED((NP, HID), _f32),
        ] + [pltpu.SemaphoreType.DMA] * 18,
    )(src2, dst2, t2s, t2d, h2p, s1wl2)


def _tc3_body(acc2a_ref, acc2b_ref, den2a_ref, den2b_ref, accs2a_ref,
              accs2b_ref, s1wr2_ref, den_ref, bg2_ref, bl2_ref, wf1_ref,
              bf1_ref, wf2_ref, bf2_ref, out_ref):
    den2 = den2a_ref[...] + den2b_ref[...]
    g2 = (acc2a_ref[...] + acc2b_ref[...]) / (den2[:, 0:1] + 1e-16)
    g2 = g2 + bg2_ref[...]
    mcnt = jnp.maximum(den_ref[:, 4:5], 1.0)
    s2 = (accs2a_ref[...] + accs2b_ref[...]) / mcnt + bl2_ref[...] + s1wr2_ref[...]
    cc = jnp.concatenate([g2, s2], axis=1)
    h = jnp.maximum(jnp.dot(cc, wf1_ref[...], preferred_element_type=_f32)
                    + bf1_ref[...], 0.0)
    out_ref[...] = jnp.dot(h, wf2_ref[...], preferred_element_type=_f32) + bf2_ref[...]


def _tc3(acc2, den2, accs2, s1wr2, den_acc, bg2, bl2, wf1, bf1, wf2p, bf2p):
    full = lambda shape: pl.BlockSpec(shape, lambda i: (0,) * len(shape))
    blk = lambda w: pl.BlockSpec((RB, w), lambda i: (i, 0))
    blk_hi = lambda w: pl.BlockSpec((RB, w), lambda i: (i + GRID, 0))
    return pl.pallas_call(
        _tc3_body,
        grid=(GRID,),
        in_specs=[
            blk(HID), blk_hi(HID), blk(16), blk_hi(16), blk(HID), blk_hi(HID),
            blk(HID), blk(16),
            full((1, HID)), full((1, HID)), full((2 * HID, HID)),
            full((1, HID)), full((HID, 128)), full((1, 128)),
        ],
        out_specs=[pl.BlockSpec((RB, 128), lambda i: (i, 0))],
        out_shape=[jax.ShapeDtypeStruct((N, 128), _f32)],
    )(acc2, acc2, den2, den2, accs2, accs2, s1wr2, den_acc, bg2, bl2, wf1,
      bf1, wf2p, bf2p)


@jax.jit
def kernel(x, edge_index, W_gat1, att_src1, att_dst1, b_gat1, bn1_gamma,
           bn1_beta, bn1_mean, bn1_var, W_gat2, att_src2, att_dst2, b_gat2,
           Wl1, bl1, Wr1, bns_gamma, bns_beta, bns_mean, bns_var, Wl2, bl2,
           Wr2, Wf1, bf1, Wf2, bf2):
    src = edge_index[0].astype(jnp.int32)
    dst = edge_index[1].astype(jnp.int32)
    loops = jnp.arange(N, dtype=jnp.int32)
    padidx = (N + (jnp.arange(EP - E - N, dtype=jnp.int32) % (NP - N)))
    src2 = jnp.concatenate([src, loops, padidx]).reshape(NCHUNK, CH)
    dst2 = jnp.concatenate([dst, loops, padidx]).reshape(NCHUNK, CH)
    xp = jnp.pad(x, ((0, NP - N), (0, 0)))

    eye4 = jnp.eye(HEADS, dtype=_f32)
    as_mat = (att_src1[:, :, None] * eye4[:, None, :]).reshape(256, HEADS)
    as_mat = jnp.concatenate([as_mat, jnp.zeros((256, 12), _f32)], axis=1)
    ad_mat = (att_dst1[:, :, None] * eye4[:, None, :]).reshape(256, HEADS)
    ad_mat = jnp.concatenate([ad_mat, jnp.zeros((256, 12), _f32)], axis=1)
    as2_mat = jnp.concatenate([att_src2.T, jnp.zeros((HID, 15), _f32)], axis=1)
    ad2_mat = jnp.concatenate([att_dst2.T, jnp.zeros((HID, 15), _f32)], axis=1)

    bn1_scale = (bn1_gamma / jnp.sqrt(bn1_var + 1e-5)).reshape(1, 256)
    bn1_shift = (bn1_beta - bn1_mean * bn1_scale[0]).reshape(1, 256)
    bns_scale = (bns_gamma / jnp.sqrt(bns_var + 1e-5)).reshape(1, HID)
    bns_shift = (bns_beta - bns_mean * bns_scale[0]).reshape(1, HID)

    h1p3, t1s, t1d, xwl, xwr = _tc1(xp, W_gat1, as_mat, ad_mat, Wl1, Wr1)
    h1p = h1p3.reshape(2 * NP, 128)

    den_o, acc1_o = _sc1(src2, dst2, t1s, t1d, h1p)
    accs_o = _sc1b(src2, dst2, xwl)

    h2p, t2s, t2d, s1wl2, s1wr2 = _tc2(
        den_o, acc1_o, accs_o, xwr, b_gat1.reshape(1, 256), bn1_scale,
        bn1_shift, W_gat2, as2_mat, ad2_mat, bl1.reshape(1, HID), bns_scale,
        bns_shift, Wl2, Wr2)

    den2_o, acc2_o, accs2_o = _sc2(src2, dst2, t2s, t2d, h2p, s1wl2)

    wf2p = jnp.concatenate([Wf2, jnp.zeros((HID, 126), _f32)], axis=1)
    bf2p = jnp.concatenate([bf2, jnp.zeros((126,), _f32)]).reshape(1, 128)
    outp = _tc3(acc2_o, den2_o, accs2_o, s1wr2, den_o,
                b_gat2.reshape(1, HID), bl2.reshape(1, HID), Wf1,
                bf1.reshape(1, HID), wf2p, bf2p)[0]
    return outp[:, :2]


# speedup vs baseline: 71.2882x; 1.0002x over previous
import jax
import jax.numpy as jnp
from jax import lax
from jax.experimental import pallas as pl
from jax.experimental.pallas import tpu as pltpu
from jax.experimental.pallas import tpu_sc as plsc

N = 10000
NP = 10240
E = 320000
F_IN = 128
HID = 64
HEADS = 4
EP = 331776
CH = 64
NCHUNK = EP // CH
REAL = E // CH
NSC = 2
NTILE = 16
NWORK = NSC * NTILE
STRIPE = NP // NTILE
K1 = 12
K2 = 9
RB = 1024
GRID = NP // RB

_f32 = jnp.float32
_SC_PARAMS = dict(
    compiler_params=pltpu.CompilerParams(
        needs_layout_passes=False, use_tc_tiling_on_sc=False),
)


def _sc_mesh():
    return plsc.VectorSubcoreMesh(core_axis_name="c", subcore_axis_name="s",
                                  num_cores=NSC, num_subcores=NTILE)


def _tc1_body(x_ref, wg1_ref, as_ref, ad_ref, wl1_ref, wr1_ref,
              h1p_ref, t1s_ref, t1d_ref, xwl_ref, xwr_ref):
    i = pl.program_id(0)
    validw = (lax.broadcasted_iota(jnp.int32, (RB, 128), 0) + i * RB) < N
    xb = jnp.where(validw, x_ref[...], 0.0)
    h1 = jnp.dot(xb, wg1_ref[...], preferred_element_type=_f32)
    h1p_ref[0] = h1[:, :128]
    h1p_ref[1] = h1[:, 128:]
    t1s_ref[...] = jnp.dot(h1, as_ref[...], preferred_element_type=_f32)
    t1d = jnp.dot(h1, ad_ref[...], preferred_element_type=_f32)
    rows = lax.broadcasted_iota(jnp.int32, (RB, 16), 0) + i * RB
    lanev = lax.broadcasted_iota(jnp.int32, (RB, 16), 1)
    valid = rows < N
    t1d_ref[...] = jnp.where(
        valid, t1d + (lanev == 4).astype(_f32),
        jnp.where(lanev < 4, -1e30, 0.0))
    xwl_ref[...] = jnp.dot(xb, wl1_ref[...], preferred_element_type=_f32)
    xwr_ref[...] = jnp.dot(xb, wr1_ref[...], preferred_element_type=_f32)


def _tc1(x, wg1, as_mat, ad_mat, wl1, wr1):
    full = lambda shape: pl.BlockSpec(shape, lambda i: (0,) * len(shape))
    return pl.pallas_call(
        _tc1_body,
        grid=(GRID,),
        in_specs=[
            pl.BlockSpec((RB, F_IN), lambda i: (i, 0)),
            full((F_IN, 256)), full((256, 16)), full((256, 16)),
            full((F_IN, HID)), full((F_IN, HID)),
        ],
        out_specs=[
            pl.BlockSpec((2, RB, 128), lambda i: (0, i, 0)),
            pl.BlockSpec((RB, 16), lambda i: (i, 0)),
            pl.BlockSpec((RB, 16), lambda i: (i, 0)),
            pl.BlockSpec((RB, HID), lambda i: (i, 0)),
            pl.BlockSpec((RB, HID), lambda i: (i, 0)),
        ],
        out_shape=[
            jax.ShapeDtypeStruct((2, NP, 128), _f32),
            jax.ShapeDtypeStruct((NP, 16), _f32),
            jax.ShapeDtypeStruct((NP, 16), _f32),
            jax.ShapeDtypeStruct((NP, HID), _f32),
            jax.ShapeDtypeStruct((NP, HID), _f32),
        ],
    )(x, wg1, as_mat, ad_mat, wl1, wr1)


def _sc1_body(src2_ref, dst2_ref, t1s_ref, t1d_ref, h1p_ref,
              den_o, acc1_o,
              idxs_blk, idxd_blk, adj0, adj1, ts0, ts1, td0, td1,
              er0, er1, hr0, hr1, hr2,
              den_sh, acc1_sh,
              sts0, sts1, std0, std1, sh0, sh1, sh2,
              sd0, sd1, sa0, sa1, sa2):
    c = lax.axis_index("c")
    s = lax.axis_index("s")
    adjb = [adj0, adj1]
    tsb = [ts0, ts1]
    tdb = [td0, td1]
    erb = [er0, er1]
    hrb = [hr0, hr1, hr2]
    sts = [sts0, sts1]
    std = [std0, std1]
    sh = [sh0, sh1, sh2]
    sd = [sd0, sd1]
    sa = [sa0, sa1, sa2]

    def zrow(j, _):
        z = jnp.zeros((16,), _f32)
        er0[j, :] = z
        for k in range(8):
            hr0[j, pl.ds(k * 16, 16)] = z
        return 0
    lax.fori_loop(0, CH, zrow, 0)

    def zstripe(k, _):
        base = s * STRIPE + k * CH
        pltpu.sync_copy(er0, den_sh.at[pl.ds(base, CH)])
        pltpu.sync_copy(hr0, acc1_sh.at[pl.ds(base, CH)])
        return 0
    lax.fori_loop(0, STRIPE // CH, zstripe, 0)
    plsc.subcore_barrier()

    lanev = lax.iota(jnp.int32, 16)
    zero16i = jnp.zeros((16,), jnp.int32)
    idxh0v = zero16i + 2 * c
    idxh1v = idxh0v + 1
    coff = c * NP
    nct = NCHUNK // NTILE
    nblk = nct // K1

    def blk_body(bi, _):
        row0 = s * nct + bi * K1
        pltpu.sync_copy(src2_ref.at[pl.ds(row0, K1)], idxs_blk)
        pltpu.sync_copy(dst2_ref.at[pl.ds(row0, K1)], idxd_blk)
        sden = [None, None]
        sacc = [None, None, None]

        def issue(jj):
            b2 = jj % 2
            b3 = jj % 3
            for k in range(CH // 16):
                adjb[b2][pl.ds(k * 16, 16)] = (
                    idxs_blk[jj, pl.ds(k * 16, 16)] + coff)
            dts = pltpu.async_copy(t1s_ref.at[idxs_blk.at[jj]], tsb[b2], sts[b2])
            dtd = pltpu.async_copy(t1d_ref.at[idxd_blk.at[jj]], tdb[b2], std[b2])
            if sacc[b3] is not None:
                sacc[b3].wait()
                sacc[b3] = None
            dh = pltpu.async_copy(h1p_ref.at[adjb[b2]], hrb[b3], sh[b3])
            return dts, dtd, dh

        d = [None, None]
        d[0] = issue(0)
        for j in range(K1):
            cur = j % 2
            nxt = 1 - cur
            c3 = j % 3
            if j + 1 < K1:
                d[nxt] = issue(j + 1)
            g = row0 + j
            realf = jnp.where(g < REAL, 1.0, 0.0).astype(_f32)
            dts, dtd, dh = d[cur]
            dts.wait()
            dtd.wait()
            dh.wait()
            if sden[cur] is not None:
                sden[cur].wait()
                sden[cur] = None
            ts_c, td_c, er_c, hr_c = (
                tsb[cur], tdb[cur], erb[cur], hrb[c3])

            @plsc.parallel_loop(0, CH, unroll=4)
            def _(jj):
                al = ts_c[jj, :] + td_c[jj, :]
                lr = jnp.where(al > 0, al, 0.2 * al)
                ev = jnp.exp(lr)
                out = jnp.where(
                    lanev < 4, ev, jnp.where(lanev == 4, al * realf, 0.0))
                er_c[jj, :] = out
                w0 = jnp.sum(jnp.where(lanev == 2 * c, out, 0.0))
                w1 = jnp.sum(jnp.where(lanev == 2 * c + 1, out, 0.0))
                for k in range(4):
                    hr_c[jj, pl.ds(k * 16, 16)] = (
                        hr_c[jj, pl.ds(k * 16, 16)] * w0)
                for k in range(4, 8):
                    hr_c[jj, pl.ds(k * 16, 16)] = (
                        hr_c[jj, pl.ds(k * 16, 16)] * w1)

            sden[cur] = pltpu.async_copy(
                er_c, den_sh.at[idxd_blk.at[j]], sd[cur], add=True)
            sacc[c3] = pltpu.async_copy(
                hr_c, acc1_sh.at[idxd_blk.at[j]], sa[c3], add=True)
        for dd in sden + sacc:
            if dd is not None:
                dd.wait()
        return 0
    lax.fori_loop(0, nblk, blk_body, 0)
    plsc.subcore_barrier()

    rbase = s * STRIPE
    obase = c * NP + rbase
    pltpu.sync_copy(den_sh.at[pl.ds(rbase, STRIPE)], den_o.at[pl.ds(obase, STRIPE)])
    pltpu.sync_copy(acc1_sh.at[pl.ds(rbase, STRIPE)], acc1_o.at[pl.ds(obase, STRIPE)])


def _sc1(src2, dst2, t1s, t1d, h1p):
    return pl.kernel(
        _sc1_body,
        out_type=(
            jax.ShapeDtypeStruct((2 * NP, 16), _f32),
            jax.ShapeDtypeStruct((2 * NP, 128), _f32),
        ),
        mesh=_sc_mesh(),
        **_SC_PARAMS,
        scratch_types=[
            pltpu.VMEM((K1, CH), jnp.int32),
            pltpu.VMEM((K1, CH), jnp.int32),
            pltpu.VMEM((CH,), jnp.int32),
            pltpu.VMEM((CH,), jnp.int32),
            pltpu.VMEM((CH, 16), _f32),
            pltpu.VMEM((CH, 16), _f32),
            pltpu.VMEM((CH, 16), _f32),
            pltpu.VMEM((CH, 16), _f32),
            pltpu.VMEM((CH, 16), _f32),
            pltpu.VMEM((CH, 16), _f32),
            pltpu.VMEM((CH, 128), _f32),
            pltpu.VMEM((CH, 128), _f32),
            pltpu.VMEM((CH, 128), _f32),
            pltpu.VMEM_SHARED((NP, 16), _f32),
            pltpu.VMEM_SHARED((NP, 128), _f32),
        ] + [pltpu.SemaphoreType.DMA] * 12,
    )(src2, dst2, t1s, t1d, h1p)


def _sc1b_body(src2_ref, dst2_ref, xwl_ref, accs_o,
               idxs_blk, idxd_blk, sg0, sg1, sg2, sg3, accs_sh,
               ss0, ss1, ss2, ss3, sa0, sa1, sa2, sa3):
    c = lax.axis_index("c")
    s = lax.axis_index("s")
    sgb = [sg0, sg1, sg2, sg3]
    ssb = [ss0, ss1, ss2, ss3]
    sab = [sa0, sa1, sa2, sa3]
    NB = 4

    def zrow(j, _):
        z = jnp.zeros((16,), _f32)
        for k in range(4):
            sg0[j, pl.ds(k * 16, 16)] = z
        return 0
    lax.fori_loop(0, CH, zrow, 0)

    def zstripe(k, _):
        pltpu.sync_copy(sg0, accs_sh.at[pl.ds(s * STRIPE + k * CH, CH)])
        return 0
    lax.fori_loop(0, STRIPE // CH, zstripe, 0)
    plsc.subcore_barrier()

    lanev = lax.iota(jnp.int32, 16)
    nct = NCHUNK // NWORK
    nblk = nct // K2
    wid = c * NTILE + s

    def blk_body(bi, _):
        row0 = wid * nct + bi * K2
        pltpu.sync_copy(src2_ref.at[pl.ds(row0, K2)], idxs_blk)
        pltpu.sync_copy(dst2_ref.at[pl.ds(row0, K2)], idxd_blk)
        scat = [None] * NB

        def issue(jj):
            b = jj % NB
            if scat[b] is not None:
                scat[b].wait()
                scat[b] = None
            return pltpu.async_copy(xwl_ref.at[idxs_blk.at[jj]], sgb[b], ssb[b])

        d = {}
        for jj in range(min(NB - 1, K2)):
            d[jj] = issue(jj)
        for j in range(K2):
            b = j % NB
            if j + NB - 1 < K2:
                d[j + NB - 1] = issue(j + NB - 1)
            g = row0 + j
            d[j].wait()

            @pl.when(g >= REAL)
            def _():
                for k in range(CH // 16):
                    idxd_blk[j, pl.ds(k * 16, 16)] = (N + k * 16) + lanev
            scat[b] = pltpu.async_copy(
                sgb[b], accs_sh.at[idxd_blk.at[j]], sab[b], add=True)
        for b in range(NB):
            if scat[b] is not None:
                scat[b].wait()
        return 0
    lax.fori_loop(0, nblk, blk_body, 0)
    plsc.subcore_barrier()

    rbase = s * STRIPE
    pltpu.sync_copy(accs_sh.at[pl.ds(rbase, STRIPE)],
                    accs_o.at[pl.ds(c * NP + rbase, STRIPE)])


def _sc1b(src2, dst2, xwl):
    return pl.kernel(
        _sc1b_body,
        out_type=jax.ShapeDtypeStruct((2 * NP, HID), _f32),
        mesh=_sc_mesh(),
        **_SC_PARAMS,
        scratch_types=[
            pltpu.VMEM((K2, CH), jnp.int32),
            pltpu.VMEM((K2, CH), jnp.int32),
            pltpu.VMEM((CH, HID), _f32),
            pltpu.VMEM((CH, HID), _f32),
            pltpu.VMEM((CH, HID), _f32),
            pltpu.VMEM((CH, HID), _f32),
            pltpu.VMEM_SHARED((NP, HID), _f32),
        ] + [pltpu.SemaphoreType.DMA] * 8,
    )(src2, dst2, xwl)


def _tc2_body(acc1a_ref, acc1b_ref, den_ref, accsa_ref, accsb_ref, xwr_ref,
              bg1_ref, s1c_ref, s1h_ref, wg2_ref, as2_ref, ad2_ref,
              bl1_ref, ssc_ref, ssh_ref, wl2_ref, wr2_ref,
              h2p_ref, t2s_ref, t2d_ref, s1wl2_ref, s1wr2_ref):
    den = den_ref[...]
    mcnt = jnp.maximum(den[:, 4:5], 1.0)
    a = acc1a_ref[...]
    b = acc1b_ref[...]
    g1 = jnp.concatenate([
        a[:, :64] / (den[:, 0:1] + 1e-16),
        a[:, 64:] / (den[:, 1:2] + 1e-16),
        b[:, :64] / (den[:, 2:3] + 1e-16),
        b[:, 64:] / (den[:, 3:4] + 1e-16)], axis=1)
    g1 = g1 + bg1_ref[...]
    g1b = g1 * s1c_ref[...] + s1h_ref[...]
    g1e = jnp.where(g1b > 0, g1b, jnp.exp(g1b) - 1.0)
    h2 = jnp.dot(g1e, wg2_ref[...], preferred_element_type=_f32)
    h2p_ref[...] = h2
    t2s_ref[...] = jnp.dot(h2, as2_ref[...], preferred_element_type=_f32)
    t2d = jnp.dot(h2, ad2_ref[...], preferred_element_type=_f32)
    i = pl.program_id(0)
    rows = lax.broadcasted_iota(jnp.int32, (RB, 16), 0) + i * RB
    lanev = lax.broadcasted_iota(jnp.int32, (RB, 16), 1)
    t2d_ref[...] = jnp.where(
        rows < N, t2d, jnp.where(lanev < 1, -1e30, 0.0))
    accs = accsa_ref[...] + accsb_ref[...]
    s1 = accs / mcnt + bl1_ref[...] + xwr_ref[...]
    s1b = s1 * ssc_ref[...] + ssh_ref[...]
    s1r = jnp.maximum(s1b, 0.0)
    s1wl2_ref[...] = jnp.dot(s1r, wl2_ref[...], preferred_element_type=_f32)
    s1wr2_ref[...] = jnp.dot(s1r, wr2_ref[...], preferred_element_type=_f32)


def _tc2(den_acc, acc1, accs, xwr, bg1, bn1_scale, bn1_shift, wg2, as2_mat,
         ad2_mat, bl1, bns_scale, bns_shift, wl2, wr2):
    full = lambda shape: pl.BlockSpec(shape, lambda i: (0,) * len(shape))
    blk = lambda w: pl.BlockSpec((RB, w), lambda i: (i, 0))
    blk_hi = lambda w: pl.BlockSpec((RB, w), lambda i: (i + GRID, 0))
    return pl.pallas_call(
        _tc2_body,
        grid=(GRID,),
        in_specs=[
            blk(128), blk_hi(128), blk(16), blk(HID), blk_hi(HID), blk(HID),
            full((1, 256)), full((1, 256)), full((1, 256)),
            full((256, HID)), full((HID, 16)), full((HID, 16)),
            full((1, HID)), full((1, HID)), full((1, HID)),
            full((HID, HID)), full((HID, HID)),
        ],
        out_specs=[blk(HID), blk(16), blk(16), blk(HID), blk(HID)],
        out_shape=[
            jax.ShapeDtypeStruct((NP, HID), _f32),
            jax.ShapeDtypeStruct((NP, 16), _f32),
            jax.ShapeDtypeStruct((NP, 16), _f32),
            jax.ShapeDtypeStruct((NP, HID), _f32),
            jax.ShapeDtypeStruct((NP, HID), _f32),
        ],
    )(acc1, acc1, den_acc, accs, accs, xwr, bg1, bn1_scale, bn1_shift,
      wg2, as2_mat, ad2_mat, bl1, bns_scale, bns_shift, wl2, wr2)


def _sc2_body(src2_ref, dst2_ref, t2s_ref, t2d_ref, h2p_ref, swl_ref,
              den_o, acc2_o, accs2_o,
              idxs_blk, idxd_blk, ixg0, ixg1, ixg2, ts0, ts1, td0, td1,
              er0, er1, hr0, hr1, hr2, sg0, sg1, sg2,
              den_sh, acc2_sh, accs2_sh,
              sts0, sts1, std0, std1, sh0, sh1, sh2, ss0, ss1, ss2,
              sd0, sd1, sa0, sa1, sa2, sb0, sb1, sb2):
    c = lax.axis_index("c")
    s = lax.axis_index("s")
    tsb = [ts0, ts1]
    tdb = [td0, td1]
    erb = [er0, er1]
    hrb = [hr0, hr1, hr2]
    sgb = [sg0, sg1, sg2]
    sts = [sts0, sts1]
    std = [std0, std1]
    sh = [sh0, sh1, sh2]
    ssb = [ss0, ss1, ss2]
    sd = [sd0, sd1]
    sa = [sa0, sa1, sa2]
    sbb = [sb0, sb1, sb2]
    ixg = [ixg0, ixg1, ixg2]

    def zrow(j, _):
        z = jnp.zeros((16,), _f32)
        er0[j, :] = z
        for k in range(4):
            hr0[j, pl.ds(k * 16, 16)] = z
        return 0
    lax.fori_loop(0, CH, zrow, 0)

    def zstripe(k, _):
        base = s * STRIPE + k * CH
        pltpu.sync_copy(er0, den_sh.at[pl.ds(base, CH)])
        pltpu.sync_copy(hr0, acc2_sh.at[pl.ds(base, CH)])
        pltpu.sync_copy(hr0, accs2_sh.at[pl.ds(base, CH)])
        return 0
    lax.fori_loop(0, STRIPE // CH, zstripe, 0)
    plsc.subcore_barrier()

    lanev = lax.iota(jnp.int32, 16)
    zero16i = jnp.zeros((16,), jnp.int32)
    nct = NCHUNK // NWORK
    nblk = nct // K2
    wid = c * NTILE + s

    def blk_body(bi, _):
        row0 = wid * nct + bi * K2
        pltpu.sync_copy(src2_ref.at[pl.ds(row0, K2)], idxs_blk)
        pltpu.sync_copy(dst2_ref.at[pl.ds(row0, K2)], idxd_blk)
        sden = [None, None]
        sacc = [None, None, None]
        ssage = [None, None, None]

        def issue(jj):
            b2 = jj % 2
            b3 = jj % 3
            dts = pltpu.async_copy(t2s_ref.at[idxs_blk.at[jj]], tsb[b2], sts[b2])
            dtd = pltpu.async_copy(t2d_ref.at[idxd_blk.at[jj]], tdb[b2], std[b2])
            if sacc[b3] is not None:
                sacc[b3].wait()
                sacc[b3] = None
            dh = pltpu.async_copy(h2p_ref.at[idxs_blk.at[jj]], hrb[b3], sh[b3])
            if ssage[b3] is not None:
                ssage[b3].wait()
                ssage[b3] = None
            dsg = pltpu.async_copy(swl_ref.at[idxs_blk.at[jj]], sgb[b3], ssb[b3])
            return dts, dtd, dh, dsg

        d = [None, None]
        d[0] = issue(0)
        for j in range(K2):
            cur = j % 2
            nxt = 1 - cur
            c3 = j % 3
            if j + 1 < K2:
                d[nxt] = issue(j + 1)
            g = row0 + j
            dts, dtd, dh, dsg = d[cur]
            dts.wait()
            dtd.wait()
            dh.wait()
            if sden[cur] is not None:
                sden[cur].wait()
                sden[cur] = None
            ts_c, td_c, er_c, hr_c = (
                tsb[cur], tdb[cur], erb[cur], hrb[c3])

            @plsc.parallel_loop(0, CH, unroll=4)
            def _(jj):
                al = ts_c[jj, :] + td_c[jj, :]
                lr = jnp.where(al > 0, al, 0.2 * al)
                ev = jnp.exp(lr)
                out = jnp.where(lanev < 1, ev, 0.0)
                er_c[jj, :] = out
                w0 = jnp.sum(jnp.where(lanev == 0, out, 0.0))
                for k in range(4):
                    hr_c[jj, pl.ds(k * 16, 16)] = (
                        hr_c[jj, pl.ds(k * 16, 16)] * w0)

            sden[cur] = pltpu.async_copy(
                er_c, den_sh.at[idxd_blk.at[j]], sd[cur], add=True)
            sacc[c3] = pltpu.async_copy(
                hr_c, acc2_sh.at[idxd_blk.at[j]], sa[c3], add=True)
            dsg.wait()
            for k in range(CH // 16):
                ixg[c3][pl.ds(k * 16, 16)] = idxd_blk[j, pl.ds(k * 16, 16)]

            @pl.when(g >= REAL)
            def _():
                for k in range(CH // 16):
                    ixg[c3][pl.ds(k * 16, 16)] = (N + k * 16) + lanev
            ssage[c3] = pltpu.async_copy(
                sgb[c3], accs2_sh.at[ixg[c3]], sbb[c3], add=True)
        for dd in sden + sacc + ssage:
            if dd is not None:
                dd.wait()
        return 0
    lax.fori_loop(0, nblk, blk_body, 0)
    plsc.subcore_barrier()

    rbase = s * STRIPE
    obase = c * NP + rbase
    pltpu.sync_copy(den_sh.at[pl.ds(rbase, STRIPE)], den_o.at[pl.ds(obase, STRIPE)])
    pltpu.sync_copy(acc2_sh.at[pl.ds(rbase, STRIPE)], acc2_o.at[pl.ds(obase, STRIPE)])
    pltpu.sync_copy(accs2_sh.at[pl.ds(rbase, STRIPE)], accs2_o.at[pl.ds(obase, STRIPE)])


def _sc2(src2, dst2, t2s, t2d, h2p, s1wl2):
    return pl.kernel(
        _sc2_body,
        out_type=(
            jax.ShapeDtypeStruct((2 * NP, 16), _f32),
            jax.ShapeDtypeStruct((2 * NP, HID), _f32),
            jax.ShapeDtypeStruct((2 * NP, HID), _f32),
        ),
        mesh=_sc_mesh(),
        **_SC_PARAMS,
        scratch_types=[
            pltpu.VMEM((K2, CH), jnp.int32),
            pltpu.VMEM((K2, CH), jnp.int32),
            pltpu.VMEM((CH,), jnp.int32),
            pltpu.VMEM((CH,), jnp.int32),
            pltpu.VMEM((CH,), jnp.int32),
            pltpu.VMEM((CH, 16), _f32),
            pltpu.VMEM((CH, 16), _f32),
            pltpu.VMEM((CH, 16), _f32),
            pltpu.VMEM((CH, 16), _f32),
            pltpu.VMEM((CH, 16), _f32),
            pltpu.VMEM((CH, 16), _f32),
            pltpu.VMEM((CH, HID), _f32),
            pltpu.VMEM((CH, HID), _f32),
            pltpu.VMEM((CH, HID), _f32),
            pltpu.VMEM((CH, HID), _f32),
            pltpu.VMEM((CH, HID), _f32),
            pltpu.VMEM((CH, HID), _f32),
            pltpu.VMEM_SHARED((NP, 16), _f32),
            pltpu.VMEM_SHARED((NP, HID), _f32),
            pltpu.VMEM_SHARED((NP, HID), _f32),
        ] + [pltpu.SemaphoreType.DMA] * 18,
    )(src2, dst2, t2s, t2d, h2p, s1wl2)


def _tc3_body(acc2a_ref, acc2b_ref, den2a_ref, den2b_ref, accs2a_ref,
              accs2b_ref, s1wr2_ref, den_ref, bg2_ref, bl2_ref, wf1_ref,
              bf1_ref, wf2_ref, bf2_ref, out_ref):
    den2 = den2a_ref[...] + den2b_ref[...]
    g2 = (acc2a_ref[...] + acc2b_ref[...]) / (den2[:, 0:1] + 1e-16)
    g2 = g2 + bg2_ref[...]
    mcnt = jnp.maximum(den_ref[:, 4:5], 1.0)
    s2 = (accs2a_ref[...] + accs2b_ref[...]) / mcnt + bl2_ref[...] + s1wr2_ref[...]
    cc = jnp.concatenate([g2, s2], axis=1)
    h = jnp.maximum(jnp.dot(cc, wf1_ref[...], preferred_element_type=_f32)
                    + bf1_ref[...], 0.0)
    out_ref[...] = jnp.dot(h, wf2_ref[...], preferred_element_type=_f32) + bf2_ref[...]


def _tc3(acc2, den2, accs2, s1wr2, den_acc, bg2, bl2, wf1, bf1, wf2p, bf2p):
    full = lambda shape: pl.BlockSpec(shape, lambda i: (0,) * len(shape))
    blk = lambda w: pl.BlockSpec((RB, w), lambda i: (i, 0))
    blk_hi = lambda w: pl.BlockSpec((RB, w), lambda i: (i + GRID, 0))
    return pl.pallas_call(
        _tc3_body,
        grid=(GRID,),
        in_specs=[
            blk(HID), blk_hi(HID), blk(16), blk_hi(16), blk(HID), blk_hi(HID),
            blk(HID), blk(16),
            full((1, HID)), full((1, HID)), full((2 * HID, HID)),
            full((1, HID)), full((HID, 128)), full((1, 128)),
        ],
        out_specs=[pl.BlockSpec((RB, 128), lambda i: (i, 0))],
        out_shape=[jax.ShapeDtypeStruct((N, 128), _f32)],
    )(acc2, acc2, den2, den2, accs2, accs2, s1wr2, den_acc, bg2, bl2, wf1,
      bf1, wf2p, bf2p)


@jax.jit
def kernel(x, edge_index, W_gat1, att_src1, att_dst1, b_gat1, bn1_gamma,
           bn1_beta, bn1_mean, bn1_var, W_gat2, att_src2, att_dst2, b_gat2,
           Wl1, bl1, Wr1, bns_gamma, bns_beta, bns_mean, bns_var, Wl2, bl2,
           Wr2, Wf1, bf1, Wf2, bf2):
    src = edge_index[0].astype(jnp.int32)
    dst = edge_index[1].astype(jnp.int32)
    loops = jnp.arange(N, dtype=jnp.int32)
    padidx = (N + (jnp.arange(EP - E - N, dtype=jnp.int32) % (NP - N)))
    src2 = jnp.concatenate([src, loops, padidx]).reshape(NCHUNK, CH)
    dst2 = jnp.concatenate([dst, loops, padidx]).reshape(NCHUNK, CH)

    eye4 = jnp.eye(HEADS, dtype=_f32)
    as_mat = (att_src1[:, :, None] * eye4[:, None, :]).reshape(256, HEADS)
    as_mat = jnp.concatenate([as_mat, jnp.zeros((256, 12), _f32)], axis=1)
    ad_mat = (att_dst1[:, :, None] * eye4[:, None, :]).reshape(256, HEADS)
    ad_mat = jnp.concatenate([ad_mat, jnp.zeros((256, 12), _f32)], axis=1)
    as2_mat = jnp.concatenate([att_src2.T, jnp.zeros((HID, 15), _f32)], axis=1)
    ad2_mat = jnp.concatenate([att_dst2.T, jnp.zeros((HID, 15), _f32)], axis=1)

    bn1_scale = (bn1_gamma / jnp.sqrt(bn1_var + 1e-5)).reshape(1, 256)
    bn1_shift = (bn1_beta - bn1_mean * bn1_scale[0]).reshape(1, 256)
    bns_scale = (bns_gamma / jnp.sqrt(bns_var + 1e-5)).reshape(1, HID)
    bns_shift = (bns_beta - bns_mean * bns_scale[0]).reshape(1, HID)

    h1p3, t1s, t1d, xwl, xwr = _tc1(x, W_gat1, as_mat, ad_mat, Wl1, Wr1)
    h1p = h1p3.reshape(2 * NP, 128)

    den_o, acc1_o = _sc1(src2, dst2, t1s, t1d, h1p)
    accs_o = _sc1b(src2, dst2, xwl)

    h2p, t2s, t2d, s1wl2, s1wr2 = _tc2(
        den_o, acc1_o, accs_o, xwr, b_gat1.reshape(1, 256), bn1_scale,
        bn1_shift, W_gat2, as2_mat, ad2_mat, bl1.reshape(1, HID), bns_scale,
        bns_shift, Wl2, Wr2)

    den2_o, acc2_o, accs2_o = _sc2(src2, dst2, t2s, t2d, h2p, s1wl2)

    wf2p = jnp.concatenate([Wf2, jnp.zeros((HID, 126), _f32)], axis=1)
    bf2p = jnp.concatenate([bf2, jnp.zeros((126,), _f32)]).reshape(1, 128)
    outp = _tc3(acc2_o, den2_o, accs2_o, s1wr2, den_o,
                b_gat2.reshape(1, HID), bl2.reshape(1, HID), Wf1,
                bf1.reshape(1, HID), wf2p, bf2p)[0]
    return outp[:, :2]


# speedup vs baseline: 77.0963x; 1.0815x over previous
import jax
import jax.numpy as jnp
from jax import lax
from jax.experimental import pallas as pl
from jax.experimental.pallas import tpu as pltpu
from jax.experimental.pallas import tpu_sc as plsc

N = 10000
NP = 10240
E = 320000
F_IN = 128
HID = 64
HEADS = 4
EP = 331776
CH = 64
NCHUNK = EP // CH
REAL = E // CH
NSC = 2
NTILE = 16
NWORK = NSC * NTILE
STRIPE = NP // NTILE
K1 = 27
K2 = 18
RB = 1024
GRID = NP // RB

_f32 = jnp.float32
_SC_PARAMS = dict(
    compiler_params=pltpu.CompilerParams(
        needs_layout_passes=False, use_tc_tiling_on_sc=False),
)


def _sc_mesh():
    return plsc.VectorSubcoreMesh(core_axis_name="c", subcore_axis_name="s",
                                  num_cores=NSC, num_subcores=NTILE)


def _tc1_body(x_ref, wg1_ref, as_ref, ad_ref, wl1_ref, wr1_ref,
              h1p_ref, t1s_ref, t1d_ref, xwl_ref, xwr_ref):
    i = pl.program_id(0)
    validw = (lax.broadcasted_iota(jnp.int32, (RB, 128), 0) + i * RB) < N
    xb = jnp.where(validw, x_ref[...], 0.0)
    h1 = jnp.dot(xb, wg1_ref[...], preferred_element_type=_f32)
    h1p_ref[0] = h1[:, :128]
    h1p_ref[1] = h1[:, 128:]
    t1s_ref[...] = jnp.dot(h1, as_ref[...], preferred_element_type=_f32)
    t1d = jnp.dot(h1, ad_ref[...], preferred_element_type=_f32)
    rows = lax.broadcasted_iota(jnp.int32, (RB, 16), 0) + i * RB
    lanev = lax.broadcasted_iota(jnp.int32, (RB, 16), 1)
    valid = rows < N
    t1d_ref[...] = jnp.where(
        valid, t1d + (lanev == 4).astype(_f32),
        jnp.where(lanev < 4, -1e30, 0.0))
    xwl_ref[...] = jnp.dot(xb, wl1_ref[...], preferred_element_type=_f32)
    xwr_ref[...] = jnp.dot(xb, wr1_ref[...], preferred_element_type=_f32)


def _tc1(x, wg1, as_mat, ad_mat, wl1, wr1):
    full = lambda shape: pl.BlockSpec(shape, lambda i: (0,) * len(shape))
    return pl.pallas_call(
        _tc1_body,
        grid=(GRID,),
        in_specs=[
            pl.BlockSpec((RB, F_IN), lambda i: (i, 0)),
            full((F_IN, 256)), full((256, 16)), full((256, 16)),
            full((F_IN, HID)), full((F_IN, HID)),
        ],
        out_specs=[
            pl.BlockSpec((2, RB, 128), lambda i: (0, i, 0)),
            pl.BlockSpec((RB, 16), lambda i: (i, 0)),
            pl.BlockSpec((RB, 16), lambda i: (i, 0)),
            pl.BlockSpec((RB, HID), lambda i: (i, 0)),
            pl.BlockSpec((RB, HID), lambda i: (i, 0)),
        ],
        out_shape=[
            jax.ShapeDtypeStruct((2, NP, 128), _f32),
            jax.ShapeDtypeStruct((NP, 16), _f32),
            jax.ShapeDtypeStruct((NP, 16), _f32),
            jax.ShapeDtypeStruct((NP, HID), _f32),
            jax.ShapeDtypeStruct((NP, HID), _f32),
        ],
    )(x, wg1, as_mat, ad_mat, wl1, wr1)


def _sc1_body(src2_ref, dst2_ref, t1s_ref, t1d_ref, h1p_ref,
              den_o, acc1_o,
              idxs_blk, idxd_blk, adj0, adj1, ts0, ts1, td0, td1,
              er0, er1, hr0, hr1, hr2,
              den_sh, acc1_sh,
              sts0, sts1, std0, std1, sh0, sh1, sh2,
              sd0, sd1, sa0, sa1, sa2):
    c = lax.axis_index("c")
    s = lax.axis_index("s")
    adjb = [adj0, adj1]
    tsb = [ts0, ts1]
    tdb = [td0, td1]
    erb = [er0, er1]
    hrb = [hr0, hr1, hr2]
    sts = [sts0, sts1]
    std = [std0, std1]
    sh = [sh0, sh1, sh2]
    sd = [sd0, sd1]
    sa = [sa0, sa1, sa2]

    def zrow(j, _):
        z = jnp.zeros((16,), _f32)
        er0[j, :] = z
        for k in range(8):
            hr0[j, pl.ds(k * 16, 16)] = z
        return 0
    lax.fori_loop(0, CH, zrow, 0)

    def zstripe(k, _):
        base = s * STRIPE + k * CH
        pltpu.sync_copy(er0, den_sh.at[pl.ds(base, CH)])
        pltpu.sync_copy(hr0, acc1_sh.at[pl.ds(base, CH)])
        return 0
    lax.fori_loop(0, STRIPE // CH, zstripe, 0)
    plsc.subcore_barrier()

    lanev = lax.iota(jnp.int32, 16)
    zero16i = jnp.zeros((16,), jnp.int32)
    idxh0v = zero16i + 2 * c
    idxh1v = idxh0v + 1
    coff = c * NP
    nct = NCHUNK // NTILE
    nblk = nct // K1

    def blk_body(bi, _):
        row0 = s * nct + bi * K1
        pltpu.sync_copy(src2_ref.at[pl.ds(row0, K1)], idxs_blk)
        pltpu.sync_copy(dst2_ref.at[pl.ds(row0, K1)], idxd_blk)
        sden = [None, None]
        sacc = [None, None, None]

        def issue(jj):
            b2 = jj % 2
            b3 = jj % 3
            for k in range(CH // 16):
                adjb[b2][pl.ds(k * 16, 16)] = (
                    idxs_blk[jj, pl.ds(k * 16, 16)] + coff)
            dts = pltpu.async_copy(t1s_ref.at[idxs_blk.at[jj]], tsb[b2], sts[b2])
            dtd = pltpu.async_copy(t1d_ref.at[idxd_blk.at[jj]], tdb[b2], std[b2])
            if sacc[b3] is not None:
                sacc[b3].wait()
                sacc[b3] = None
            dh = pltpu.async_copy(h1p_ref.at[adjb[b2]], hrb[b3], sh[b3])
            return dts, dtd, dh

        d = [None, None]
        d[0] = issue(0)
        for j in range(K1):
            cur = j % 2
            nxt = 1 - cur
            c3 = j % 3
            if j + 1 < K1:
                d[nxt] = issue(j + 1)
            g = row0 + j
            realf = jnp.where(g < REAL, 1.0, 0.0).astype(_f32)
            dts, dtd, dh = d[cur]
            dts.wait()
            dtd.wait()
            dh.wait()
            if sden[cur] is not None:
                sden[cur].wait()
                sden[cur] = None
            ts_c, td_c, er_c, hr_c = (
                tsb[cur], tdb[cur], erb[cur], hrb[c3])

            @plsc.parallel_loop(0, CH, unroll=4)
            def _(jj):
                al = ts_c[jj, :] + td_c[jj, :]
                lr = jnp.where(al > 0, al, 0.2 * al)
                ev = jnp.exp(lr)
                out = jnp.where(
                    lanev < 4, ev, jnp.where(lanev == 4, al * realf, 0.0))
                er_c[jj, :] = out
                w0 = jnp.sum(jnp.where(lanev == 2 * c, out, 0.0))
                w1 = jnp.sum(jnp.where(lanev == 2 * c + 1, out, 0.0))
                for k in range(4):
                    hr_c[jj, pl.ds(k * 16, 16)] = (
                        hr_c[jj, pl.ds(k * 16, 16)] * w0)
                for k in range(4, 8):
                    hr_c[jj, pl.ds(k * 16, 16)] = (
                        hr_c[jj, pl.ds(k * 16, 16)] * w1)

            sden[cur] = pltpu.async_copy(
                er_c, den_sh.at[idxd_blk.at[j]], sd[cur], add=True)
            sacc[c3] = pltpu.async_copy(
                hr_c, acc1_sh.at[idxd_blk.at[j]], sa[c3], add=True)
        for dd in sden + sacc:
            if dd is not None:
                dd.wait()
        return 0
    lax.fori_loop(0, nblk, blk_body, 0)
    plsc.subcore_barrier()

    rbase = s * STRIPE
    obase = c * NP + rbase
    pltpu.sync_copy(den_sh.at[pl.ds(rbase, STRIPE)], den_o.at[pl.ds(obase, STRIPE)])
    pltpu.sync_copy(acc1_sh.at[pl.ds(rbase, STRIPE)], acc1_o.at[pl.ds(obase, STRIPE)])


def _sc1(src2, dst2, t1s, t1d, h1p):
    return pl.kernel(
        _sc1_body,
        out_type=(
            jax.ShapeDtypeStruct((2 * NP, 16), _f32),
            jax.ShapeDtypeStruct((2 * NP, 128), _f32),
        ),
        mesh=_sc_mesh(),
        **_SC_PARAMS,
        scratch_types=[
            pltpu.VMEM((K1, CH), jnp.int32),
            pltpu.VMEM((K1, CH), jnp.int32),
            pltpu.VMEM((CH,), jnp.int32),
            pltpu.VMEM((CH,), jnp.int32),
            pltpu.VMEM((CH, 16), _f32),
            pltpu.VMEM((CH, 16), _f32),
            pltpu.VMEM((CH, 16), _f32),
            pltpu.VMEM((CH, 16), _f32),
            pltpu.VMEM((CH, 16), _f32),
            pltpu.VMEM((CH, 16), _f32),
            pltpu.VMEM((CH, 128), _f32),
            pltpu.VMEM((CH, 128), _f32),
            pltpu.VMEM((CH, 128), _f32),
            pltpu.VMEM_SHARED((NP, 16), _f32),
            pltpu.VMEM_SHARED((NP, 128), _f32),
        ] + [pltpu.SemaphoreType.DMA] * 12,
    )(src2, dst2, t1s, t1d, h1p)


def _sc1b_body(src2_ref, dst2_ref, xwl_ref, accs_o,
               idxs_blk, idxd_blk, sg0, sg1, sg2, sg3, accs_sh,
               ss0, ss1, ss2, ss3, sa0, sa1, sa2, sa3):
    c = lax.axis_index("c")
    s = lax.axis_index("s")
    sgb = [sg0, sg1, sg2, sg3]
    ssb = [ss0, ss1, ss2, ss3]
    sab = [sa0, sa1, sa2, sa3]
    NB = 4

    def zrow(j, _):
        z = jnp.zeros((16,), _f32)
        for k in range(4):
            sg0[j, pl.ds(k * 16, 16)] = z
        return 0
    lax.fori_loop(0, CH, zrow, 0)

    def zstripe(k, _):
        pltpu.sync_copy(sg0, accs_sh.at[pl.ds(s * STRIPE + k * CH, CH)])
        return 0
    lax.fori_loop(0, STRIPE // CH, zstripe, 0)
    plsc.subcore_barrier()

    lanev = lax.iota(jnp.int32, 16)
    nct = NCHUNK // NWORK
    nblk = nct // K2
    wid = c * NTILE + s

    def blk_body(bi, _):
        row0 = wid * nct + bi * K2
        pltpu.sync_copy(src2_ref.at[pl.ds(row0, K2)], idxs_blk)
        pltpu.sync_copy(dst2_ref.at[pl.ds(row0, K2)], idxd_blk)
        scat = [None] * NB

        def issue(jj):
            b = jj % NB
            if scat[b] is not None:
                scat[b].wait()
                scat[b] = None
            return pltpu.async_copy(xwl_ref.at[idxs_blk.at[jj]], sgb[b], ssb[b])

        d = {}
        for jj in range(min(NB - 1, K2)):
            d[jj] = issue(jj)
        for j in range(K2):
            b = j % NB
            if j + NB - 1 < K2:
                d[j + NB - 1] = issue(j + NB - 1)
            g = row0 + j
            d[j].wait()

            @pl.when(g >= REAL)
            def _():
                for k in range(CH // 16):
                    idxd_blk[j, pl.ds(k * 16, 16)] = (N + k * 16) + lanev
            scat[b] = pltpu.async_copy(
                sgb[b], accs_sh.at[idxd_blk.at[j]], sab[b], add=True)
        for b in range(NB):
            if scat[b] is not None:
                scat[b].wait()
        return 0
    lax.fori_loop(0, nblk, blk_body, 0)
    plsc.subcore_barrier()

    rbase = s * STRIPE
    pltpu.sync_copy(accs_sh.at[pl.ds(rbase, STRIPE)],
                    accs_o.at[pl.ds(c * NP + rbase, STRIPE)])


def _sc1b(src2, dst2, xwl):
    return pl.kernel(
        _sc1b_body,
        out_type=jax.ShapeDtypeStruct((2 * NP, HID), _f32),
        mesh=_sc_mesh(),
        **_SC_PARAMS,
        scratch_types=[
            pltpu.VMEM((K2, CH), jnp.int32),
            pltpu.VMEM((K2, CH), jnp.int32),
            pltpu.VMEM((CH, HID), _f32),
            pltpu.VMEM((CH, HID), _f32),
            pltpu.VMEM((CH, HID), _f32),
            pltpu.VMEM((CH, HID), _f32),
            pltpu.VMEM_SHARED((NP, HID), _f32),
        ] + [pltpu.SemaphoreType.DMA] * 8,
    )(src2, dst2, xwl)


def _tc2_body(acc1a_ref, acc1b_ref, den_ref, accsa_ref, accsb_ref, xwr_ref,
              bg1_ref, s1c_ref, s1h_ref, wg2_ref, as2_ref, ad2_ref,
              bl1_ref, ssc_ref, ssh_ref, wl2_ref, wr2_ref,
              h2p_ref, t2s_ref, t2d_ref, s1wl2_ref, s1wr2_ref):
    den = den_ref[...]
    mcnt = jnp.maximum(den[:, 4:5], 1.0)
    a = acc1a_ref[...]
    b = acc1b_ref[...]
    g1 = jnp.concatenate([
        a[:, :64] / (den[:, 0:1] + 1e-16),
        a[:, 64:] / (den[:, 1:2] + 1e-16),
        b[:, :64] / (den[:, 2:3] + 1e-16),
        b[:, 64:] / (den[:, 3:4] + 1e-16)], axis=1)
    g1 = g1 + bg1_ref[...]
    g1b = g1 * s1c_ref[...] + s1h_ref[...]
    g1e = jnp.where(g1b > 0, g1b, jnp.exp(g1b) - 1.0)
    h2 = jnp.dot(g1e, wg2_ref[...], preferred_element_type=_f32)
    h2p_ref[...] = h2
    t2s_ref[...] = jnp.dot(h2, as2_ref[...], preferred_element_type=_f32)
    t2d = jnp.dot(h2, ad2_ref[...], preferred_element_type=_f32)
    i = pl.program_id(0)
    rows = lax.broadcasted_iota(jnp.int32, (RB, 16), 0) + i * RB
    lanev = lax.broadcasted_iota(jnp.int32, (RB, 16), 1)
    t2d_ref[...] = jnp.where(
        rows < N, t2d, jnp.where(lanev < 1, -1e30, 0.0))
    accs = accsa_ref[...] + accsb_ref[...]
    s1 = accs / mcnt + bl1_ref[...] + xwr_ref[...]
    s1b = s1 * ssc_ref[...] + ssh_ref[...]
    s1r = jnp.maximum(s1b, 0.0)
    s1wl2_ref[...] = jnp.dot(s1r, wl2_ref[...], preferred_element_type=_f32)
    s1wr2_ref[...] = jnp.dot(s1r, wr2_ref[...], preferred_element_type=_f32)


def _tc2(den_acc, acc1, accs, xwr, bg1, bn1_scale, bn1_shift, wg2, as2_mat,
         ad2_mat, bl1, bns_scale, bns_shift, wl2, wr2):
    full = lambda shape: pl.BlockSpec(shape, lambda i: (0,) * len(shape))
    blk = lambda w: pl.BlockSpec((RB, w), lambda i: (i, 0))
    blk_hi = lambda w: pl.BlockSpec((RB, w), lambda i: (i + GRID, 0))
    return pl.pallas_call(
        _tc2_body,
        grid=(GRID,),
        in_specs=[
            blk(128), blk_hi(128), blk(16), blk(HID), blk_hi(HID), blk(HID),
            full((1, 256)), full((1, 256)), full((1, 256)),
            full((256, HID)), full((HID, 16)), full((HID, 16)),
            full((1, HID)), full((1, HID)), full((1, HID)),
            full((HID, HID)), full((HID, HID)),
        ],
        out_specs=[blk(HID), blk(16), blk(16), blk(HID), blk(HID)],
        out_shape=[
            jax.ShapeDtypeStruct((NP, HID), _f32),
            jax.ShapeDtypeStruct((NP, 16), _f32),
            jax.ShapeDtypeStruct((NP, 16), _f32),
            jax.ShapeDtypeStruct((NP, HID), _f32),
            jax.ShapeDtypeStruct((NP, HID), _f32),
        ],
    )(acc1, acc1, den_acc, accs, accs, xwr, bg1, bn1_scale, bn1_shift,
      wg2, as2_mat, ad2_mat, bl1, bns_scale, bns_shift, wl2, wr2)


def _sc2_body(src2_ref, dst2_ref, t2s_ref, t2d_ref, h2p_ref, swl_ref,
              den_o, acc2_o, accs2_o,
              idxs_blk, idxd_blk, ixg0, ixg1, ixg2, ts0, ts1, td0, td1,
              er0, er1, hr0, hr1, hr2, sg0, sg1, sg2,
              den_sh, acc2_sh, accs2_sh,
              sts0, sts1, std0, std1, sh0, sh1, sh2, ss0, ss1, ss2,
              sd0, sd1, sa0, sa1, sa2, sb0, sb1, sb2):
    c = lax.axis_index("c")
    s = lax.axis_index("s")
    tsb = [ts0, ts1]
    tdb = [td0, td1]
    erb = [er0, er1]
    hrb = [hr0, hr1, hr2]
    sgb = [sg0, sg1, sg2]
    sts = [sts0, sts1]
    std = [std0, std1]
    sh = [sh0, sh1, sh2]
    ssb = [ss0, ss1, ss2]
    sd = [sd0, sd1]
    sa = [sa0, sa1, sa2]
    sbb = [sb0, sb1, sb2]
    ixg = [ixg0, ixg1, ixg2]

    def zrow(j, _):
        z = jnp.zeros((16,), _f32)
        er0[j, :] = z
        for k in range(4):
            hr0[j, pl.ds(k * 16, 16)] = z
        return 0
    lax.fori_loop(0, CH, zrow, 0)

    def zstripe(k, _):
        base = s * STRIPE + k * CH
        pltpu.sync_copy(er0, den_sh.at[pl.ds(base, CH)])
        pltpu.sync_copy(hr0, acc2_sh.at[pl.ds(base, CH)])
        pltpu.sync_copy(hr0, accs2_sh.at[pl.ds(base, CH)])
        return 0
    lax.fori_loop(0, STRIPE // CH, zstripe, 0)
    plsc.subcore_barrier()

    lanev = lax.iota(jnp.int32, 16)
    zero16i = jnp.zeros((16,), jnp.int32)
    nct = NCHUNK // NWORK
    nblk = nct // K2
    wid = c * NTILE + s

    def blk_body(bi, _):
        row0 = wid * nct + bi * K2
        pltpu.sync_copy(src2_ref.at[pl.ds(row0, K2)], idxs_blk)
        pltpu.sync_copy(dst2_ref.at[pl.ds(row0, K2)], idxd_blk)
        sden = [None, None]
        sacc = [None, None, None]
        ssage = [None, None, None]

        def issue(jj):
            b2 = jj % 2
            b3 = jj % 3
            dts = pltpu.async_copy(t2s_ref.at[idxs_blk.at[jj]], tsb[b2], sts[b2])
            dtd = pltpu.async_copy(t2d_ref.at[idxd_blk.at[jj]], tdb[b2], std[b2])
            if sacc[b3] is not None:
                sacc[b3].wait()
                sacc[b3] = None
            dh = pltpu.async_copy(h2p_ref.at[idxs_blk.at[jj]], hrb[b3], sh[b3])
            if ssage[b3] is not None:
                ssage[b3].wait()
                ssage[b3] = None
            dsg = pltpu.async_copy(swl_ref.at[idxs_blk.at[jj]], sgb[b3], ssb[b3])
            return dts, dtd, dh, dsg

        d = [None, None]
        d[0] = issue(0)
        for j in range(K2):
            cur = j % 2
            nxt = 1 - cur
            c3 = j % 3
            if j + 1 < K2:
                d[nxt] = issue(j + 1)
            g = row0 + j
            dts, dtd, dh, dsg = d[cur]
            dts.wait()
            dtd.wait()
            dh.wait()
            if sden[cur] is not None:
                sden[cur].wait()
                sden[cur] = None
            ts_c, td_c, er_c, hr_c = (
                tsb[cur], tdb[cur], erb[cur], hrb[c3])

            @plsc.parallel_loop(0, CH, unroll=4)
            def _(jj):
                al = ts_c[jj, :] + td_c[jj, :]
                lr = jnp.where(al > 0, al, 0.2 * al)
                ev = jnp.exp(lr)
                out = jnp.where(lanev < 1, ev, 0.0)
                er_c[jj, :] = out
                w0 = jnp.sum(jnp.where(lanev == 0, out, 0.0))
                for k in range(4):
                    hr_c[jj, pl.ds(k * 16, 16)] = (
                        hr_c[jj, pl.ds(k * 16, 16)] * w0)

            sden[cur] = pltpu.async_copy(
                er_c, den_sh.at[idxd_blk.at[j]], sd[cur], add=True)
            sacc[c3] = pltpu.async_copy(
                hr_c, acc2_sh.at[idxd_blk.at[j]], sa[c3], add=True)
            dsg.wait()
            for k in range(CH // 16):
                ixg[c3][pl.ds(k * 16, 16)] = idxd_blk[j, pl.ds(k * 16, 16)]

            @pl.when(g >= REAL)
            def _():
                for k in range(CH // 16):
                    ixg[c3][pl.ds(k * 16, 16)] = (N + k * 16) + lanev
            ssage[c3] = pltpu.async_copy(
                sgb[c3], accs2_sh.at[ixg[c3]], sbb[c3], add=True)
        for dd in sden + sacc + ssage:
            if dd is not None:
                dd.wait()
        return 0
    lax.fori_loop(0, nblk, blk_body, 0)
    plsc.subcore_barrier()

    rbase = s * STRIPE
    obase = c * NP + rbase
    pltpu.sync_copy(den_sh.at[pl.ds(rbase, STRIPE)], den_o.at[pl.ds(obase, STRIPE)])
    pltpu.sync_copy(acc2_sh.at[pl.ds(rbase, STRIPE)], acc2_o.at[pl.ds(obase, STRIPE)])
    pltpu.sync_copy(accs2_sh.at[pl.ds(rbase, STRIPE)], accs2_o.at[pl.ds(obase, STRIPE)])


def _sc2(src2, dst2, t2s, t2d, h2p, s1wl2):
    return pl.kernel(
        _sc2_body,
        out_type=(
            jax.ShapeDtypeStruct((2 * NP, 16), _f32),
            jax.ShapeDtypeStruct((2 * NP, HID), _f32),
            jax.ShapeDtypeStruct((2 * NP, HID), _f32),
        ),
        mesh=_sc_mesh(),
        **_SC_PARAMS,
        scratch_types=[
            pltpu.VMEM((K2, CH), jnp.int32),
            pltpu.VMEM((K2, CH), jnp.int32),
            pltpu.VMEM((CH,), jnp.int32),
            pltpu.VMEM((CH,), jnp.int32),
            pltpu.VMEM((CH,), jnp.int32),
            pltpu.VMEM((CH, 16), _f32),
            pltpu.VMEM((CH, 16), _f32),
            pltpu.VMEM((CH, 16), _f32),
            pltpu.VMEM((CH, 16), _f32),
            pltpu.VMEM((CH, 16), _f32),
            pltpu.VMEM((CH, 16), _f32),
            pltpu.VMEM((CH, HID), _f32),
            pltpu.VMEM((CH, HID), _f32),
            pltpu.VMEM((CH, HID), _f32),
            pltpu.VMEM((CH, HID), _f32),
            pltpu.VMEM((CH, HID), _f32),
            pltpu.VMEM((CH, HID), _f32),
            pltpu.VMEM_SHARED((NP, 16), _f32),
            pltpu.VMEM_SHARED((NP, HID), _f32),
            pltpu.VMEM_SHARED((NP, HID), _f32),
        ] + [pltpu.SemaphoreType.DMA] * 18,
    )(src2, dst2, t2s, t2d, h2p, s1wl2)


def _tc3_body(acc2a_ref, acc2b_ref, den2a_ref, den2b_ref, accs2a_ref,
              accs2b_ref, s1wr2_ref, den_ref, bg2_ref, bl2_ref, wf1_ref,
              bf1_ref, wf2_ref, bf2_ref, out_ref):
    den2 = den2a_ref[...] + den2b_ref[...]
    g2 = (acc2a_ref[...] + acc2b_ref[...]) / (den2[:, 0:1] + 1e-16)
    g2 = g2 + bg2_ref[...]
    mcnt = jnp.maximum(den_ref[:, 4:5], 1.0)
    s2 = (accs2a_ref[...] + accs2b_ref[...]) / mcnt + bl2_ref[...] + s1wr2_ref[...]
    cc = jnp.concatenate([g2, s2], axis=1)
    h = jnp.maximum(jnp.dot(cc, wf1_ref[...], preferred_element_type=_f32)
                    + bf1_ref[...], 0.0)
    out_ref[...] = jnp.dot(h, wf2_ref[...], preferred_element_type=_f32) + bf2_ref[...]


def _tc3(acc2, den2, accs2, s1wr2, den_acc, bg2, bl2, wf1, bf1, wf2p, bf2p):
    full = lambda shape: pl.BlockSpec(shape, lambda i: (0,) * len(shape))
    blk = lambda w: pl.BlockSpec((RB, w), lambda i: (i, 0))
    blk_hi = lambda w: pl.BlockSpec((RB, w), lambda i: (i + GRID, 0))
    return pl.pallas_call(
        _tc3_body,
        grid=(GRID,),
        in_specs=[
            blk(HID), blk_hi(HID), blk(16), blk_hi(16), blk(HID), blk_hi(HID),
            blk(HID), blk(16),
            full((1, HID)), full((1, HID)), full((2 * HID, HID)),
            full((1, HID)), full((HID, 128)), full((1, 128)),
        ],
        out_specs=[pl.BlockSpec((RB, 128), lambda i: (i, 0))],
        out_shape=[jax.ShapeDtypeStruct((N, 128), _f32)],
    )(acc2, acc2, den2, den2, accs2, accs2, s1wr2, den_acc, bg2, bl2, wf1,
      bf1, wf2p, bf2p)


@jax.jit
def kernel(x, edge_index, W_gat1, att_src1, att_dst1, b_gat1, bn1_gamma,
           bn1_beta, bn1_mean, bn1_var, W_gat2, att_src2, att_dst2, b_gat2,
           Wl1, bl1, Wr1, bns_gamma, bns_beta, bns_mean, bns_var, Wl2, bl2,
           Wr2, Wf1, bf1, Wf2, bf2):
    src = edge_index[0].astype(jnp.int32)
    dst = edge_index[1].astype(jnp.int32)
    loops = jnp.arange(N, dtype=jnp.int32)
    padidx = (N + (jnp.arange(EP - E - N, dtype=jnp.int32) % (NP - N)))
    src2 = jnp.concatenate([src, loops, padidx]).reshape(NCHUNK, CH)
    dst2 = jnp.concatenate([dst, loops, padidx]).reshape(NCHUNK, CH)

    eye4 = jnp.eye(HEADS, dtype=_f32)
    as_mat = (att_src1[:, :, None] * eye4[:, None, :]).reshape(256, HEADS)
    as_mat = jnp.concatenate([as_mat, jnp.zeros((256, 12), _f32)], axis=1)
    ad_mat = (att_dst1[:, :, None] * eye4[:, None, :]).reshape(256, HEADS)
    ad_mat = jnp.concatenate([ad_mat, jnp.zeros((256, 12), _f32)], axis=1)
    as2_mat = jnp.concatenate([att_src2.T, jnp.zeros((HID, 15), _f32)], axis=1)
    ad2_mat = jnp.concatenate([att_dst2.T, jnp.zeros((HID, 15), _f32)], axis=1)

    bn1_scale = (bn1_gamma / jnp.sqrt(bn1_var + 1e-5)).reshape(1, 256)
    bn1_shift = (bn1_beta - bn1_mean * bn1_scale[0]).reshape(1, 256)
    bns_scale = (bns_gamma / jnp.sqrt(bns_var + 1e-5)).reshape(1, HID)
    bns_shift = (bns_beta - bns_mean * bns_scale[0]).reshape(1, HID)

    h1p3, t1s, t1d, xwl, xwr = _tc1(x, W_gat1, as_mat, ad_mat, Wl1, Wr1)
    h1p = h1p3.reshape(2 * NP, 128)

    den_o, acc1_o = _sc1(src2, dst2, t1s, t1d, h1p)
    accs_o = _sc1b(src2, dst2, xwl)

    h2p, t2s, t2d, s1wl2, s1wr2 = _tc2(
        den_o, acc1_o, accs_o, xwr, b_gat1.reshape(1, 256), bn1_scale,
        bn1_shift, W_gat2, as2_mat, ad2_mat, bl1.reshape(1, HID), bns_scale,
        bns_shift, Wl2, Wr2)

    den2_o, acc2_o, accs2_o = _sc2(src2, dst2, t2s, t2d, h2p, s1wl2)

    wf2p = jnp.concatenate([Wf2, jnp.zeros((HID, 126), _f32)], axis=1)
    bf2p = jnp.concatenate([bf2, jnp.zeros((126,), _f32)]).reshape(1, 128)
    outp = _tc3(acc2_o, den2_o, accs2_o, s1wr2, den_o,
                b_gat2.reshape(1, HID), bl2.reshape(1, HID), Wf1,
                bf1.reshape(1, HID), wf2p, bf2p)[0]
    return outp[:, :2]


# speedup vs baseline: 78.9922x; 1.0246x over previous
import jax
import jax.numpy as jnp
from jax import lax
from jax.experimental import pallas as pl
from jax.experimental.pallas import tpu as pltpu
from jax.experimental.pallas import tpu_sc as plsc

N = 10000
NP = 10240
E = 320000
F_IN = 128
HID = 64
HEADS = 4
EP = 331776
CH = 64
NCHUNK = EP // CH
REAL = E // CH
NSC = 2
NTILE = 16
NWORK = NSC * NTILE
STRIPE = NP // NTILE
K1 = 27
K2 = 18
RB = 1024
GRID = NP // RB

_f32 = jnp.float32
_SC_PARAMS = dict(
    compiler_params=pltpu.CompilerParams(
        needs_layout_passes=False, use_tc_tiling_on_sc=False),
)


def _sc_mesh():
    return plsc.VectorSubcoreMesh(core_axis_name="c", subcore_axis_name="s",
                                  num_cores=NSC, num_subcores=NTILE)


def _tc1_body(x_ref, wg1_ref, as_ref, ad_ref, wl1_ref, wr1_ref,
              h1p_ref, t1s_ref, t1d_ref, xwl_ref, xwr_ref):
    i = pl.program_id(0)
    validw = (lax.broadcasted_iota(jnp.int32, (RB, 128), 0) + i * RB) < N
    xb = jnp.where(validw, x_ref[...], 0.0)
    h1 = jnp.dot(xb, wg1_ref[...], preferred_element_type=_f32)
    h1p_ref[0] = h1[:, :128]
    h1p_ref[1] = h1[:, 128:]
    t1s_ref[...] = jnp.dot(h1, as_ref[...], preferred_element_type=_f32)
    t1d = jnp.dot(h1, ad_ref[...], preferred_element_type=_f32)
    rows = lax.broadcasted_iota(jnp.int32, (RB, 16), 0) + i * RB
    lanev = lax.broadcasted_iota(jnp.int32, (RB, 16), 1)
    valid = rows < N
    t1d_ref[...] = jnp.where(
        valid, t1d + (lanev == 4).astype(_f32),
        jnp.where(lanev < 4, -1e30, 0.0))
    xwl_ref[...] = jnp.dot(xb, wl1_ref[...], preferred_element_type=_f32)
    xwr_ref[...] = jnp.dot(xb, wr1_ref[...], preferred_element_type=_f32)


def _tc1(x, wg1, as_mat, ad_mat, wl1, wr1):
    full = lambda shape: pl.BlockSpec(shape, lambda i: (0,) * len(shape))
    return pl.pallas_call(
        _tc1_body,
        grid=(GRID,),
        in_specs=[
            pl.BlockSpec((RB, F_IN), lambda i: (i, 0)),
            full((F_IN, 256)), full((256, 16)), full((256, 16)),
            full((F_IN, HID)), full((F_IN, HID)),
        ],
        out_specs=[
            pl.BlockSpec((2, RB, 128), lambda i: (0, i, 0)),
            pl.BlockSpec((RB, 16), lambda i: (i, 0)),
            pl.BlockSpec((RB, 16), lambda i: (i, 0)),
            pl.BlockSpec((RB, HID), lambda i: (i, 0)),
            pl.BlockSpec((RB, HID), lambda i: (i, 0)),
        ],
        out_shape=[
            jax.ShapeDtypeStruct((2, NP, 128), _f32),
            jax.ShapeDtypeStruct((NP, 16), _f32),
            jax.ShapeDtypeStruct((NP, 16), _f32),
            jax.ShapeDtypeStruct((NP, HID), _f32),
            jax.ShapeDtypeStruct((NP, HID), _f32),
        ],
    )(x, wg1, as_mat, ad_mat, wl1, wr1)


def _sc1_body(src2_ref, dst2_ref, t1s_ref, t1d_ref, h1p_ref,
              den_o, acc1_o,
              idxs_blk, idxd_blk, adj0, adj1, ts0, ts1, td0, td1,
              er0, er1, hr0, hr1, hr2,
              den_sh, acc1_sh,
              sts0, sts1, std0, std1, sh0, sh1, sh2,
              sd0, sd1, sa0, sa1, sa2):
    c = lax.axis_index("c")
    s = lax.axis_index("s")
    adjb = [adj0, adj1]
    tsb = [ts0, ts1]
    tdb = [td0, td1]
    erb = [er0, er1]
    hrb = [hr0, hr1, hr2]
    sts = [sts0, sts1]
    std = [std0, std1]
    sh = [sh0, sh1, sh2]
    sd = [sd0, sd1]
    sa = [sa0, sa1, sa2]

    def zrow(j, _):
        z = jnp.zeros((16,), _f32)
        er0[j, :] = z
        for k in range(8):
            hr0[j, pl.ds(k * 16, 16)] = z
        return 0
    lax.fori_loop(0, CH, zrow, 0)

    def zstripe(k, _):
        base = s * STRIPE + k * CH
        pltpu.sync_copy(er0, den_sh.at[pl.ds(base, CH)])
        pltpu.sync_copy(hr0, acc1_sh.at[pl.ds(base, CH)])
        return 0
    lax.fori_loop(0, STRIPE // CH, zstripe, 0)
    plsc.subcore_barrier()

    lanev = lax.iota(jnp.int32, 16)
    zero16i = jnp.zeros((16,), jnp.int32)
    idxh0v = zero16i + 2 * c
    idxh1v = idxh0v + 1
    coff = c * NP
    nct = NCHUNK // NTILE
    nblk = nct // K1

    def blk_body(bi, _):
        row0 = s * nct + bi * K1
        di1 = pltpu.async_copy(src2_ref.at[pl.ds(row0, K1)], idxs_blk, sts[0])
        di2 = pltpu.async_copy(dst2_ref.at[pl.ds(row0, K1)], idxd_blk, std[0])
        di1.wait()
        di2.wait()
        sden = [None, None]
        sacc = [None, None, None]

        def issue(jj):
            b2 = jj % 2
            b3 = jj % 3
            for k in range(CH // 16):
                adjb[b2][pl.ds(k * 16, 16)] = (
                    idxs_blk[jj, pl.ds(k * 16, 16)] + coff)
            dts = pltpu.async_copy(t1s_ref.at[idxs_blk.at[jj]], tsb[b2], sts[b2])
            dtd = pltpu.async_copy(t1d_ref.at[idxd_blk.at[jj]], tdb[b2], std[b2])
            if sacc[b3] is not None:
                sacc[b3].wait()
                sacc[b3] = None
            dh = pltpu.async_copy(h1p_ref.at[adjb[b2]], hrb[b3], sh[b3])
            return dts, dtd, dh

        d = [None, None]
        d[0] = issue(0)
        for j in range(K1):
            cur = j % 2
            nxt = 1 - cur
            c3 = j % 3
            if j + 1 < K1:
                d[nxt] = issue(j + 1)
            g = row0 + j
            realf = jnp.where(g < REAL, 1.0, 0.0).astype(_f32)
            dts, dtd, dh = d[cur]
            dts.wait()
            dtd.wait()
            dh.wait()
            if sden[cur] is not None:
                sden[cur].wait()
                sden[cur] = None
            ts_c, td_c, er_c, hr_c = (
                tsb[cur], tdb[cur], erb[cur], hrb[c3])

            @plsc.parallel_loop(0, CH, unroll=4)
            def _(jj):
                al = ts_c[jj, :] + td_c[jj, :]
                lr = jnp.where(al > 0, al, 0.2 * al)
                ev = jnp.exp(lr)
                out = jnp.where(
                    lanev < 4, ev, jnp.where(lanev == 4, al * realf, 0.0))
                er_c[jj, :] = out
                w0 = jnp.sum(jnp.where(lanev == 2 * c, out, 0.0))
                w1 = jnp.sum(jnp.where(lanev == 2 * c + 1, out, 0.0))
                for k in range(4):
                    hr_c[jj, pl.ds(k * 16, 16)] = (
                        hr_c[jj, pl.ds(k * 16, 16)] * w0)
                for k in range(4, 8):
                    hr_c[jj, pl.ds(k * 16, 16)] = (
                        hr_c[jj, pl.ds(k * 16, 16)] * w1)

            sden[cur] = pltpu.async_copy(
                er_c, den_sh.at[idxd_blk.at[j]], sd[cur], add=True)
            sacc[c3] = pltpu.async_copy(
                hr_c, acc1_sh.at[idxd_blk.at[j]], sa[c3], add=True)
        for dd in sden + sacc:
            if dd is not None:
                dd.wait()
        return 0
    lax.fori_loop(0, nblk, blk_body, 0)
    plsc.subcore_barrier()

    rbase = s * STRIPE
    obase = c * NP + rbase
    pltpu.sync_copy(den_sh.at[pl.ds(rbase, STRIPE)], den_o.at[pl.ds(obase, STRIPE)])
    pltpu.sync_copy(acc1_sh.at[pl.ds(rbase, STRIPE)], acc1_o.at[pl.ds(obase, STRIPE)])


def _sc1(src2, dst2, t1s, t1d, h1p):
    return pl.kernel(
        _sc1_body,
        out_type=(
            jax.ShapeDtypeStruct((2 * NP, 16), _f32),
            jax.ShapeDtypeStruct((2 * NP, 128), _f32),
        ),
        mesh=_sc_mesh(),
        **_SC_PARAMS,
        scratch_types=[
            pltpu.VMEM((K1, CH), jnp.int32),
            pltpu.VMEM((K1, CH), jnp.int32),
            pltpu.VMEM((CH,), jnp.int32),
            pltpu.VMEM((CH,), jnp.int32),
            pltpu.VMEM((CH, 16), _f32),
            pltpu.VMEM((CH, 16), _f32),
            pltpu.VMEM((CH, 16), _f32),
            pltpu.VMEM((CH, 16), _f32),
            pltpu.VMEM((CH, 16), _f32),
            pltpu.VMEM((CH, 16), _f32),
            pltpu.VMEM((CH, 128), _f32),
            pltpu.VMEM((CH, 128), _f32),
            pltpu.VMEM((CH, 128), _f32),
            pltpu.VMEM_SHARED((NP, 16), _f32),
            pltpu.VMEM_SHARED((NP, 128), _f32),
        ] + [pltpu.SemaphoreType.DMA] * 12,
    )(src2, dst2, t1s, t1d, h1p)


def _sc1b_body(src2_ref, dst2_ref, xwl_ref, accs_o,
               idxs_blk, idxd_blk, sg0, sg1, sg2, sg3, accs_sh,
               ss0, ss1, ss2, ss3, sa0, sa1, sa2, sa3):
    c = lax.axis_index("c")
    s = lax.axis_index("s")
    sgb = [sg0, sg1, sg2, sg3]
    ssb = [ss0, ss1, ss2, ss3]
    sab = [sa0, sa1, sa2, sa3]
    NB = 4

    def zrow(j, _):
        z = jnp.zeros((16,), _f32)
        for k in range(4):
            sg0[j, pl.ds(k * 16, 16)] = z
        return 0
    lax.fori_loop(0, CH, zrow, 0)

    def zstripe(k, _):
        pltpu.sync_copy(sg0, accs_sh.at[pl.ds(s * STRIPE + k * CH, CH)])
        return 0
    lax.fori_loop(0, STRIPE // CH, zstripe, 0)
    plsc.subcore_barrier()

    lanev = lax.iota(jnp.int32, 16)
    nct = NCHUNK // NWORK
    nblk = nct // K2
    wid = c * NTILE + s

    def blk_body(bi, _):
        row0 = wid * nct + bi * K2
        di1 = pltpu.async_copy(src2_ref.at[pl.ds(row0, K2)], idxs_blk, ssb[0])
        di2 = pltpu.async_copy(dst2_ref.at[pl.ds(row0, K2)], idxd_blk, ssb[1])
        di1.wait()
        di2.wait()
        scat = [None] * NB

        def issue(jj):
            b = jj % NB
            if scat[b] is not None:
                scat[b].wait()
                scat[b] = None
            return pltpu.async_copy(xwl_ref.at[idxs_blk.at[jj]], sgb[b], ssb[b])

        d = {}
        for jj in range(min(NB - 1, K2)):
            d[jj] = issue(jj)
        for j in range(K2):
            b = j % NB
            if j + NB - 1 < K2:
                d[j + NB - 1] = issue(j + NB - 1)
            g = row0 + j
            d[j].wait()

            @pl.when(g >= REAL)
            def _():
                for k in range(CH // 16):
                    idxd_blk[j, pl.ds(k * 16, 16)] = (N + k * 16) + lanev
            scat[b] = pltpu.async_copy(
                sgb[b], accs_sh.at[idxd_blk.at[j]], sab[b], add=True)
        for b in range(NB):
            if scat[b] is not None:
                scat[b].wait()
        return 0
    lax.fori_loop(0, nblk, blk_body, 0)
    plsc.subcore_barrier()

    rbase = s * STRIPE
    pltpu.sync_copy(accs_sh.at[pl.ds(rbase, STRIPE)],
                    accs_o.at[pl.ds(c * NP + rbase, STRIPE)])


def _sc1b(src2, dst2, xwl):
    return pl.kernel(
        _sc1b_body,
        out_type=jax.ShapeDtypeStruct((2 * NP, HID), _f32),
        mesh=_sc_mesh(),
        **_SC_PARAMS,
        scratch_types=[
            pltpu.VMEM((K2, CH), jnp.int32),
            pltpu.VMEM((K2, CH), jnp.int32),
            pltpu.VMEM((CH, HID), _f32),
            pltpu.VMEM((CH, HID), _f32),
            pltpu.VMEM((CH, HID), _f32),
            pltpu.VMEM((CH, HID), _f32),
            pltpu.VMEM_SHARED((NP, HID), _f32),
        ] + [pltpu.SemaphoreType.DMA] * 8,
    )(src2, dst2, xwl)


def _tc2_body(acc1a_ref, acc1b_ref, den_ref, accsa_ref, accsb_ref, xwr_ref,
              bg1_ref, s1c_ref, s1h_ref, wg2_ref, as2_ref, ad2_ref,
              bl1_ref, ssc_ref, ssh_ref, wl2_ref, wr2_ref,
              h2p_ref, t2s_ref, t2d_ref, s1wl2_ref, s1wr2_ref):
    den = den_ref[...]
    mcnt = jnp.maximum(den[:, 4:5], 1.0)
    a = acc1a_ref[...]
    b = acc1b_ref[...]
    g1 = jnp.concatenate([
        a[:, :64] / (den[:, 0:1] + 1e-16),
        a[:, 64:] / (den[:, 1:2] + 1e-16),
        b[:, :64] / (den[:, 2:3] + 1e-16),
        b[:, 64:] / (den[:, 3:4] + 1e-16)], axis=1)
    g1 = g1 + bg1_ref[...]
    g1b = g1 * s1c_ref[...] + s1h_ref[...]
    g1e = jnp.where(g1b > 0, g1b, jnp.exp(g1b) - 1.0)
    h2 = jnp.dot(g1e, wg2_ref[...], preferred_element_type=_f32)
    h2p_ref[...] = h2
    t2s_ref[...] = jnp.dot(h2, as2_ref[...], preferred_element_type=_f32)
    t2d = jnp.dot(h2, ad2_ref[...], preferred_element_type=_f32)
    i = pl.program_id(0)
    rows = lax.broadcasted_iota(jnp.int32, (RB, 16), 0) + i * RB
    lanev = lax.broadcasted_iota(jnp.int32, (RB, 16), 1)
    t2d_ref[...] = jnp.where(
        rows < N, t2d, jnp.where(lanev < 1, -1e30, 0.0))
    accs = accsa_ref[...] + accsb_ref[...]
    s1 = accs / mcnt + bl1_ref[...] + xwr_ref[...]
    s1b = s1 * ssc_ref[...] + ssh_ref[...]
    s1r = jnp.maximum(s1b, 0.0)
    s1wl2_ref[...] = jnp.dot(s1r, wl2_ref[...], preferred_element_type=_f32)
    s1wr2_ref[...] = jnp.dot(s1r, wr2_ref[...], preferred_element_type=_f32)


def _tc2(den_acc, acc1, accs, xwr, bg1, bn1_scale, bn1_shift, wg2, as2_mat,
         ad2_mat, bl1, bns_scale, bns_shift, wl2, wr2):
    full = lambda shape: pl.BlockSpec(shape, lambda i: (0,) * len(shape))
    blk = lambda w: pl.BlockSpec((RB, w), lambda i: (i, 0))
    blk_hi = lambda w: pl.BlockSpec((RB, w), lambda i: (i + GRID, 0))
    return pl.pallas_call(
        _tc2_body,
        grid=(GRID,),
        in_specs=[
            blk(128), blk_hi(128), blk(16), blk(HID), blk_hi(HID), blk(HID),
            full((1, 256)), full((1, 256)), full((1, 256)),
            full((256, HID)), full((HID, 16)), full((HID, 16)),
            full((1, HID)), full((1, HID)), full((1, HID)),
            full((HID, HID)), full((HID, HID)),
        ],
        out_specs=[blk(HID), blk(16), blk(16), blk(HID), blk(HID)],
        out_shape=[
            jax.ShapeDtypeStruct((NP, HID), _f32),
            jax.ShapeDtypeStruct((NP, 16), _f32),
            jax.ShapeDtypeStruct((NP, 16), _f32),
            jax.ShapeDtypeStruct((NP, HID), _f32),
            jax.ShapeDtypeStruct((NP, HID), _f32),
        ],
    )(acc1, acc1, den_acc, accs, accs, xwr, bg1, bn1_scale, bn1_shift,
      wg2, as2_mat, ad2_mat, bl1, bns_scale, bns_shift, wl2, wr2)


def _sc2_body(src2_ref, dst2_ref, t2s_ref, t2d_ref, h2p_ref, swl_ref,
              den_o, acc2_o, accs2_o,
              idxs_blk, idxd_blk, ixg0, ixg1, ixg2, ts0, ts1, td0, td1,
              er0, er1, hr0, hr1, hr2, sg0, sg1, sg2,
              den_sh, acc2_sh, accs2_sh,
              sts0, sts1, std0, std1, sh0, sh1, sh2, ss0, ss1, ss2,
              sd0, sd1, sa0, sa1, sa2, sb0, sb1, sb2):
    c = lax.axis_index("c")
    s = lax.axis_index("s")
    tsb = [ts0, ts1]
    tdb = [td0, td1]
    erb = [er0, er1]
    hrb = [hr0, hr1, hr2]
    sgb = [sg0, sg1, sg2]
    sts = [sts0, sts1]
    std = [std0, std1]
    sh = [sh0, sh1, sh2]
    ssb = [ss0, ss1, ss2]
    sd = [sd0, sd1]
    sa = [sa0, sa1, sa2]
    sbb = [sb0, sb1, sb2]
    ixg = [ixg0, ixg1, ixg2]

    def zrow(j, _):
        z = jnp.zeros((16,), _f32)
        er0[j, :] = z
        for k in range(4):
            hr0[j, pl.ds(k * 16, 16)] = z
        return 0
    lax.fori_loop(0, CH, zrow, 0)

    def zstripe(k, _):
        base = s * STRIPE + k * CH
        pltpu.sync_copy(er0, den_sh.at[pl.ds(base, CH)])
        pltpu.sync_copy(hr0, acc2_sh.at[pl.ds(base, CH)])
        pltpu.sync_copy(hr0, accs2_sh.at[pl.ds(base, CH)])
        return 0
    lax.fori_loop(0, STRIPE // CH, zstripe, 0)
    plsc.subcore_barrier()

    lanev = lax.iota(jnp.int32, 16)
    zero16i = jnp.zeros((16,), jnp.int32)
    nct = NCHUNK // NWORK
    nblk = nct // K2
    wid = c * NTILE + s

    def blk_body(bi, _):
        row0 = wid * nct + bi * K2
        di1 = pltpu.async_copy(src2_ref.at[pl.ds(row0, K2)], idxs_blk, sts[0])
        di2 = pltpu.async_copy(dst2_ref.at[pl.ds(row0, K2)], idxd_blk, std[0])
        di1.wait()
        di2.wait()
        sden = [None, None]
        sacc = [None, None, None]
        ssage = [None, None, None]

        def issue(jj):
            b2 = jj % 2
            b3 = jj % 3
            dts = pltpu.async_copy(t2s_ref.at[idxs_blk.at[jj]], tsb[b2], sts[b2])
            dtd = pltpu.async_copy(t2d_ref.at[idxd_blk.at[jj]], tdb[b2], std[b2])
            if sacc[b3] is not None:
                sacc[b3].wait()
                sacc[b3] = None
            dh = pltpu.async_copy(h2p_ref.at[idxs_blk.at[jj]], hrb[b3], sh[b3])
            if ssage[b3] is not None:
                ssage[b3].wait()
                ssage[b3] = None
            dsg = pltpu.async_copy(swl_ref.at[idxs_blk.at[jj]], sgb[b3], ssb[b3])
            return dts, dtd, dh, dsg

        d = [None, None]
        d[0] = issue(0)
        for j in range(K2):
            cur = j % 2
            nxt = 1 - cur
            c3 = j % 3
            if j + 1 < K2:
                d[nxt] = issue(j + 1)
            g = row0 + j
            dts, dtd, dh, dsg = d[cur]
            dts.wait()
            dtd.wait()
            dh.wait()
            if sden[cur] is not None:
                sden[cur].wait()
                sden[cur] = None
            ts_c, td_c, er_c, hr_c = (
                tsb[cur], tdb[cur], erb[cur], hrb[c3])

            @plsc.parallel_loop(0, CH, unroll=4)
            def _(jj):
                al = ts_c[jj, :] + td_c[jj, :]
                lr = jnp.where(al > 0, al, 0.2 * al)
                ev = jnp.exp(lr)
                out = jnp.where(lanev < 1, ev, 0.0)
                er_c[jj, :] = out
                w0 = jnp.sum(jnp.where(lanev == 0, out, 0.0))
                for k in range(4):
                    hr_c[jj, pl.ds(k * 16, 16)] = (
                        hr_c[jj, pl.ds(k * 16, 16)] * w0)

            sden[cur] = pltpu.async_copy(
                er_c, den_sh.at[idxd_blk.at[j]], sd[cur], add=True)
            sacc[c3] = pltpu.async_copy(
                hr_c, acc2_sh.at[idxd_blk.at[j]], sa[c3], add=True)
            dsg.wait()
            for k in range(CH // 16):
                ixg[c3][pl.ds(k * 16, 16)] = idxd_blk[j, pl.ds(k * 16, 16)]

            @pl.when(g >= REAL)
            def _():
                for k in range(CH // 16):
                    ixg[c3][pl.ds(k * 16, 16)] = (N + k * 16) + lanev
            ssage[c3] = pltpu.async_copy(
                sgb[c3], accs2_sh.at[ixg[c3]], sbb[c3], add=True)
        for dd in sden + sacc + ssage:
            if dd is not None:
                dd.wait()
        return 0
    lax.fori_loop(0, nblk, blk_body, 0)
    plsc.subcore_barrier()

    rbase = s * STRIPE
    obase = c * NP + rbase
    pltpu.sync_copy(den_sh.at[pl.ds(rbase, STRIPE)], den_o.at[pl.ds(obase, STRIPE)])
    pltpu.sync_copy(acc2_sh.at[pl.ds(rbase, STRIPE)], acc2_o.at[pl.ds(obase, STRIPE)])
    pltpu.sync_copy(accs2_sh.at[pl.ds(rbase, STRIPE)], accs2_o.at[pl.ds(obase, STRIPE)])


def _sc2(src2, dst2, t2s, t2d, h2p, s1wl2):
    return pl.kernel(
        _sc2_body,
        out_type=(
            jax.ShapeDtypeStruct((2 * NP, 16), _f32),
            jax.ShapeDtypeStruct((2 * NP, HID), _f32),
            jax.ShapeDtypeStruct((2 * NP, HID), _f32),
        ),
        mesh=_sc_mesh(),
        **_SC_PARAMS,
        scratch_types=[
            pltpu.VMEM((K2, CH), jnp.int32),
            pltpu.VMEM((K2, CH), jnp.int32),
            pltpu.VMEM((CH,), jnp.int32),
            pltpu.VMEM((CH,), jnp.int32),
            pltpu.VMEM((CH,), jnp.int32),
            pltpu.VMEM((CH, 16), _f32),
            pltpu.VMEM((CH, 16), _f32),
            pltpu.VMEM((CH, 16), _f32),
            pltpu.VMEM((CH, 16), _f32),
            pltpu.VMEM((CH, 16), _f32),
            pltpu.VMEM((CH, 16), _f32),
            pltpu.VMEM((CH, HID), _f32),
            pltpu.VMEM((CH, HID), _f32),
            pltpu.VMEM((CH, HID), _f32),
            pltpu.VMEM((CH, HID), _f32),
            pltpu.VMEM((CH, HID), _f32),
            pltpu.VMEM((CH, HID), _f32),
            pltpu.VMEM_SHARED((NP, 16), _f32),
            pltpu.VMEM_SHARED((NP, HID), _f32),
            pltpu.VMEM_SHARED((NP, HID), _f32),
        ] + [pltpu.SemaphoreType.DMA] * 18,
    )(src2, dst2, t2s, t2d, h2p, s1wl2)


def _tc3_body(acc2a_ref, acc2b_ref, den2a_ref, den2b_ref, accs2a_ref,
              accs2b_ref, s1wr2_ref, den_ref, bg2_ref, bl2_ref, wf1_ref,
              bf1_ref, wf2_ref, bf2_ref, out_ref):
    den2 = den2a_ref[...] + den2b_ref[...]
    g2 = (acc2a_ref[...] + acc2b_ref[...]) / (den2[:, 0:1] + 1e-16)
    g2 = g2 + bg2_ref[...]
    mcnt = jnp.maximum(den_ref[:, 4:5], 1.0)
    s2 = (accs2a_ref[...] + accs2b_ref[...]) / mcnt + bl2_ref[...] + s1wr2_ref[...]
    cc = jnp.concatenate([g2, s2], axis=1)
    h = jnp.maximum(jnp.dot(cc, wf1_ref[...], preferred_element_type=_f32)
                    + bf1_ref[...], 0.0)
    out_ref[...] = jnp.dot(h, wf2_ref[...], preferred_element_type=_f32) + bf2_ref[...]


def _tc3(acc2, den2, accs2, s1wr2, den_acc, bg2, bl2, wf1, bf1, wf2p, bf2p):
    full = lambda shape: pl.BlockSpec(shape, lambda i: (0,) * len(shape))
    blk = lambda w: pl.BlockSpec((RB, w), lambda i: (i, 0))
    blk_hi = lambda w: pl.BlockSpec((RB, w), lambda i: (i + GRID, 0))
    return pl.pallas_call(
        _tc3_body,
        grid=(GRID,),
        in_specs=[
            blk(HID), blk_hi(HID), blk(16), blk_hi(16), blk(HID), blk_hi(HID),
            blk(HID), blk(16),
            full((1, HID)), full((1, HID)), full((2 * HID, HID)),
            full((1, HID)), full((HID, 128)), full((1, 128)),
        ],
        out_specs=[pl.BlockSpec((RB, 128), lambda i: (i, 0))],
        out_shape=[jax.ShapeDtypeStruct((N, 128), _f32)],
    )(acc2, acc2, den2, den2, accs2, accs2, s1wr2, den_acc, bg2, bl2, wf1,
      bf1, wf2p, bf2p)


@jax.jit
def kernel(x, edge_index, W_gat1, att_src1, att_dst1, b_gat1, bn1_gamma,
           bn1_beta, bn1_mean, bn1_var, W_gat2, att_src2, att_dst2, b_gat2,
           Wl1, bl1, Wr1, bns_gamma, bns_beta, bns_mean, bns_var, Wl2, bl2,
           Wr2, Wf1, bf1, Wf2, bf2):
    src = edge_index[0].astype(jnp.int32)
    dst = edge_index[1].astype(jnp.int32)
    loops = jnp.arange(N, dtype=jnp.int32)
    padidx = (N + (jnp.arange(EP - E - N, dtype=jnp.int32) % (NP - N)))
    src2 = jnp.concatenate([src, loops, padidx]).reshape(NCHUNK, CH)
    dst2 = jnp.concatenate([dst, loops, padidx]).reshape(NCHUNK, CH)

    eye4 = jnp.eye(HEADS, dtype=_f32)
    as_mat = (att_src1[:, :, None] * eye4[:, None, :]).reshape(256, HEADS)
    as_mat = jnp.concatenate([as_mat, jnp.zeros((256, 12), _f32)], axis=1)
    ad_mat = (att_dst1[:, :, None] * eye4[:, None, :]).reshape(256, HEADS)
    ad_mat = jnp.concatenate([ad_mat, jnp.zeros((256, 12), _f32)], axis=1)
    as2_mat = jnp.concatenate([att_src2.T, jnp.zeros((HID, 15), _f32)], axis=1)
    ad2_mat = jnp.concatenate([att_dst2.T, jnp.zeros((HID, 15), _f32)], axis=1)

    bn1_scale = (bn1_gamma / jnp.sqrt(bn1_var + 1e-5)).reshape(1, 256)
    bn1_shift = (bn1_beta - bn1_mean * bn1_scale[0]).reshape(1, 256)
    bns_scale = (bns_gamma / jnp.sqrt(bns_var + 1e-5)).reshape(1, HID)
    bns_shift = (bns_beta - bns_mean * bns_scale[0]).reshape(1, HID)

    h1p3, t1s, t1d, xwl, xwr = _tc1(x, W_gat1, as_mat, ad_mat, Wl1, Wr1)
    h1p = h1p3.reshape(2 * NP, 128)

    den_o, acc1_o = _sc1(src2, dst2, t1s, t1d, h1p)
    accs_o = _sc1b(src2, dst2, xwl)

    h2p, t2s, t2d, s1wl2, s1wr2 = _tc2(
        den_o, acc1_o, accs_o, xwr, b_gat1.reshape(1, 256), bn1_scale,
        bn1_shift, W_gat2, as2_mat, ad2_mat, bl1.reshape(1, HID), bns_scale,
        bns_shift, Wl2, Wr2)

    den2_o, acc2_o, accs2_o = _sc2(src2, dst2, t2s, t2d, h2p, s1wl2)

    wf2p = jnp.concatenate([Wf2, jnp.zeros((HID, 126), _f32)], axis=1)
    bf2p = jnp.concatenate([bf2, jnp.zeros((126,), _f32)]).reshape(1, 128)
    outp = _tc3(acc2_o, den2_o, accs2_o, s1wr2, den_o,
                b_gat2.reshape(1, HID), bl2.reshape(1, HID), Wf1,
                bf1.reshape(1, HID), wf2p, bf2p)[0]
    return outp[:, :2]
